# Initial kernel scaffold; baseline (speedup 1.0000x reference)
import jax
import jax.numpy as jnp
from jax.experimental import pallas as pl


def kernel(node_features, edge_index, edge_features, params):
    raise NotImplementedError("write your pallas kernel here")


# speedup vs baseline: 16.4059x; 16.4059x over previous
import functools

import jax
import jax.numpy as jnp
from jax import lax
from jax.experimental import pallas as pl
from jax.experimental.pallas import tpu as pltpu
from jax.experimental.pallas import tpu_sc as plsc

N_NODES = 10000
N_EDGES = 160000
HID = 256
LAT = 128
MAXN = 100
N_PAD = 10112

_LN_EPS = 1e-5


def _silu(x):
    return x * jax.nn.sigmoid(x)


def _lnorm_rows(x, g, b):
    m = jnp.mean(x, axis=-1, keepdims=True)
    v = jnp.mean((x - m) * (x - m), axis=-1, keepdims=True)
    return (x - m) * jax.lax.rsqrt(v + _LN_EPS) * g + b


def _halves(x):
    return jnp.concatenate([x[:, 0:128][None], x[:, 128:256][None]], axis=0)


def _node_enc_body(nf_ref, emb_ref, wce1_ref, bce1_ref, wce2_ref, bce2_ref,
                   wnc_ref, bnc_ref, lng_ref, lnb_ref,
                   wl_ref, bl_ref, wr_ref, br_ref,
                   xl_ref, xr_ref):
    nf = nf_ref[...]
    col = lax.broadcasted_iota(jnp.int32, nf.shape, 1)
    val = jnp.where(col < 100, nf, -1e30)
    rowmax = jnp.max(val, axis=1, keepdims=True)
    idx = jnp.min(jnp.where(val == rowmax, col, 10 ** 9), axis=1)
    onehot = (lax.broadcasted_iota(jnp.int32, (nf.shape[0], 128), 1)
              == idx[:, None]).astype(jnp.float32)
    ef = jnp.dot(onehot, emb_ref[...], preferred_element_type=jnp.float32)
    cf_pre = jnp.dot(nf, wce1_ref[...], preferred_element_type=jnp.float32) + bce1_ref[...]
    cf = jnp.dot(_silu(cf_pre), wce2_ref[...], preferred_element_type=jnp.float32) + bce2_ref[...]
    pre = jnp.dot(jnp.concatenate([ef, cf], axis=1), wnc_ref[...],
                  preferred_element_type=jnp.float32) + bnc_ref[...]
    x = _silu(_lnorm_rows(pre, lng_ref[...], lnb_ref[...]))
    xl = jnp.dot(x, wl_ref[...], preferred_element_type=jnp.float32) + bl_ref[...]
    xr = jnp.dot(x, wr_ref[...], preferred_element_type=jnp.float32) + br_ref[...]
    xl_ref[...] = _halves(xl)
    xr_ref[...] = _halves(xr)


def _node_encoder(nf, emb_p, wce1_p, bce1, wce2t, bce2, wnct, bnc, lng, lnb,
                  wlt, bl, wrt, br):
    Bn = 1264
    grid = (N_PAD // Bn,)
    full = lambda shape: pl.BlockSpec(shape, lambda i: (0,) * len(shape))
    out_shape = (jax.ShapeDtypeStruct((2, N_PAD, 128), jnp.float32),
                 jax.ShapeDtypeStruct((2, N_PAD, 128), jnp.float32))
    return pl.pallas_call(
        _node_enc_body,
        grid=grid,
        in_specs=[
            pl.BlockSpec((Bn, 103), lambda i: (i, 0)),
            full((128, 64)), full((103, 256)), full((1, 256)),
            full((256, 64)), full((1, 64)),
            full((128, 256)), full((1, 256)), full((1, 256)), full((1, 256)),
            full((256, 256)), full((1, 256)), full((256, 256)), full((1, 256)),
        ],
        out_specs=(pl.BlockSpec((2, Bn, 128), lambda i: (0, i, 0)),
                   pl.BlockSpec((2, Bn, 128), lambda i: (0, i, 0))),
        out_shape=out_shape,
        compiler_params=pltpu.CompilerParams(
            dimension_semantics=("parallel",)),
    )(nf, emb_p, wce1_p, bce1, wce2t, bce2, wnct, bnc, lng, lnb,
      wlt, bl, wrt, br)


def _edge_enc_body(ef_ref, wde1_ref, bde1_ref, wde2_ref, bde2_ref,
                   wpe1_ref, bpe1_ref, wpe2_ref, bpe2_ref,
                   wec_ref, bec_ref, lng_ref, lnb_ref,
                   we1_ref, we2_ref, we3_ref,
                   e1_ref, e2_ref, e3_ref, easum_ref):
    ef = ef_ref[...]
    de_pre = jnp.dot(ef, wde1_ref[...], preferred_element_type=jnp.float32) + bde1_ref[...]
    de = jnp.dot(_silu(de_pre), wde2_ref[...], preferred_element_type=jnp.float32) + bde2_ref[...]
    pe_pre = jnp.dot(ef, wpe1_ref[...], preferred_element_type=jnp.float32) + bpe1_ref[...]
    pe = jnp.dot(_silu(pe_pre), wpe2_ref[...], preferred_element_type=jnp.float32) + bpe2_ref[...]
    pre = jnp.dot(jnp.concatenate([de, pe], axis=1), wec_ref[...],
                  preferred_element_type=jnp.float32) + bec_ref[...]
    ea = _silu(_lnorm_rows(pre, lng_ref[...], lnb_ref[...]))
    e1_ref[...] = _halves(
        jnp.dot(ea, we1_ref[...], preferred_element_type=jnp.float32))
    e2_ref[...] = _halves(
        jnp.dot(ea, we2_ref[...], preferred_element_type=jnp.float32))
    e3_ref[...] = _halves(
        jnp.dot(ea, we3_ref[...], preferred_element_type=jnp.float32))
    part = jnp.sum(ea, axis=0, keepdims=True)
    @pl.when(pl.program_id(0) == 0)
    def _():
        easum_ref[...] = jnp.zeros_like(easum_ref)
    easum_ref[...] += part


def _edge_encoder(ef, wde1_p, bde1, wde2t, bde2, wpe1_p, bpe1, wpe2t, bpe2,
                  wect, bec, lng, lnb, we1t, we2t, we3t):
    Be = 2000
    grid = (N_EDGES // Be,)
    full = lambda shape: pl.BlockSpec(shape, lambda i: (0,) * len(shape))
    out_shape = (jax.ShapeDtypeStruct((2, N_EDGES, 128), jnp.float32),
                 jax.ShapeDtypeStruct((2, N_EDGES, 128), jnp.float32),
                 jax.ShapeDtypeStruct((2, N_EDGES, 128), jnp.float32),
                 jax.ShapeDtypeStruct((1, 256), jnp.float32))
    espec = pl.BlockSpec((2, Be, 128), lambda i: (0, i, 0))
    return pl.pallas_call(
        _edge_enc_body,
        grid=grid,
        in_specs=[
            pl.BlockSpec((Be, 4), lambda i: (i, 0)),
            full((4, 128)), full((1, 128)), full((128, 32)), full((1, 32)),
            full((4, 128)), full((1, 128)), full((128, 32)), full((1, 32)),
            full((64, 256)), full((1, 256)), full((1, 256)), full((1, 256)),
            full((256, 256)), full((256, 256)), full((256, 256)),
        ],
        out_specs=(espec, espec, espec,
                   pl.BlockSpec((1, 256), lambda i: (0, 0))),
        out_shape=out_shape,
        compiler_params=pltpu.CompilerParams(
            dimension_semantics=("arbitrary",)),
    )(ef, wde1_p, bde1, wde2t, bde2, wpe1_p, bpe1, wpe2t, bpe2,
      wect, bec, lng, lnb, we1t, we2t, we3t)


def _make_gat_sc(sub):
    SUB = sub
    SV = (128 // SUB) // 16
    EPT = N_EDGES // 16
    B = 40
    NCH = EPT // B
    SCH = 10
    ZR = 8
    RPT = N_PAD // 16

    mesh = plsc.VectorSubcoreMesh(core_axis_name="c", subcore_axis_name="s")

    @functools.partial(
        pl.kernel,
        out_type=(jax.ShapeDtypeStruct((2, N_PAD, 128), jnp.float32),
                  jax.ShapeDtypeStruct((2, 16, SUB * N_PAD), jnp.float32)),
        mesh=mesh,
        compiler_params=pltpu.CompilerParams(needs_layout_passes=False),
        scratch_types=[
            pltpu.VMEM((SCH * B,), jnp.int32),
            pltpu.VMEM((SCH * B,), jnp.int32),
            pltpu.VMEM((B,), jnp.int32),
            pltpu.VMEM((B,), jnp.int32),
            pltpu.VMEM((B, 128), jnp.float32),
            pltpu.VMEM((B, 128), jnp.float32),
            pltpu.VMEM((B, 128), jnp.float32),
            pltpu.VMEM((B, 128), jnp.float32),
            pltpu.VMEM((B, 128), jnp.float32),
            pltpu.VMEM((SUB * N_PAD,), jnp.float32),
            pltpu.VMEM((128,), jnp.float32),
            pltpu.VMEM_SHARED((N_PAD, 128), jnp.float32),
            pltpu.SemaphoreType.DMA,
            pltpu.SemaphoreType.DMA,
        ],
    )
    def gat_sc(xl_hbm, xr_hbm, el_hbm, gsrc_hbm, gdst_hbm, dst_hbm, att_hbm,
               acc_out, den_out,
               jidx_s, iidx_s, dstv0, dstv1,
               xjv0, xiv0, xjv1, xiv1, elv, den_t, attv,
               acc_s, sem0, sem1):
        c = lax.axis_index("c")
        s = lax.axis_index("s")
        zrow16 = jnp.zeros((16,), jnp.float32)
        lanes = lax.iota(jnp.int32, 16)
        bufs = ((xjv0, xiv0, dstv0, sem0), (xjv1, xiv1, dstv1, sem1))

        def zfill(i, _):
            for k in range(8):
                xjv0[i, pl.ds(k * 16, 16)] = zrow16
            return 0
        lax.fori_loop(0, ZR, zfill, 0)

        def zden(i, _):
            den_t[pl.ds(i * 16, 16)] = zrow16
            return 0
        lax.fori_loop(0, SUB * N_PAD // 16, zden, 0)

        r0 = s * RPT

        def zcopy(j, _):
            pltpu.sync_copy(xjv0.at[pl.ds(0, ZR)],
                            acc_s.at[pl.ds(r0 + j * ZR, ZR)])
            return 0
        lax.fori_loop(0, RPT // ZR, zcopy, 0)

        plsc.subcore_barrier()

        pltpu.sync_copy(att_hbm.at[pl.ds(c * 128, 128)], attv)
        att_regs = [attv[pl.ds(k * 16, 16)] for k in range(8)]

        def issue(j, b):
            xj, xi, dv, sem = bufs[b]
            jc = jnp.minimum(j, SCH - 1)
            pltpu.async_copy(xl_hbm.at[jidx_s.at[pl.ds(jc * B, B)]], xj, sem)
            pltpu.async_copy(xr_hbm.at[iidx_s.at[pl.ds(jc * B, B)]], xi, sem)

        def wait(b):
            xj, xi, dv, sem = bufs[b]
            pltpu.make_async_copy(xl_hbm.at[jidx_s.at[pl.ds(0, B)]], xj,
                                  sem).wait()
            pltpu.make_async_copy(xr_hbm.at[iidx_s.at[pl.ds(0, B)]], xi,
                                  sem).wait()

        def compute(sbase, j, b):
            xj, xi, dv, sem = bufs[b]
            pltpu.sync_copy(dst_hbm.at[pl.ds(sbase + j * B, B)], dv)
            pltpu.sync_copy(
                el_hbm.at[pl.ds(c * N_EDGES + sbase + j * B, B)], elv)

            def edge(i2, _):
                accs = []
                for u in range(2):
                    i = i2 * 2 + u
                    for sb in range(SUB):
                        a = jnp.zeros((16,), jnp.float32)
                        for k in range(SV):
                            kk = sb * SV + k
                            sl = pl.ds(kk * 16, 16)
                            svv = xj[i, sl] + xi[i, sl] + elv[i, sl]
                            m = jnp.where(svv > 0, svv, 0.2 * svv)
                            a = a + m * att_regs[kk]
                        accs.append(a)
                tvs = [jnp.exp(jnp.broadcast_to(jnp.sum(a), (16,)))
                       for a in accs]
                for u in range(2):
                    i = i2 * 2 + u
                    g16 = (i // 16) * 16
                    dvec = dv[pl.ds(g16, 16)]
                    lmask = lanes == (i - g16)
                    for sb in range(SUB):
                        tv = tvs[u * SUB + sb]
                        for k in range(SV):
                            kk = sb * SV + k
                            sl = pl.ds(kk * 16, 16)
                            xj[i, sl] = xj[i, sl] * tv
                        plsc.addupdate_scatter(
                            den_t, [dvec * SUB + sb], tv, mask=lmask)
                return 0
            lax.fori_loop(0, B // 2, edge, 0)

            pltpu.sync_copy(xj, acc_s.at[dv], add=True)

        def superchunk(su, _):
            sbase = s * EPT + su * SCH * B
            pltpu.sync_copy(
                gsrc_hbm.at[pl.ds(c * N_EDGES + sbase, SCH * B)], jidx_s)
            pltpu.sync_copy(
                gdst_hbm.at[pl.ds(c * N_EDGES + sbase, SCH * B)], iidx_s)
            issue(jnp.int32(0), 0)
            issue(jnp.int32(1), 1)

            def pairq(q, _):
                j0 = 2 * q
                wait(0)
                compute(sbase, j0, 0)
                issue(j0 + 2, 0)
                wait(1)
                compute(sbase, j0 + 1, 1)
                issue(j0 + 3, 1)
                return 0
            lax.fori_loop(0, SCH // 2, pairq, 0)
            wait(0)
            wait(1)
            return 0
        lax.fori_loop(0, NCH // SCH, superchunk, 0)

        plsc.subcore_barrier()

        pltpu.sync_copy(acc_s.at[pl.ds(s * RPT, RPT)],
                        acc_out.at[c, pl.ds(s * RPT, RPT)])
        pltpu.sync_copy(den_t, den_out.at[c, s])

    return gat_sc


def _norm_x(acc_ref, den_ref, bias_ref, SUB):
    cols = []
    for c in range(2):
        dpart = den_ref[c]
        dsum = jnp.sum(dpart, axis=0)
        if SUB == 2:
            cols.append(acc_ref[c][:, 0:64] / (dsum[:, 0:1] + 1e-16))
            cols.append(acc_ref[c][:, 64:128] / (dsum[:, 1:2] + 1e-16))
        else:
            cols.append(acc_ref[c] / (dsum[:, 0:1] + 1e-16))
    return jnp.maximum(jnp.concatenate(cols, axis=1) + bias_ref[...], 0.0)


def _combine_proj_body(acc_ref, den_ref, bias_ref, wl_ref, bl_ref,
                       wr_ref, br_ref, xl_ref, xr_ref, *, SUB):
    x = _norm_x(acc_ref, den_ref, bias_ref, SUB)
    xl = jnp.dot(x, wl_ref[...], preferred_element_type=jnp.float32) + bl_ref[...]
    xr = jnp.dot(x, wr_ref[...], preferred_element_type=jnp.float32) + br_ref[...]
    xl_ref[...] = _halves(xl)
    xr_ref[...] = _halves(xr)


def _combine_proj(acc, den, bias, wlt, bl, wrt, br, SUB):
    Bn = 1264
    grid = (N_PAD // Bn,)
    full = lambda shape: pl.BlockSpec(shape, lambda i: (0,) * len(shape))
    out_shape = (jax.ShapeDtypeStruct((2, N_PAD, 128), jnp.float32),
                 jax.ShapeDtypeStruct((2, N_PAD, 128), jnp.float32))
    return pl.pallas_call(
        functools.partial(_combine_proj_body, SUB=SUB),
        grid=grid,
        in_specs=[
            pl.BlockSpec((2, Bn, 128), lambda i: (0, i, 0)),
            pl.BlockSpec((2, 16, Bn, SUB), lambda i: (0, 0, i, 0)),
            full((1, 256)),
            full((256, 256)), full((1, 256)), full((256, 256)), full((1, 256)),
        ],
        out_specs=(pl.BlockSpec((2, Bn, 128), lambda i: (0, i, 0)),
                   pl.BlockSpec((2, Bn, 128), lambda i: (0, i, 0))),
        out_shape=out_shape,
        compiler_params=pltpu.CompilerParams(
            dimension_semantics=("parallel",)),
    )(acc, den, bias, wlt, bl, wrt, br)


def _combine_final_body(acc_ref, den_ref, bias_ref, xsum_ref, *, SUB, Bn):
    x = _norm_x(acc_ref, den_ref, bias_ref, SUB)
    ridx = (pl.program_id(0) * Bn
            + lax.broadcasted_iota(jnp.int32, x.shape, 0))
    x = jnp.where(ridx < N_NODES, x, 0.0)
    part = jnp.sum(x, axis=0, keepdims=True)
    @pl.when(pl.program_id(0) == 0)
    def _():
        xsum_ref[...] = jnp.zeros_like(xsum_ref)
    xsum_ref[...] += part


def _combine_final(acc, den, bias, SUB):
    Bn = 1264
    grid = (N_PAD // Bn,)
    return pl.pallas_call(
        functools.partial(_combine_final_body, SUB=SUB, Bn=Bn),
        grid=grid,
        in_specs=[
            pl.BlockSpec((2, Bn, 128), lambda i: (0, i, 0)),
            pl.BlockSpec((2, 16, Bn, SUB), lambda i: (0, 0, i, 0)),
            pl.BlockSpec((1, 256), lambda i: (0, 0)),
        ],
        out_specs=pl.BlockSpec((1, 256), lambda i: (0, 0)),
        out_shape=jax.ShapeDtypeStruct((1, 256), jnp.float32),
        compiler_params=pltpu.CompilerParams(
            dimension_semantics=("arbitrary",)),
    )(acc, den, bias)


def _dec1_body(xsum_ref, easum_ref, eps_ref,
               fp_ref, bfp_ref, fplg_ref, fplb_ref,
               mu_ref_w, bmu_ref, lv_ref_w, blv_ref,
               l1_ref, bl1_ref, l1g_ref, l1b_ref,
               l2_ref, bl2_ref, l2g_ref, l2b_ref,
               a1_ref, ba1_ref, c1_ref, bc1_ref, c2_ref, bc2_ref,
               e1_ref, be1_ref, n1_ref, bn1_ref, n2_ref, bn2_ref,
               p1_ref, bp1_ref, p2_ref, bp2_ref,
               h_ref, mu_ref, lv_ref, aact_ref, eact_ref,
               nn_ref, coords_ref, cell_ref):
    g = jnp.concatenate([xsum_ref[...] / N_NODES, easum_ref[...] / N_EDGES],
                        axis=1)
    g = _silu(_lnorm_rows(
        jnp.dot(g, fp_ref[...], preferred_element_type=jnp.float32) + bfp_ref[...],
        fplg_ref[...], fplb_ref[...]))
    mu = jnp.dot(g, mu_ref_w[...], preferred_element_type=jnp.float32) + bmu_ref[...]
    lv = jnp.dot(g, lv_ref_w[...], preferred_element_type=jnp.float32) + blv_ref[...]
    z = mu + eps_ref[...] * jnp.exp(0.5 * lv)
    h1 = _silu(_lnorm_rows(
        jnp.dot(z, l1_ref[...], preferred_element_type=jnp.float32) + bl1_ref[...],
        l1g_ref[...], l1b_ref[...]))
    h = _silu(_lnorm_rows(
        jnp.dot(h1, l2_ref[...], preferred_element_type=jnp.float32) + bl2_ref[...],
        l2g_ref[...], l2b_ref[...]))
    aact = _silu(jnp.dot(h, a1_ref[...], preferred_element_type=jnp.float32) + ba1_ref[...])
    cact = _silu(jnp.dot(h, c1_ref[...], preferred_element_type=jnp.float32) + bc1_ref[...])
    coords = jnp.dot(cact, c2_ref[...], preferred_element_type=jnp.float32) + bc2_ref[...]
    eact = _silu(jnp.dot(h, e1_ref[...], preferred_element_type=jnp.float32) + be1_ref[...])
    nact = _silu(jnp.dot(h, n1_ref[...], preferred_element_type=jnp.float32) + bn1_ref[...])
    nn = jnp.dot(nact, n2_ref[...], preferred_element_type=jnp.float32) + bn2_ref[...]
    pact = _silu(jnp.dot(h, p1_ref[...], preferred_element_type=jnp.float32) + bp1_ref[...])
    cp = jnp.dot(pact, p2_ref[...], preferred_element_type=jnp.float32) + bp2_ref[...]
    lane = lax.broadcasted_iota(jnp.int32, cp.shape, 1)
    sp = jnp.log1p(jnp.exp(-jnp.abs(cp))) + jnp.maximum(cp, 0.0)
    sg = 30.0 + 120.0 * jax.nn.sigmoid(cp)
    cell = jnp.where(lane < 3, sp, sg)
    h_ref[...] = h
    mu_ref[...] = mu
    lv_ref[...] = lv
    aact_ref[...] = aact
    eact_ref[...] = eact
    nn_ref[...] = nn
    coords_ref[...] = coords
    cell_ref[...] = cell


def _decoder_small(xsum, easum, eps, w):
    full = lambda shape: pl.BlockSpec(shape, lambda: (0,) * len(shape))
    out_shape = (jax.ShapeDtypeStruct((1, 512), jnp.float32),
                 jax.ShapeDtypeStruct((1, 128), jnp.float32),
                 jax.ShapeDtypeStruct((1, 128), jnp.float32),
                 jax.ShapeDtypeStruct((1, 256), jnp.float32),
                 jax.ShapeDtypeStruct((1, 512), jnp.float32),
                 jax.ShapeDtypeStruct((1, 100), jnp.float32),
                 jax.ShapeDtypeStruct((1, 300), jnp.float32),
                 jax.ShapeDtypeStruct((1, 8), jnp.float32))
    args = [xsum, easum, eps] + w
    return pl.pallas_call(
        _dec1_body,
        in_specs=[full(a.shape) for a in args],
        out_specs=tuple(full(s.shape) for s in out_shape),
        out_shape=out_shape,
    )(*args)


def _dec2_body(aact_ref, eact_ref, wa_ref, ba_ref, we_ref, be_ref,
               atom_ref, edge_ref):
    atom_ref[...] = jnp.dot(aact_ref[...], wa_ref[...],
                            preferred_element_type=jnp.float32) + ba_ref[...]
    edge_ref[...] = jnp.dot(eact_ref[...], we_ref[...],
                            preferred_element_type=jnp.float32) + be_ref[...]


def _decoder_wide(aact, eact, wat, ba, wet, be):
    T = 1280
    grid = (10240 // T,)
    full = lambda shape: pl.BlockSpec(shape, lambda i: (0,) * len(shape))
    out_shape = (jax.ShapeDtypeStruct((1, 10240), jnp.float32),
                 jax.ShapeDtypeStruct((1, 10240), jnp.float32))
    return pl.pallas_call(
        _dec2_body,
        grid=grid,
        in_specs=[
            full((1, 256)), full((1, 512)),
            pl.BlockSpec((256, T), lambda i: (0, i)),
            pl.BlockSpec((1, T), lambda i: (0, i)),
            pl.BlockSpec((512, T), lambda i: (0, i)),
            pl.BlockSpec((1, T), lambda i: (0, i)),
        ],
        out_specs=(pl.BlockSpec((1, T), lambda i: (0, i)),
                   pl.BlockSpec((1, T), lambda i: (0, i))),
        out_shape=out_shape,
        compiler_params=pltpu.CompilerParams(
            dimension_semantics=("parallel",)),
    )(aact, eact, wat, ba, wet, be)


def _row(v):
    return v.reshape(1, -1)


def kernel(node_features, edge_index, edge_features, params):
    p = params
    src = edge_index[0]
    dst = edge_index[1]

    emb_p = jnp.zeros((128, 64), jnp.float32).at[:100].set(p["ne_emb"])
    wce1_p = jnp.zeros((103, 256), jnp.float32).at[100:103].set(p["ne_ce1"]["W"].T)
    wde1_p = jnp.zeros((4, 128), jnp.float32).at[0:1].set(p["ee_de1"]["W"].T)
    wpe1_p = jnp.zeros((4, 128), jnp.float32).at[1:4].set(p["ee_pe1"]["W"].T)

    c1, c2, c3 = p["c1"], p["c2"], p["c3"]

    nf_pad = jnp.zeros((N_PAD, 103), jnp.float32).at[:N_NODES].set(node_features)
    xl1, xr1 = _node_encoder(
        nf_pad, emb_p, wce1_p, _row(p["ne_ce1"]["b"]),
        p["ne_ce2"]["W"].T, _row(p["ne_ce2"]["b"]),
        p["ne_nc"]["W"].T, _row(p["ne_nc"]["b"]),
        _row(p["ne_ln"]["g"]), _row(p["ne_ln"]["b"]),
        c1["Wl"]["W"].T, _row(c1["Wl"]["b"]),
        c1["Wr"]["W"].T, _row(c1["Wr"]["b"]))

    e1, e2, e3, ea_sum = _edge_encoder(
        edge_features, wde1_p, _row(p["ee_de1"]["b"]),
        p["ee_de2"]["W"].T, _row(p["ee_de2"]["b"]),
        wpe1_p, _row(p["ee_pe1"]["b"]),
        p["ee_pe2"]["W"].T, _row(p["ee_pe2"]["b"]),
        p["ee_ec"]["W"].T, _row(p["ee_ec"]["b"]),
        _row(p["ee_ln"]["g"]), _row(p["ee_ln"]["b"]),
        c1["We"]["W"].T, c2["We"]["W"].T, c3["We"]["W"].T)

    gat12 = _make_gat_sc(2)
    gat3 = _make_gat_sc(1)

    gsrc = jnp.concatenate([src, src + N_PAD])
    gdst = jnp.concatenate([dst, dst + N_PAD])

    def run_gat(gat, xl, xr, el, att, SUB):
        acc, den = gat(xl.reshape(2 * N_PAD, 128), xr.reshape(2 * N_PAD, 128),
                       el.reshape(2 * N_EDGES, 128), gsrc, gdst, dst,
                       att.reshape(-1))
        return acc, den.reshape(2, 16, N_PAD, SUB)

    acc1, den1 = run_gat(gat12, xl1, xr1, e1, c1["att"], 2)
    xl2, xr2 = _combine_proj(acc1, den1, _row(c1["bias"]),
                             c2["Wl"]["W"].T, _row(c2["Wl"]["b"]),
                             c2["Wr"]["W"].T, _row(c2["Wr"]["b"]), SUB=2)

    acc2, den2 = run_gat(gat12, xl2, xr2, e2, c2["att"], 2)
    xl3, xr3 = _combine_proj(acc2, den2, _row(c2["bias"]),
                             c3["Wl"]["W"].T, _row(c3["Wl"]["b"]),
                             c3["Wr"]["W"].T, _row(c3["Wr"]["b"]), SUB=2)

    acc3, den3 = run_gat(gat3, xl3, xr3, e3, c3["att"], 1)
    xsum = _combine_final(acc3, den3, _row(c3["bias"]), SUB=1)

    eps = jax.random.normal(jax.random.key(42), (1, LAT), dtype=jnp.float32)

    dec_w = [
        p["fp"]["W"].T, _row(p["fp"]["b"]),
        _row(p["fp_ln"]["g"]), _row(p["fp_ln"]["b"]),
        p["mu"]["W"].T, _row(p["mu"]["b"]),
        p["lv"]["W"].T, _row(p["lv"]["b"]),
        p["d_l1"]["W"].T, _row(p["d_l1"]["b"]),
        _row(p["d_ln1"]["g"]), _row(p["d_ln1"]["b"]),
        p["d_l2"]["W"].T, _row(p["d_l2"]["b"]),
        _row(p["d_ln2"]["g"]), _row(p["d_ln2"]["b"]),
        p["d_a1"]["W"].T, _row(p["d_a1"]["b"]),
        p["d_c1"]["W"].T, _row(p["d_c1"]["b"]),
        p["d_c2"]["W"].T, _row(p["d_c2"]["b"]),
        p["d_e1"]["W"].T, _row(p["d_e1"]["b"]),
        p["d_n1"]["W"].T, _row(p["d_n1"]["b"]),
        p["d_n2"]["W"].T, _row(p["d_n2"]["b"]),
        p["d_p1"]["W"].T, _row(p["d_p1"]["b"]),
        jnp.zeros((256, 8), jnp.float32).at[:, :6].set(p["d_p2"]["W"].T),
        jnp.zeros((1, 8), jnp.float32).at[:, :6].set(_row(p["d_p2"]["b"])),
    ]
    (h, mu, log_var, aact, eact, nn_logits, coords, cell8) = _decoder_small(
        xsum, ea_sum, eps, dec_w)

    wa_p = jnp.zeros((256, 10240), jnp.float32).at[:, :10000].set(p["d_a2"]["W"].T)
    we_p = jnp.zeros((512, 10240), jnp.float32).at[:, :10000].set(p["d_e2"]["W"].T)
    ba_p = jnp.zeros((1, 10240), jnp.float32).at[:, :10000].set(_row(p["d_a2"]["b"]))
    be_p = jnp.zeros((1, 10240), jnp.float32).at[:, :10000].set(_row(p["d_e2"]["b"]))
    atom_p, edge_p = _decoder_wide(aact, eact, wa_p, ba_p, we_p, be_p)
    atom = atom_p[:, :10000]
    edge_flat = edge_p[:, :10000]

    node_out = jnp.concatenate(
        [atom.reshape(1, MAXN, 100), coords.reshape(1, MAXN, 3)], axis=-1)
    edge_logits = edge_flat.reshape(1, MAXN, MAXN)
    cell = cell8[:, :6]
    return (node_out, edge_logits, nn_logits, cell, h, mu, log_var)


# speedup vs baseline: 17.4149x; 1.0615x over previous
import functools

import jax
import jax.numpy as jnp
from jax import lax
from jax.experimental import pallas as pl
from jax.experimental.pallas import tpu as pltpu
from jax.experimental.pallas import tpu_sc as plsc

N_NODES = 10000
N_EDGES = 160000
HID = 256
LAT = 128
MAXN = 100
N_PAD = 10112

_LN_EPS = 1e-5


def _silu(x):
    return x * jax.nn.sigmoid(x)


def _lnorm_rows(x, g, b):
    m = jnp.mean(x, axis=-1, keepdims=True)
    v = jnp.mean((x - m) * (x - m), axis=-1, keepdims=True)
    return (x - m) * jax.lax.rsqrt(v + _LN_EPS) * g + b


def _halves(x):
    return jnp.concatenate([x[:, 0:128][None], x[:, 128:256][None]], axis=0)


def _node_enc_body(nf_ref, emb_ref, wce1_ref, bce1_ref, wce2_ref, bce2_ref,
                   wnc_ref, bnc_ref, lng_ref, lnb_ref,
                   wl_ref, bl_ref, wr_ref, br_ref,
                   xl_ref, xr_ref):
    nf = nf_ref[...]
    col = lax.broadcasted_iota(jnp.int32, nf.shape, 1)
    val = jnp.where(col < 100, nf, -1e30)
    rowmax = jnp.max(val, axis=1, keepdims=True)
    idx = jnp.min(jnp.where(val == rowmax, col, 10 ** 9), axis=1)
    onehot = (lax.broadcasted_iota(jnp.int32, (nf.shape[0], 128), 1)
              == idx[:, None]).astype(jnp.float32)
    ef = jnp.dot(onehot, emb_ref[...], preferred_element_type=jnp.float32)
    cf_pre = jnp.dot(nf, wce1_ref[...], preferred_element_type=jnp.float32) + bce1_ref[...]
    cf = jnp.dot(_silu(cf_pre), wce2_ref[...], preferred_element_type=jnp.float32) + bce2_ref[...]
    pre = jnp.dot(jnp.concatenate([ef, cf], axis=1), wnc_ref[...],
                  preferred_element_type=jnp.float32) + bnc_ref[...]
    x = _silu(_lnorm_rows(pre, lng_ref[...], lnb_ref[...]))
    xl = jnp.dot(x, wl_ref[...], preferred_element_type=jnp.float32) + bl_ref[...]
    xr = jnp.dot(x, wr_ref[...], preferred_element_type=jnp.float32) + br_ref[...]
    xl_ref[...] = _halves(xl)
    xr_ref[...] = _halves(xr)


def _node_encoder(nf, emb_p, wce1_p, bce1, wce2t, bce2, wnct, bnc, lng, lnb,
                  wlt, bl, wrt, br):
    Bn = 1264
    grid = (N_PAD // Bn,)
    full = lambda shape: pl.BlockSpec(shape, lambda i: (0,) * len(shape))
    out_shape = (jax.ShapeDtypeStruct((2, N_PAD, 128), jnp.float32),
                 jax.ShapeDtypeStruct((2, N_PAD, 128), jnp.float32))
    return pl.pallas_call(
        _node_enc_body,
        grid=grid,
        in_specs=[
            pl.BlockSpec((Bn, 103), lambda i: (i, 0)),
            full((128, 64)), full((103, 256)), full((1, 256)),
            full((256, 64)), full((1, 64)),
            full((128, 256)), full((1, 256)), full((1, 256)), full((1, 256)),
            full((256, 256)), full((1, 256)), full((256, 256)), full((1, 256)),
        ],
        out_specs=(pl.BlockSpec((2, Bn, 128), lambda i: (0, i, 0)),
                   pl.BlockSpec((2, Bn, 128), lambda i: (0, i, 0))),
        out_shape=out_shape,
        compiler_params=pltpu.CompilerParams(
            dimension_semantics=("parallel",)),
    )(nf, emb_p, wce1_p, bce1, wce2t, bce2, wnct, bnc, lng, lnb,
      wlt, bl, wrt, br)


def _edge_enc_body(ef_ref, wde1_ref, bde1_ref, wde2_ref, bde2_ref,
                   wpe1_ref, bpe1_ref, wpe2_ref, bpe2_ref,
                   wec_ref, bec_ref, lng_ref, lnb_ref,
                   we1_ref, we2_ref, we3_ref,
                   e1_ref, e2_ref, e3_ref, easum_ref):
    ef = ef_ref[...]
    de_pre = jnp.dot(ef, wde1_ref[...], preferred_element_type=jnp.float32) + bde1_ref[...]
    de = jnp.dot(_silu(de_pre), wde2_ref[...], preferred_element_type=jnp.float32) + bde2_ref[...]
    pe_pre = jnp.dot(ef, wpe1_ref[...], preferred_element_type=jnp.float32) + bpe1_ref[...]
    pe = jnp.dot(_silu(pe_pre), wpe2_ref[...], preferred_element_type=jnp.float32) + bpe2_ref[...]
    pre = jnp.dot(jnp.concatenate([de, pe], axis=1), wec_ref[...],
                  preferred_element_type=jnp.float32) + bec_ref[...]
    ea = _silu(_lnorm_rows(pre, lng_ref[...], lnb_ref[...]))
    e1_ref[...] = _halves(
        jnp.dot(ea, we1_ref[...], preferred_element_type=jnp.float32))
    e2_ref[...] = _halves(
        jnp.dot(ea, we2_ref[...], preferred_element_type=jnp.float32))
    e3_ref[...] = _halves(
        jnp.dot(ea, we3_ref[...], preferred_element_type=jnp.float32))
    part = jnp.sum(ea, axis=0, keepdims=True)
    @pl.when(pl.program_id(0) == 0)
    def _():
        easum_ref[...] = jnp.zeros_like(easum_ref)
    easum_ref[...] += part


def _edge_encoder(ef, wde1_p, bde1, wde2t, bde2, wpe1_p, bpe1, wpe2t, bpe2,
                  wect, bec, lng, lnb, we1t, we2t, we3t):
    Be = 2000
    grid = (N_EDGES // Be,)
    full = lambda shape: pl.BlockSpec(shape, lambda i: (0,) * len(shape))
    out_shape = (jax.ShapeDtypeStruct((2, N_EDGES, 128), jnp.float32),
                 jax.ShapeDtypeStruct((2, N_EDGES, 128), jnp.float32),
                 jax.ShapeDtypeStruct((2, N_EDGES, 128), jnp.float32),
                 jax.ShapeDtypeStruct((1, 256), jnp.float32))
    espec = pl.BlockSpec((2, Be, 128), lambda i: (0, i, 0))
    return pl.pallas_call(
        _edge_enc_body,
        grid=grid,
        in_specs=[
            pl.BlockSpec((Be, 4), lambda i: (i, 0)),
            full((4, 128)), full((1, 128)), full((128, 32)), full((1, 32)),
            full((4, 128)), full((1, 128)), full((128, 32)), full((1, 32)),
            full((64, 256)), full((1, 256)), full((1, 256)), full((1, 256)),
            full((256, 256)), full((256, 256)), full((256, 256)),
        ],
        out_specs=(espec, espec, espec,
                   pl.BlockSpec((1, 256), lambda i: (0, 0))),
        out_shape=out_shape,
        compiler_params=pltpu.CompilerParams(
            dimension_semantics=("arbitrary",)),
    )(ef, wde1_p, bde1, wde2t, bde2, wpe1_p, bpe1, wpe2t, bpe2,
      wect, bec, lng, lnb, we1t, we2t, we3t)


def _make_gat_sc(sub):
    SUB = sub
    SV = (128 // SUB) // 16
    EPT = N_EDGES // 16
    B = 40
    NCH = EPT // B
    SCH = 10
    ZR = 8
    RPT = N_PAD // 16

    mesh = plsc.VectorSubcoreMesh(core_axis_name="c", subcore_axis_name="s")

    @functools.partial(
        pl.kernel,
        out_type=(jax.ShapeDtypeStruct((2, N_PAD, 128), jnp.float32),
                  jax.ShapeDtypeStruct((2, 16, SUB * N_PAD), jnp.float32)),
        mesh=mesh,
        compiler_params=pltpu.CompilerParams(needs_layout_passes=False),
        scratch_types=[
            pltpu.VMEM((SCH * B,), jnp.int32),
            pltpu.VMEM((SCH * B,), jnp.int32),
            pltpu.VMEM((SCH * B,), jnp.int32),
            pltpu.VMEM((B, 128), jnp.float32),
            pltpu.VMEM((B, 128), jnp.float32),
            pltpu.VMEM((B, 128), jnp.float32),
            pltpu.VMEM((B, 128), jnp.float32),
            pltpu.VMEM((B, 128), jnp.float32),
            pltpu.VMEM((SUB * N_PAD,), jnp.float32),
            pltpu.VMEM((128,), jnp.float32),
            pltpu.VMEM_SHARED((N_PAD, 128), jnp.float32),
            pltpu.SemaphoreType.DMA,
            pltpu.SemaphoreType.DMA,
        ],
    )
    def gat_sc(xl_hbm, xr_hbm, el_hbm, gsrc_hbm, gdst_hbm, dst_hbm, att_hbm,
               acc_out, den_out,
               jidx_s, iidx_s, dst_s,
               xjv0, xiv0, xjv1, xiv1, elv, den_t, attv,
               acc_s, sem0, sem1):
        c = lax.axis_index("c")
        s = lax.axis_index("s")
        zrow16 = jnp.zeros((16,), jnp.float32)
        lanes = lax.iota(jnp.int32, 16)
        bufs = ((xjv0, xiv0, sem0), (xjv1, xiv1, sem1))

        def zfill(i, _):
            for k in range(8):
                xjv0[i, pl.ds(k * 16, 16)] = zrow16
            return 0
        lax.fori_loop(0, ZR, zfill, 0)

        def zden(i, _):
            den_t[pl.ds(i * 16, 16)] = zrow16
            return 0
        lax.fori_loop(0, SUB * N_PAD // 16, zden, 0)

        r0 = s * RPT

        def zcopy(j, _):
            pltpu.sync_copy(xjv0.at[pl.ds(0, ZR)],
                            acc_s.at[pl.ds(r0 + j * ZR, ZR)])
            return 0
        lax.fori_loop(0, RPT // ZR, zcopy, 0)

        plsc.subcore_barrier()

        pltpu.sync_copy(att_hbm.at[pl.ds(c * 128, 128)], attv)
        att_regs = [attv[pl.ds(k * 16, 16)] for k in range(8)]

        def issue(j, b):
            xj, xi, sem = bufs[b]
            jc = jnp.minimum(j, SCH - 1)
            pltpu.async_copy(xl_hbm.at[jidx_s.at[pl.ds(jc * B, B)]], xj, sem)
            pltpu.async_copy(xr_hbm.at[iidx_s.at[pl.ds(jc * B, B)]], xi, sem)

        def wait(b):
            xj, xi, sem = bufs[b]
            pltpu.make_async_copy(xl_hbm.at[jidx_s.at[pl.ds(0, B)]], xj,
                                  sem).wait()
            pltpu.make_async_copy(xr_hbm.at[iidx_s.at[pl.ds(0, B)]], xi,
                                  sem).wait()

        def compute(sbase, j, b):
            xj, xi, sem = bufs[b]
            pltpu.sync_copy(
                el_hbm.at[pl.ds(c * N_EDGES + sbase + j * B, B)], elv)

            def edge(i2, _):
                accs = []
                for u in range(2):
                    i = i2 * 2 + u
                    for sb in range(SUB):
                        a = jnp.zeros((16,), jnp.float32)
                        for k in range(SV):
                            kk = sb * SV + k
                            sl = pl.ds(kk * 16, 16)
                            svv = xj[i, sl] + xi[i, sl] + elv[i, sl]
                            m = jnp.where(svv > 0, svv, 0.2 * svv)
                            a = a + m * att_regs[kk]
                        accs.append(a)
                tvs = [jnp.exp(jnp.broadcast_to(jnp.sum(a), (16,)))
                       for a in accs]
                for u in range(2):
                    i = i2 * 2 + u
                    g16 = (i // 16) * 16
                    dvec = dst_s[pl.ds(j * B + g16, 16)]
                    lmask = lanes == (i - g16)
                    for sb in range(SUB):
                        tv = tvs[u * SUB + sb]
                        for k in range(SV):
                            kk = sb * SV + k
                            sl = pl.ds(kk * 16, 16)
                            xj[i, sl] = xj[i, sl] * tv
                        plsc.addupdate_scatter(
                            den_t, [dvec * SUB + sb], tv, mask=lmask)
                return 0
            lax.fori_loop(0, B // 2, edge, 0)

            pltpu.sync_copy(xj, acc_s.at[dst_s.at[pl.ds(j * B, B)]], add=True)

        def superchunk(su, _):
            sbase = s * EPT + su * SCH * B
            pltpu.sync_copy(
                gsrc_hbm.at[pl.ds(c * N_EDGES + sbase, SCH * B)], jidx_s)
            pltpu.sync_copy(
                gdst_hbm.at[pl.ds(c * N_EDGES + sbase, SCH * B)], iidx_s)
            pltpu.sync_copy(dst_hbm.at[pl.ds(sbase, SCH * B)], dst_s)
            issue(jnp.int32(0), 0)
            issue(jnp.int32(1), 1)

            def pairq(q, _):
                j0 = 2 * q
                wait(0)
                compute(sbase, j0, 0)
                issue(j0 + 2, 0)
                wait(1)
                compute(sbase, j0 + 1, 1)
                issue(j0 + 3, 1)
                return 0
            lax.fori_loop(0, SCH // 2, pairq, 0)
            wait(0)
            wait(1)
            return 0
        lax.fori_loop(0, NCH // SCH, superchunk, 0)

        plsc.subcore_barrier()

        pltpu.sync_copy(acc_s.at[pl.ds(s * RPT, RPT)],
                        acc_out.at[c, pl.ds(s * RPT, RPT)])
        pltpu.sync_copy(den_t, den_out.at[c, s])

    return gat_sc


def _norm_x(acc_ref, den_ref, bias_ref, SUB):
    cols = []
    for c in range(2):
        dpart = den_ref[c]
        dsum = jnp.sum(dpart, axis=0)
        if SUB == 2:
            cols.append(acc_ref[c][:, 0:64] / (dsum[:, 0:1] + 1e-16))
            cols.append(acc_ref[c][:, 64:128] / (dsum[:, 1:2] + 1e-16))
        else:
            cols.append(acc_ref[c] / (dsum[:, 0:1] + 1e-16))
    return jnp.maximum(jnp.concatenate(cols, axis=1) + bias_ref[...], 0.0)


def _combine_proj_body(acc_ref, den_ref, bias_ref, wl_ref, bl_ref,
                       wr_ref, br_ref, xl_ref, xr_ref, *, SUB):
    x = _norm_x(acc_ref, den_ref, bias_ref, SUB)
    xl = jnp.dot(x, wl_ref[...], preferred_element_type=jnp.float32) + bl_ref[...]
    xr = jnp.dot(x, wr_ref[...], preferred_element_type=jnp.float32) + br_ref[...]
    xl_ref[...] = _halves(xl)
    xr_ref[...] = _halves(xr)


def _combine_proj(acc, den, bias, wlt, bl, wrt, br, SUB):
    Bn = 1264
    grid = (N_PAD // Bn,)
    full = lambda shape: pl.BlockSpec(shape, lambda i: (0,) * len(shape))
    out_shape = (jax.ShapeDtypeStruct((2, N_PAD, 128), jnp.float32),
                 jax.ShapeDtypeStruct((2, N_PAD, 128), jnp.float32))
    return pl.pallas_call(
        functools.partial(_combine_proj_body, SUB=SUB),
        grid=grid,
        in_specs=[
            pl.BlockSpec((2, Bn, 128), lambda i: (0, i, 0)),
            pl.BlockSpec((2, 16, Bn, SUB), lambda i: (0, 0, i, 0)),
            full((1, 256)),
            full((256, 256)), full((1, 256)), full((256, 256)), full((1, 256)),
        ],
        out_specs=(pl.BlockSpec((2, Bn, 128), lambda i: (0, i, 0)),
                   pl.BlockSpec((2, Bn, 128), lambda i: (0, i, 0))),
        out_shape=out_shape,
        compiler_params=pltpu.CompilerParams(
            dimension_semantics=("parallel",)),
    )(acc, den, bias, wlt, bl, wrt, br)


def _combine_final_body(acc_ref, den_ref, bias_ref, xsum_ref, *, SUB, Bn):
    x = _norm_x(acc_ref, den_ref, bias_ref, SUB)
    ridx = (pl.program_id(0) * Bn
            + lax.broadcasted_iota(jnp.int32, x.shape, 0))
    x = jnp.where(ridx < N_NODES, x, 0.0)
    part = jnp.sum(x, axis=0, keepdims=True)
    @pl.when(pl.program_id(0) == 0)
    def _():
        xsum_ref[...] = jnp.zeros_like(xsum_ref)
    xsum_ref[...] += part


def _combine_final(acc, den, bias, SUB):
    Bn = 1264
    grid = (N_PAD // Bn,)
    return pl.pallas_call(
        functools.partial(_combine_final_body, SUB=SUB, Bn=Bn),
        grid=grid,
        in_specs=[
            pl.BlockSpec((2, Bn, 128), lambda i: (0, i, 0)),
            pl.BlockSpec((2, 16, Bn, SUB), lambda i: (0, 0, i, 0)),
            pl.BlockSpec((1, 256), lambda i: (0, 0)),
        ],
        out_specs=pl.BlockSpec((1, 256), lambda i: (0, 0)),
        out_shape=jax.ShapeDtypeStruct((1, 256), jnp.float32),
        compiler_params=pltpu.CompilerParams(
            dimension_semantics=("arbitrary",)),
    )(acc, den, bias)


def _dec1_body(xsum_ref, easum_ref, eps_ref,
               fp_ref, bfp_ref, fplg_ref, fplb_ref,
               mu_ref_w, bmu_ref, lv_ref_w, blv_ref,
               l1_ref, bl1_ref, l1g_ref, l1b_ref,
               l2_ref, bl2_ref, l2g_ref, l2b_ref,
               a1_ref, ba1_ref, c1_ref, bc1_ref, c2_ref, bc2_ref,
               e1_ref, be1_ref, n1_ref, bn1_ref, n2_ref, bn2_ref,
               p1_ref, bp1_ref, p2_ref, bp2_ref,
               h_ref, mu_ref, lv_ref, aact_ref, eact_ref,
               nn_ref, coords_ref, cell_ref):
    g = jnp.concatenate([xsum_ref[...] / N_NODES, easum_ref[...] / N_EDGES],
                        axis=1)
    g = _silu(_lnorm_rows(
        jnp.dot(g, fp_ref[...], preferred_element_type=jnp.float32) + bfp_ref[...],
        fplg_ref[...], fplb_ref[...]))
    mu = jnp.dot(g, mu_ref_w[...], preferred_element_type=jnp.float32) + bmu_ref[...]
    lv = jnp.dot(g, lv_ref_w[...], preferred_element_type=jnp.float32) + blv_ref[...]
    z = mu + eps_ref[...] * jnp.exp(0.5 * lv)
    h1 = _silu(_lnorm_rows(
        jnp.dot(z, l1_ref[...], preferred_element_type=jnp.float32) + bl1_ref[...],
        l1g_ref[...], l1b_ref[...]))
    h = _silu(_lnorm_rows(
        jnp.dot(h1, l2_ref[...], preferred_element_type=jnp.float32) + bl2_ref[...],
        l2g_ref[...], l2b_ref[...]))
    aact = _silu(jnp.dot(h, a1_ref[...], preferred_element_type=jnp.float32) + ba1_ref[...])
    cact = _silu(jnp.dot(h, c1_ref[...], preferred_element_type=jnp.float32) + bc1_ref[...])
    coords = jnp.dot(cact, c2_ref[...], preferred_element_type=jnp.float32) + bc2_ref[...]
    eact = _silu(jnp.dot(h, e1_ref[...], preferred_element_type=jnp.float32) + be1_ref[...])
    nact = _silu(jnp.dot(h, n1_ref[...], preferred_element_type=jnp.float32) + bn1_ref[...])
    nn = jnp.dot(nact, n2_ref[...], preferred_element_type=jnp.float32) + bn2_ref[...]
    pact = _silu(jnp.dot(h, p1_ref[...], preferred_element_type=jnp.float32) + bp1_ref[...])
    cp = jnp.dot(pact, p2_ref[...], preferred_element_type=jnp.float32) + bp2_ref[...]
    lane = lax.broadcasted_iota(jnp.int32, cp.shape, 1)
    sp = jnp.log1p(jnp.exp(-jnp.abs(cp))) + jnp.maximum(cp, 0.0)
    sg = 30.0 + 120.0 * jax.nn.sigmoid(cp)
    cell = jnp.where(lane < 3, sp, sg)
    h_ref[...] = h
    mu_ref[...] = mu
    lv_ref[...] = lv
    aact_ref[...] = aact
    eact_ref[...] = eact
    nn_ref[...] = nn
    coords_ref[...] = coords
    cell_ref[...] = cell


def _decoder_small(xsum, easum, eps, w):
    full = lambda shape: pl.BlockSpec(shape, lambda: (0,) * len(shape))
    out_shape = (jax.ShapeDtypeStruct((1, 512), jnp.float32),
                 jax.ShapeDtypeStruct((1, 128), jnp.float32),
                 jax.ShapeDtypeStruct((1, 128), jnp.float32),
                 jax.ShapeDtypeStruct((1, 256), jnp.float32),
                 jax.ShapeDtypeStruct((1, 512), jnp.float32),
                 jax.ShapeDtypeStruct((1, 100), jnp.float32),
                 jax.ShapeDtypeStruct((1, 300), jnp.float32),
                 jax.ShapeDtypeStruct((1, 8), jnp.float32))
    args = [xsum, easum, eps] + w
    return pl.pallas_call(
        _dec1_body,
        in_specs=[full(a.shape) for a in args],
        out_specs=tuple(full(s.shape) for s in out_shape),
        out_shape=out_shape,
    )(*args)


def _dec2_body(aact_ref, eact_ref, wa_ref, ba_ref, we_ref, be_ref,
               atom_ref, edge_ref):
    atom_ref[...] = jnp.dot(aact_ref[...], wa_ref[...],
                            preferred_element_type=jnp.float32) + ba_ref[...]
    edge_ref[...] = jnp.dot(eact_ref[...], we_ref[...],
                            preferred_element_type=jnp.float32) + be_ref[...]


def _decoder_wide(aact, eact, wat, ba, wet, be):
    T = 1280
    grid = (10240 // T,)
    full = lambda shape: pl.BlockSpec(shape, lambda i: (0,) * len(shape))
    out_shape = (jax.ShapeDtypeStruct((1, 10240), jnp.float32),
                 jax.ShapeDtypeStruct((1, 10240), jnp.float32))
    return pl.pallas_call(
        _dec2_body,
        grid=grid,
        in_specs=[
            full((1, 256)), full((1, 512)),
            pl.BlockSpec((256, T), lambda i: (0, i)),
            pl.BlockSpec((1, T), lambda i: (0, i)),
            pl.BlockSpec((512, T), lambda i: (0, i)),
            pl.BlockSpec((1, T), lambda i: (0, i)),
        ],
        out_specs=(pl.BlockSpec((1, T), lambda i: (0, i)),
                   pl.BlockSpec((1, T), lambda i: (0, i))),
        out_shape=out_shape,
        compiler_params=pltpu.CompilerParams(
            dimension_semantics=("parallel",)),
    )(aact, eact, wat, ba, wet, be)


def _row(v):
    return v.reshape(1, -1)


def kernel(node_features, edge_index, edge_features, params):
    p = params
    src = edge_index[0]
    dst = edge_index[1]

    emb_p = jnp.zeros((128, 64), jnp.float32).at[:100].set(p["ne_emb"])
    wce1_p = jnp.zeros((103, 256), jnp.float32).at[100:103].set(p["ne_ce1"]["W"].T)
    wde1_p = jnp.zeros((4, 128), jnp.float32).at[0:1].set(p["ee_de1"]["W"].T)
    wpe1_p = jnp.zeros((4, 128), jnp.float32).at[1:4].set(p["ee_pe1"]["W"].T)

    c1, c2, c3 = p["c1"], p["c2"], p["c3"]

    nf_pad = jnp.zeros((N_PAD, 103), jnp.float32).at[:N_NODES].set(node_features)
    xl1, xr1 = _node_encoder(
        nf_pad, emb_p, wce1_p, _row(p["ne_ce1"]["b"]),
        p["ne_ce2"]["W"].T, _row(p["ne_ce2"]["b"]),
        p["ne_nc"]["W"].T, _row(p["ne_nc"]["b"]),
        _row(p["ne_ln"]["g"]), _row(p["ne_ln"]["b"]),
        c1["Wl"]["W"].T, _row(c1["Wl"]["b"]),
        c1["Wr"]["W"].T, _row(c1["Wr"]["b"]))

    e1, e2, e3, ea_sum = _edge_encoder(
        edge_features, wde1_p, _row(p["ee_de1"]["b"]),
        p["ee_de2"]["W"].T, _row(p["ee_de2"]["b"]),
        wpe1_p, _row(p["ee_pe1"]["b"]),
        p["ee_pe2"]["W"].T, _row(p["ee_pe2"]["b"]),
        p["ee_ec"]["W"].T, _row(p["ee_ec"]["b"]),
        _row(p["ee_ln"]["g"]), _row(p["ee_ln"]["b"]),
        c1["We"]["W"].T, c2["We"]["W"].T, c3["We"]["W"].T)

    gat12 = _make_gat_sc(2)
    gat3 = _make_gat_sc(1)

    gsrc = jnp.concatenate([src, src + N_PAD])
    gdst = jnp.concatenate([dst, dst + N_PAD])

    def run_gat(gat, xl, xr, el, att, SUB):
        acc, den = gat(xl.reshape(2 * N_PAD, 128), xr.reshape(2 * N_PAD, 128),
                       el.reshape(2 * N_EDGES, 128), gsrc, gdst, dst,
                       att.reshape(-1))
        return acc, den.reshape(2, 16, N_PAD, SUB)

    acc1, den1 = run_gat(gat12, xl1, xr1, e1, c1["att"], 2)
    xl2, xr2 = _combine_proj(acc1, den1, _row(c1["bias"]),
                             c2["Wl"]["W"].T, _row(c2["Wl"]["b"]),
                             c2["Wr"]["W"].T, _row(c2["Wr"]["b"]), SUB=2)

    acc2, den2 = run_gat(gat12, xl2, xr2, e2, c2["att"], 2)
    xl3, xr3 = _combine_proj(acc2, den2, _row(c2["bias"]),
                             c3["Wl"]["W"].T, _row(c3["Wl"]["b"]),
                             c3["Wr"]["W"].T, _row(c3["Wr"]["b"]), SUB=2)

    acc3, den3 = run_gat(gat3, xl3, xr3, e3, c3["att"], 1)
    xsum = _combine_final(acc3, den3, _row(c3["bias"]), SUB=1)

    eps = jax.random.normal(jax.random.key(42), (1, LAT), dtype=jnp.float32)

    dec_w = [
        p["fp"]["W"].T, _row(p["fp"]["b"]),
        _row(p["fp_ln"]["g"]), _row(p["fp_ln"]["b"]),
        p["mu"]["W"].T, _row(p["mu"]["b"]),
        p["lv"]["W"].T, _row(p["lv"]["b"]),
        p["d_l1"]["W"].T, _row(p["d_l1"]["b"]),
        _row(p["d_ln1"]["g"]), _row(p["d_ln1"]["b"]),
        p["d_l2"]["W"].T, _row(p["d_l2"]["b"]),
        _row(p["d_ln2"]["g"]), _row(p["d_ln2"]["b"]),
        p["d_a1"]["W"].T, _row(p["d_a1"]["b"]),
        p["d_c1"]["W"].T, _row(p["d_c1"]["b"]),
        p["d_c2"]["W"].T, _row(p["d_c2"]["b"]),
        p["d_e1"]["W"].T, _row(p["d_e1"]["b"]),
        p["d_n1"]["W"].T, _row(p["d_n1"]["b"]),
        p["d_n2"]["W"].T, _row(p["d_n2"]["b"]),
        p["d_p1"]["W"].T, _row(p["d_p1"]["b"]),
        jnp.zeros((256, 8), jnp.float32).at[:, :6].set(p["d_p2"]["W"].T),
        jnp.zeros((1, 8), jnp.float32).at[:, :6].set(_row(p["d_p2"]["b"])),
    ]
    (h, mu, log_var, aact, eact, nn_logits, coords, cell8) = _decoder_small(
        xsum, ea_sum, eps, dec_w)

    wa_p = jnp.zeros((256, 10240), jnp.float32).at[:, :10000].set(p["d_a2"]["W"].T)
    we_p = jnp.zeros((512, 10240), jnp.float32).at[:, :10000].set(p["d_e2"]["W"].T)
    ba_p = jnp.zeros((1, 10240), jnp.float32).at[:, :10000].set(_row(p["d_a2"]["b"]))
    be_p = jnp.zeros((1, 10240), jnp.float32).at[:, :10000].set(_row(p["d_e2"]["b"]))
    atom_p, edge_p = _decoder_wide(aact, eact, wa_p, ba_p, we_p, be_p)
    atom = atom_p[:, :10000]
    edge_flat = edge_p[:, :10000]

    node_out = jnp.concatenate(
        [atom.reshape(1, MAXN, 100), coords.reshape(1, MAXN, 3)], axis=-1)
    edge_logits = edge_flat.reshape(1, MAXN, MAXN)
    cell = cell8[:, :6]
    return (node_out, edge_logits, nn_logits, cell, h, mu, log_var)


# speedup vs baseline: 17.5630x; 1.0085x over previous
import functools

import jax
import jax.numpy as jnp
from jax import lax
from jax.experimental import pallas as pl
from jax.experimental.pallas import tpu as pltpu
from jax.experimental.pallas import tpu_sc as plsc

N_NODES = 10000
N_EDGES = 160000
HID = 256
LAT = 128
MAXN = 100
N_PAD = 10112

_LN_EPS = 1e-5


def _silu(x):
    return x * jax.nn.sigmoid(x)


def _lnorm_rows(x, g, b):
    m = jnp.mean(x, axis=-1, keepdims=True)
    v = jnp.mean((x - m) * (x - m), axis=-1, keepdims=True)
    return (x - m) * jax.lax.rsqrt(v + _LN_EPS) * g + b


def _halves(x):
    return jnp.concatenate([x[:, 0:128][None], x[:, 128:256][None]], axis=0)


def _node_enc_body(nf_ref, emb_ref, wce1_ref, bce1_ref, wce2_ref, bce2_ref,
                   wnc_ref, bnc_ref, lng_ref, lnb_ref,
                   wl_ref, bl_ref, wr_ref, br_ref,
                   xl_ref, xr_ref):
    nf = nf_ref[...]
    col = lax.broadcasted_iota(jnp.int32, nf.shape, 1)
    val = jnp.where(col < 100, nf, -1e30)
    rowmax = jnp.max(val, axis=1, keepdims=True)
    idx = jnp.min(jnp.where(val == rowmax, col, 10 ** 9), axis=1)
    onehot = (lax.broadcasted_iota(jnp.int32, (nf.shape[0], 128), 1)
              == idx[:, None]).astype(jnp.float32)
    ef = jnp.dot(onehot, emb_ref[...], preferred_element_type=jnp.float32)
    cf_pre = jnp.dot(nf, wce1_ref[...], preferred_element_type=jnp.float32) + bce1_ref[...]
    cf = jnp.dot(_silu(cf_pre), wce2_ref[...], preferred_element_type=jnp.float32) + bce2_ref[...]
    pre = jnp.dot(jnp.concatenate([ef, cf], axis=1), wnc_ref[...],
                  preferred_element_type=jnp.float32) + bnc_ref[...]
    x = _silu(_lnorm_rows(pre, lng_ref[...], lnb_ref[...]))
    xl = jnp.dot(x, wl_ref[...], preferred_element_type=jnp.float32) + bl_ref[...]
    xr = jnp.dot(x, wr_ref[...], preferred_element_type=jnp.float32) + br_ref[...]
    xl_ref[...] = _halves(xl)
    xr_ref[...] = _halves(xr)


def _node_encoder(nf, emb_p, wce1_p, bce1, wce2t, bce2, wnct, bnc, lng, lnb,
                  wlt, bl, wrt, br):
    Bn = 1264
    grid = (N_PAD // Bn,)
    full = lambda shape: pl.BlockSpec(shape, lambda i: (0,) * len(shape))
    out_shape = (jax.ShapeDtypeStruct((2, N_PAD, 128), jnp.float32),
                 jax.ShapeDtypeStruct((2, N_PAD, 128), jnp.float32))
    return pl.pallas_call(
        _node_enc_body,
        grid=grid,
        in_specs=[
            pl.BlockSpec((Bn, 103), lambda i: (i, 0)),
            full((128, 64)), full((103, 256)), full((1, 256)),
            full((256, 64)), full((1, 64)),
            full((128, 256)), full((1, 256)), full((1, 256)), full((1, 256)),
            full((256, 256)), full((1, 256)), full((256, 256)), full((1, 256)),
        ],
        out_specs=(pl.BlockSpec((2, Bn, 128), lambda i: (0, i, 0)),
                   pl.BlockSpec((2, Bn, 128), lambda i: (0, i, 0))),
        out_shape=out_shape,
        compiler_params=pltpu.CompilerParams(
            dimension_semantics=("parallel",)),
    )(nf, emb_p, wce1_p, bce1, wce2t, bce2, wnct, bnc, lng, lnb,
      wlt, bl, wrt, br)


def _edge_enc_core(ef_ref, wde1_ref, bde1_ref, wde2_ref, bde2_ref,
                   wpe1_ref, bpe1_ref, wpe2_ref, bpe2_ref,
                   wec_ref, bec_ref, lng_ref, lnb_ref):
    ef = ef_ref[...]
    de_pre = jnp.dot(ef, wde1_ref[...], preferred_element_type=jnp.float32) + bde1_ref[...]
    de = jnp.dot(_silu(de_pre), wde2_ref[...], preferred_element_type=jnp.float32) + bde2_ref[...]
    pe_pre = jnp.dot(ef, wpe1_ref[...], preferred_element_type=jnp.float32) + bpe1_ref[...]
    pe = jnp.dot(_silu(pe_pre), wpe2_ref[...], preferred_element_type=jnp.float32) + bpe2_ref[...]
    pre = jnp.dot(jnp.concatenate([de, pe], axis=1), wec_ref[...],
                  preferred_element_type=jnp.float32) + bec_ref[...]
    return _silu(_lnorm_rows(pre, lng_ref[...], lnb_ref[...]))


def _edge_enc1_body(ef_ref, wde1_ref, bde1_ref, wde2_ref, bde2_ref,
                    wpe1_ref, bpe1_ref, wpe2_ref, bpe2_ref,
                    wec_ref, bec_ref, lng_ref, lnb_ref,
                    we1_ref, e1_ref, easum_ref):
    ea = _edge_enc_core(ef_ref, wde1_ref, bde1_ref, wde2_ref, bde2_ref,
                        wpe1_ref, bpe1_ref, wpe2_ref, bpe2_ref,
                        wec_ref, bec_ref, lng_ref, lnb_ref)
    e1_ref[...] = _halves(
        jnp.dot(ea, we1_ref[...], preferred_element_type=jnp.float32))
    part = jnp.sum(ea, axis=0, keepdims=True)
    @pl.when(pl.program_id(0) == 0)
    def _():
        easum_ref[...] = jnp.zeros_like(easum_ref)
    easum_ref[...] += part


def _edge_enc23_body(ef_ref, wde1_ref, bde1_ref, wde2_ref, bde2_ref,
                     wpe1_ref, bpe1_ref, wpe2_ref, bpe2_ref,
                     wec_ref, bec_ref, lng_ref, lnb_ref,
                     we2_ref, we3_ref, e2_ref, e3_ref):
    ea = _edge_enc_core(ef_ref, wde1_ref, bde1_ref, wde2_ref, bde2_ref,
                        wpe1_ref, bpe1_ref, wpe2_ref, bpe2_ref,
                        wec_ref, bec_ref, lng_ref, lnb_ref)
    e2_ref[...] = _halves(
        jnp.dot(ea, we2_ref[...], preferred_element_type=jnp.float32))
    e3_ref[...] = _halves(
        jnp.dot(ea, we3_ref[...], preferred_element_type=jnp.float32))


_EE_SPECS = None


def _edge_encoder1(ef, enc_w, we1t):
    Be = 2000
    grid = (N_EDGES // Be,)
    full = lambda shape: pl.BlockSpec(shape, lambda i: (0,) * len(shape))
    espec = pl.BlockSpec((2, Be, 128), lambda i: (0, i, 0))
    return pl.pallas_call(
        _edge_enc1_body,
        grid=grid,
        in_specs=[
            pl.BlockSpec((Be, 4), lambda i: (i, 0)),
            full((4, 128)), full((1, 128)), full((128, 32)), full((1, 32)),
            full((4, 128)), full((1, 128)), full((128, 32)), full((1, 32)),
            full((64, 256)), full((1, 256)), full((1, 256)), full((1, 256)),
            full((256, 256)),
        ],
        out_specs=(espec, pl.BlockSpec((1, 256), lambda i: (0, 0))),
        out_shape=(jax.ShapeDtypeStruct((2, N_EDGES, 128), jnp.float32),
                   jax.ShapeDtypeStruct((1, 256), jnp.float32)),
        compiler_params=pltpu.CompilerParams(
            dimension_semantics=("arbitrary",)),
    )(ef, *enc_w, we1t)


def _edge_encoder23(ef, enc_w, we2t, we3t):
    Be = 2000
    grid = (N_EDGES // Be,)
    full = lambda shape: pl.BlockSpec(shape, lambda i: (0,) * len(shape))
    espec = pl.BlockSpec((2, Be, 128), lambda i: (0, i, 0))
    return pl.pallas_call(
        _edge_enc23_body,
        grid=grid,
        in_specs=[
            pl.BlockSpec((Be, 4), lambda i: (i, 0)),
            full((4, 128)), full((1, 128)), full((128, 32)), full((1, 32)),
            full((4, 128)), full((1, 128)), full((128, 32)), full((1, 32)),
            full((64, 256)), full((1, 256)), full((1, 256)), full((1, 256)),
            full((256, 256)), full((256, 256)),
        ],
        out_specs=(espec, espec),
        out_shape=(jax.ShapeDtypeStruct((2, N_EDGES, 128), jnp.float32),
                   jax.ShapeDtypeStruct((2, N_EDGES, 128), jnp.float32)),
        compiler_params=pltpu.CompilerParams(
            dimension_semantics=("parallel",)),
    )(ef, *enc_w, we2t, we3t)


def _make_gat_sc(sub):
    SUB = sub
    SV = (128 // SUB) // 16
    EPT = N_EDGES // 16
    B = 40
    NCH = EPT // B
    SCH = 10
    ZR = 8
    RPT = N_PAD // 16

    mesh = plsc.VectorSubcoreMesh(core_axis_name="c", subcore_axis_name="s")

    @functools.partial(
        pl.kernel,
        out_type=(jax.ShapeDtypeStruct((2, N_PAD, 128), jnp.float32),
                  jax.ShapeDtypeStruct((2, 16, SUB * N_PAD), jnp.float32)),
        mesh=mesh,
        compiler_params=pltpu.CompilerParams(needs_layout_passes=False),
        scratch_types=[
            pltpu.VMEM((SCH * B,), jnp.int32),
            pltpu.VMEM((SCH * B,), jnp.int32),
            pltpu.VMEM((SCH * B,), jnp.int32),
            pltpu.VMEM((B, 128), jnp.float32),
            pltpu.VMEM((B, 128), jnp.float32),
            pltpu.VMEM((B, 128), jnp.float32),
            pltpu.VMEM((B, 128), jnp.float32),
            pltpu.VMEM((B, 128), jnp.float32),
            pltpu.VMEM((SUB * N_PAD,), jnp.float32),
            pltpu.VMEM((128,), jnp.float32),
            pltpu.VMEM_SHARED((N_PAD, 128), jnp.float32),
            pltpu.SemaphoreType.DMA,
            pltpu.SemaphoreType.DMA,
        ],
    )
    def gat_sc(xl_hbm, xr_hbm, el_hbm, gsrc_hbm, gdst_hbm, dst_hbm, att_hbm,
               acc_out, den_out,
               jidx_s, iidx_s, dst_s,
               xjv0, xiv0, xjv1, xiv1, elv, den_t, attv,
               acc_s, sem0, sem1):
        c = lax.axis_index("c")
        s = lax.axis_index("s")
        zrow16 = jnp.zeros((16,), jnp.float32)
        lanes = lax.iota(jnp.int32, 16)
        bufs = ((xjv0, xiv0, sem0), (xjv1, xiv1, sem1))

        def zfill(i, _):
            for k in range(8):
                xjv0[i, pl.ds(k * 16, 16)] = zrow16
            return 0
        lax.fori_loop(0, ZR, zfill, 0)

        def zden(i, _):
            den_t[pl.ds(i * 16, 16)] = zrow16
            return 0
        lax.fori_loop(0, SUB * N_PAD // 16, zden, 0)

        r0 = s * RPT

        def zcopy(j, _):
            pltpu.sync_copy(xjv0.at[pl.ds(0, ZR)],
                            acc_s.at[pl.ds(r0 + j * ZR, ZR)])
            return 0
        lax.fori_loop(0, RPT // ZR, zcopy, 0)

        plsc.subcore_barrier()

        pltpu.sync_copy(att_hbm.at[pl.ds(c * 128, 128)], attv)
        att_regs = [attv[pl.ds(k * 16, 16)] for k in range(8)]

        def issue(j, b):
            xj, xi, sem = bufs[b]
            jc = jnp.minimum(j, SCH - 1)
            pltpu.async_copy(xl_hbm.at[jidx_s.at[pl.ds(jc * B, B)]], xj, sem)
            pltpu.async_copy(xr_hbm.at[iidx_s.at[pl.ds(jc * B, B)]], xi, sem)

        def wait(b):
            xj, xi, sem = bufs[b]
            pltpu.make_async_copy(xl_hbm.at[jidx_s.at[pl.ds(0, B)]], xj,
                                  sem).wait()
            pltpu.make_async_copy(xr_hbm.at[iidx_s.at[pl.ds(0, B)]], xi,
                                  sem).wait()

        def compute(sbase, j, b):
            xj, xi, sem = bufs[b]
            pltpu.sync_copy(
                el_hbm.at[pl.ds(c * N_EDGES + sbase + j * B, B)], elv)

            def edge(i2, _):
                accs = []
                for u in range(2):
                    i = i2 * 2 + u
                    for sb in range(SUB):
                        a = jnp.zeros((16,), jnp.float32)
                        for k in range(SV):
                            kk = sb * SV + k
                            sl = pl.ds(kk * 16, 16)
                            svv = xj[i, sl] + xi[i, sl] + elv[i, sl]
                            m = jnp.where(svv > 0, svv, 0.2 * svv)
                            a = a + m * att_regs[kk]
                        accs.append(a)
                tvs = [jnp.exp(jnp.broadcast_to(jnp.sum(a), (16,)))
                       for a in accs]
                for u in range(2):
                    i = i2 * 2 + u
                    g16 = (i // 16) * 16
                    dvec = dst_s[pl.ds(j * B + g16, 16)]
                    lmask = lanes == (i - g16)
                    for sb in range(SUB):
                        tv = tvs[u * SUB + sb]
                        for k in range(SV):
                            kk = sb * SV + k
                            sl = pl.ds(kk * 16, 16)
                            xj[i, sl] = xj[i, sl] * tv
                        plsc.addupdate_scatter(
                            den_t, [dvec * SUB + sb], tv, mask=lmask)
                return 0
            lax.fori_loop(0, B // 2, edge, 0)

            pltpu.sync_copy(xj, acc_s.at[dst_s.at[pl.ds(j * B, B)]], add=True)

        def superchunk(su, _):
            sbase = s * EPT + su * SCH * B
            pltpu.sync_copy(
                gsrc_hbm.at[pl.ds(c * N_EDGES + sbase, SCH * B)], jidx_s)
            pltpu.sync_copy(
                gdst_hbm.at[pl.ds(c * N_EDGES + sbase, SCH * B)], iidx_s)
            pltpu.sync_copy(dst_hbm.at[pl.ds(sbase, SCH * B)], dst_s)
            issue(jnp.int32(0), 0)
            issue(jnp.int32(1), 1)

            def pairq(q, _):
                j0 = 2 * q
                wait(0)
                compute(sbase, j0, 0)
                issue(j0 + 2, 0)
                wait(1)
                compute(sbase, j0 + 1, 1)
                issue(j0 + 3, 1)
                return 0
            lax.fori_loop(0, SCH // 2, pairq, 0)
            wait(0)
            wait(1)
            return 0
        lax.fori_loop(0, NCH // SCH, superchunk, 0)

        plsc.subcore_barrier()

        pltpu.sync_copy(acc_s.at[pl.ds(s * RPT, RPT)],
                        acc_out.at[c, pl.ds(s * RPT, RPT)])
        pltpu.sync_copy(den_t, den_out.at[c, s])

    return gat_sc


def _norm_x(acc_ref, den_ref, bias_ref, SUB):
    cols = []
    for c in range(2):
        dpart = den_ref[c]
        dsum = jnp.sum(dpart, axis=0)
        if SUB == 2:
            cols.append(acc_ref[c][:, 0:64] / (dsum[:, 0:1] + 1e-16))
            cols.append(acc_ref[c][:, 64:128] / (dsum[:, 1:2] + 1e-16))
        else:
            cols.append(acc_ref[c] / (dsum[:, 0:1] + 1e-16))
    return jnp.maximum(jnp.concatenate(cols, axis=1) + bias_ref[...], 0.0)


def _combine_proj_body(acc_ref, den_ref, bias_ref, wl_ref, bl_ref,
                       wr_ref, br_ref, xl_ref, xr_ref, *, SUB):
    x = _norm_x(acc_ref, den_ref, bias_ref, SUB)
    xl = jnp.dot(x, wl_ref[...], preferred_element_type=jnp.float32) + bl_ref[...]
    xr = jnp.dot(x, wr_ref[...], preferred_element_type=jnp.float32) + br_ref[...]
    xl_ref[...] = _halves(xl)
    xr_ref[...] = _halves(xr)


def _combine_proj(acc, den, bias, wlt, bl, wrt, br, SUB):
    Bn = 1264
    grid = (N_PAD // Bn,)
    full = lambda shape: pl.BlockSpec(shape, lambda i: (0,) * len(shape))
    out_shape = (jax.ShapeDtypeStruct((2, N_PAD, 128), jnp.float32),
                 jax.ShapeDtypeStruct((2, N_PAD, 128), jnp.float32))
    return pl.pallas_call(
        functools.partial(_combine_proj_body, SUB=SUB),
        grid=grid,
        in_specs=[
            pl.BlockSpec((2, Bn, 128), lambda i: (0, i, 0)),
            pl.BlockSpec((2, 16, Bn, SUB), lambda i: (0, 0, i, 0)),
            full((1, 256)),
            full((256, 256)), full((1, 256)), full((256, 256)), full((1, 256)),
        ],
        out_specs=(pl.BlockSpec((2, Bn, 128), lambda i: (0, i, 0)),
                   pl.BlockSpec((2, Bn, 128), lambda i: (0, i, 0))),
        out_shape=out_shape,
        compiler_params=pltpu.CompilerParams(
            dimension_semantics=("parallel",)),
    )(acc, den, bias, wlt, bl, wrt, br)


def _combine_final_body(acc_ref, den_ref, bias_ref, xsum_ref, *, SUB, Bn):
    x = _norm_x(acc_ref, den_ref, bias_ref, SUB)
    ridx = (pl.program_id(0) * Bn
            + lax.broadcasted_iota(jnp.int32, x.shape, 0))
    x = jnp.where(ridx < N_NODES, x, 0.0)
    part = jnp.sum(x, axis=0, keepdims=True)
    @pl.when(pl.program_id(0) == 0)
    def _():
        xsum_ref[...] = jnp.zeros_like(xsum_ref)
    xsum_ref[...] += part


def _combine_final(acc, den, bias, SUB):
    Bn = 1264
    grid = (N_PAD // Bn,)
    return pl.pallas_call(
        functools.partial(_combine_final_body, SUB=SUB, Bn=Bn),
        grid=grid,
        in_specs=[
            pl.BlockSpec((2, Bn, 128), lambda i: (0, i, 0)),
            pl.BlockSpec((2, 16, Bn, SUB), lambda i: (0, 0, i, 0)),
            pl.BlockSpec((1, 256), lambda i: (0, 0)),
        ],
        out_specs=pl.BlockSpec((1, 256), lambda i: (0, 0)),
        out_shape=jax.ShapeDtypeStruct((1, 256), jnp.float32),
        compiler_params=pltpu.CompilerParams(
            dimension_semantics=("arbitrary",)),
    )(acc, den, bias)


def _dec1_body(xsum_ref, easum_ref, eps_ref,
               fp_ref, bfp_ref, fplg_ref, fplb_ref,
               mu_ref_w, bmu_ref, lv_ref_w, blv_ref,
               l1_ref, bl1_ref, l1g_ref, l1b_ref,
               l2_ref, bl2_ref, l2g_ref, l2b_ref,
               a1_ref, ba1_ref, c1_ref, bc1_ref, c2_ref, bc2_ref,
               e1_ref, be1_ref, n1_ref, bn1_ref, n2_ref, bn2_ref,
               p1_ref, bp1_ref, p2_ref, bp2_ref,
               h_ref, mu_ref, lv_ref, aact_ref, eact_ref,
               nn_ref, coords_ref, cell_ref):
    g = jnp.concatenate([xsum_ref[...] / N_NODES, easum_ref[...] / N_EDGES],
                        axis=1)
    g = _silu(_lnorm_rows(
        jnp.dot(g, fp_ref[...], preferred_element_type=jnp.float32) + bfp_ref[...],
        fplg_ref[...], fplb_ref[...]))
    mu = jnp.dot(g, mu_ref_w[...], preferred_element_type=jnp.float32) + bmu_ref[...]
    lv = jnp.dot(g, lv_ref_w[...], preferred_element_type=jnp.float32) + blv_ref[...]
    z = mu + eps_ref[...] * jnp.exp(0.5 * lv)
    h1 = _silu(_lnorm_rows(
        jnp.dot(z, l1_ref[...], preferred_element_type=jnp.float32) + bl1_ref[...],
        l1g_ref[...], l1b_ref[...]))
    h = _silu(_lnorm_rows(
        jnp.dot(h1, l2_ref[...], preferred_element_type=jnp.float32) + bl2_ref[...],
        l2g_ref[...], l2b_ref[...]))
    aact = _silu(jnp.dot(h, a1_ref[...], preferred_element_type=jnp.float32) + ba1_ref[...])
    cact = _silu(jnp.dot(h, c1_ref[...], preferred_element_type=jnp.float32) + bc1_ref[...])
    coords = jnp.dot(cact, c2_ref[...], preferred_element_type=jnp.float32) + bc2_ref[...]
    eact = _silu(jnp.dot(h, e1_ref[...], preferred_element_type=jnp.float32) + be1_ref[...])
    nact = _silu(jnp.dot(h, n1_ref[...], preferred_element_type=jnp.float32) + bn1_ref[...])
    nn = jnp.dot(nact, n2_ref[...], preferred_element_type=jnp.float32) + bn2_ref[...]
    pact = _silu(jnp.dot(h, p1_ref[...], preferred_element_type=jnp.float32) + bp1_ref[...])
    cp = jnp.dot(pact, p2_ref[...], preferred_element_type=jnp.float32) + bp2_ref[...]
    lane = lax.broadcasted_iota(jnp.int32, cp.shape, 1)
    sp = jnp.log1p(jnp.exp(-jnp.abs(cp))) + jnp.maximum(cp, 0.0)
    sg = 30.0 + 120.0 * jax.nn.sigmoid(cp)
    cell = jnp.where(lane < 3, sp, sg)
    h_ref[...] = h
    mu_ref[...] = mu
    lv_ref[...] = lv
    aact_ref[...] = aact
    eact_ref[...] = eact
    nn_ref[...] = nn
    coords_ref[...] = coords
    cell_ref[...] = cell


def _decoder_small(xsum, easum, eps, w):
    full = lambda shape: pl.BlockSpec(shape, lambda: (0,) * len(shape))
    out_shape = (jax.ShapeDtypeStruct((1, 512), jnp.float32),
                 jax.ShapeDtypeStruct((1, 128), jnp.float32),
                 jax.ShapeDtypeStruct((1, 128), jnp.float32),
                 jax.ShapeDtypeStruct((1, 256), jnp.float32),
                 jax.ShapeDtypeStruct((1, 512), jnp.float32),
                 jax.ShapeDtypeStruct((1, 100), jnp.float32),
                 jax.ShapeDtypeStruct((1, 300), jnp.float32),
                 jax.ShapeDtypeStruct((1, 8), jnp.float32))
    args = [xsum, easum, eps] + w
    return pl.pallas_call(
        _dec1_body,
        in_specs=[full(a.shape) for a in args],
        out_specs=tuple(full(s.shape) for s in out_shape),
        out_shape=out_shape,
    )(*args)


def _dec2_body(aact_ref, eact_ref, wa_ref, ba_ref, we_ref, be_ref,
               atom_ref, edge_ref):
    atom_ref[...] = jnp.dot(aact_ref[...], wa_ref[...],
                            preferred_element_type=jnp.float32) + ba_ref[...]
    edge_ref[...] = jnp.dot(eact_ref[...], we_ref[...],
                            preferred_element_type=jnp.float32) + be_ref[...]


def _decoder_wide(aact, eact, wat, ba, wet, be):
    T = 1280
    grid = (10240 // T,)
    full = lambda shape: pl.BlockSpec(shape, lambda i: (0,) * len(shape))
    out_shape = (jax.ShapeDtypeStruct((1, 10240), jnp.float32),
                 jax.ShapeDtypeStruct((1, 10240), jnp.float32))
    return pl.pallas_call(
        _dec2_body,
        grid=grid,
        in_specs=[
            full((1, 256)), full((1, 512)),
            pl.BlockSpec((256, T), lambda i: (0, i)),
            pl.BlockSpec((1, T), lambda i: (0, i)),
            pl.BlockSpec((512, T), lambda i: (0, i)),
            pl.BlockSpec((1, T), lambda i: (0, i)),
        ],
        out_specs=(pl.BlockSpec((1, T), lambda i: (0, i)),
                   pl.BlockSpec((1, T), lambda i: (0, i))),
        out_shape=out_shape,
        compiler_params=pltpu.CompilerParams(
            dimension_semantics=("parallel",)),
    )(aact, eact, wat, ba, wet, be)


def _row(v):
    return v.reshape(1, -1)


def kernel(node_features, edge_index, edge_features, params):
    p = params
    src = edge_index[0]
    dst = edge_index[1]

    emb_p = jnp.zeros((128, 64), jnp.float32).at[:100].set(p["ne_emb"])
    wce1_p = jnp.zeros((103, 256), jnp.float32).at[100:103].set(p["ne_ce1"]["W"].T)
    wde1_p = jnp.zeros((4, 128), jnp.float32).at[0:1].set(p["ee_de1"]["W"].T)
    wpe1_p = jnp.zeros((4, 128), jnp.float32).at[1:4].set(p["ee_pe1"]["W"].T)

    c1, c2, c3 = p["c1"], p["c2"], p["c3"]

    nf_pad = jnp.zeros((N_PAD, 103), jnp.float32).at[:N_NODES].set(node_features)
    xl1, xr1 = _node_encoder(
        nf_pad, emb_p, wce1_p, _row(p["ne_ce1"]["b"]),
        p["ne_ce2"]["W"].T, _row(p["ne_ce2"]["b"]),
        p["ne_nc"]["W"].T, _row(p["ne_nc"]["b"]),
        _row(p["ne_ln"]["g"]), _row(p["ne_ln"]["b"]),
        c1["Wl"]["W"].T, _row(c1["Wl"]["b"]),
        c1["Wr"]["W"].T, _row(c1["Wr"]["b"]))

    enc_w = [wde1_p, _row(p["ee_de1"]["b"]),
             p["ee_de2"]["W"].T, _row(p["ee_de2"]["b"]),
             wpe1_p, _row(p["ee_pe1"]["b"]),
             p["ee_pe2"]["W"].T, _row(p["ee_pe2"]["b"]),
             p["ee_ec"]["W"].T, _row(p["ee_ec"]["b"]),
             _row(p["ee_ln"]["g"]), _row(p["ee_ln"]["b"])]
    e1, ea_sum = _edge_encoder1(edge_features, enc_w, c1["We"]["W"].T)

    gat12 = _make_gat_sc(2)
    gat3 = _make_gat_sc(1)

    gsrc = jnp.concatenate([src, src + N_PAD])
    gdst = jnp.concatenate([dst, dst + N_PAD])

    def run_gat(gat, xl, xr, el, att, SUB):
        acc, den = gat(xl.reshape(2 * N_PAD, 128), xr.reshape(2 * N_PAD, 128),
                       el.reshape(2 * N_EDGES, 128), gsrc, gdst, dst,
                       att.reshape(-1))
        return acc, den.reshape(2, 16, N_PAD, SUB)

    acc1, den1 = run_gat(gat12, xl1, xr1, e1, c1["att"], 2)
    e2, e3 = _edge_encoder23(edge_features, enc_w,
                             c2["We"]["W"].T, c3["We"]["W"].T)
    xl2, xr2 = _combine_proj(acc1, den1, _row(c1["bias"]),
                             c2["Wl"]["W"].T, _row(c2["Wl"]["b"]),
                             c2["Wr"]["W"].T, _row(c2["Wr"]["b"]), SUB=2)

    acc2, den2 = run_gat(gat12, xl2, xr2, e2, c2["att"], 2)
    xl3, xr3 = _combine_proj(acc2, den2, _row(c2["bias"]),
                             c3["Wl"]["W"].T, _row(c3["Wl"]["b"]),
                             c3["Wr"]["W"].T, _row(c3["Wr"]["b"]), SUB=2)

    acc3, den3 = run_gat(gat3, xl3, xr3, e3, c3["att"], 1)
    xsum = _combine_final(acc3, den3, _row(c3["bias"]), SUB=1)

    eps = jax.random.normal(jax.random.key(42), (1, LAT), dtype=jnp.float32)

    dec_w = [
        p["fp"]["W"].T, _row(p["fp"]["b"]),
        _row(p["fp_ln"]["g"]), _row(p["fp_ln"]["b"]),
        p["mu"]["W"].T, _row(p["mu"]["b"]),
        p["lv"]["W"].T, _row(p["lv"]["b"]),
        p["d_l1"]["W"].T, _row(p["d_l1"]["b"]),
        _row(p["d_ln1"]["g"]), _row(p["d_ln1"]["b"]),
        p["d_l2"]["W"].T, _row(p["d_l2"]["b"]),
        _row(p["d_ln2"]["g"]), _row(p["d_ln2"]["b"]),
        p["d_a1"]["W"].T, _row(p["d_a1"]["b"]),
        p["d_c1"]["W"].T, _row(p["d_c1"]["b"]),
        p["d_c2"]["W"].T, _row(p["d_c2"]["b"]),
        p["d_e1"]["W"].T, _row(p["d_e1"]["b"]),
        p["d_n1"]["W"].T, _row(p["d_n1"]["b"]),
        p["d_n2"]["W"].T, _row(p["d_n2"]["b"]),
        p["d_p1"]["W"].T, _row(p["d_p1"]["b"]),
        jnp.zeros((256, 8), jnp.float32).at[:, :6].set(p["d_p2"]["W"].T),
        jnp.zeros((1, 8), jnp.float32).at[:, :6].set(_row(p["d_p2"]["b"])),
    ]
    (h, mu, log_var, aact, eact, nn_logits, coords, cell8) = _decoder_small(
        xsum, ea_sum, eps, dec_w)

    wa_p = jnp.zeros((256, 10240), jnp.float32).at[:, :10000].set(p["d_a2"]["W"].T)
    we_p = jnp.zeros((512, 10240), jnp.float32).at[:, :10000].set(p["d_e2"]["W"].T)
    ba_p = jnp.zeros((1, 10240), jnp.float32).at[:, :10000].set(_row(p["d_a2"]["b"]))
    be_p = jnp.zeros((1, 10240), jnp.float32).at[:, :10000].set(_row(p["d_e2"]["b"]))
    atom_p, edge_p = _decoder_wide(aact, eact, wa_p, ba_p, we_p, be_p)
    atom = atom_p[:, :10000]
    edge_flat = edge_p[:, :10000]

    node_out = jnp.concatenate(
        [atom.reshape(1, MAXN, 100), coords.reshape(1, MAXN, 3)], axis=-1)
    edge_logits = edge_flat.reshape(1, MAXN, MAXN)
    cell = cell8[:, :6]
    return (node_out, edge_logits, nn_logits, cell, h, mu, log_var)


# speedup vs baseline: 18.6809x; 1.0636x over previous
import functools

import jax
import jax.numpy as jnp
from jax import lax
from jax.experimental import pallas as pl
from jax.experimental.pallas import tpu as pltpu
from jax.experimental.pallas import tpu_sc as plsc

N_NODES = 10000
N_EDGES = 160000
HID = 256
LAT = 128
MAXN = 100
N_PAD = 10112

_LN_EPS = 1e-5


def _silu(x):
    return x * jax.nn.sigmoid(x)


def _lnorm_rows(x, g, b):
    m = jnp.mean(x, axis=-1, keepdims=True)
    v = jnp.mean((x - m) * (x - m), axis=-1, keepdims=True)
    return (x - m) * jax.lax.rsqrt(v + _LN_EPS) * g + b


def _halves(x):
    return jnp.concatenate([x[:, 0:128][None], x[:, 128:256][None]], axis=0)


def _node_enc_body(nf_ref, emb_ref, wce1_ref, bce1_ref, wce2_ref, bce2_ref,
                   wnc_ref, bnc_ref, lng_ref, lnb_ref,
                   wl_ref, bl_ref, wr_ref, br_ref,
                   xl_ref, xr_ref):
    nf = nf_ref[...]
    col = lax.broadcasted_iota(jnp.int32, nf.shape, 1)
    val = jnp.where(col < 100, nf, -1e30)
    rowmax = jnp.max(val, axis=1, keepdims=True)
    idx = jnp.min(jnp.where(val == rowmax, col, 10 ** 9), axis=1)
    onehot = (lax.broadcasted_iota(jnp.int32, (nf.shape[0], 128), 1)
              == idx[:, None]).astype(jnp.float32)
    ef = jnp.dot(onehot, emb_ref[...], preferred_element_type=jnp.float32)
    cf_pre = jnp.dot(nf, wce1_ref[...], preferred_element_type=jnp.float32) + bce1_ref[...]
    cf = jnp.dot(_silu(cf_pre), wce2_ref[...], preferred_element_type=jnp.float32) + bce2_ref[...]
    pre = jnp.dot(jnp.concatenate([ef, cf], axis=1), wnc_ref[...],
                  preferred_element_type=jnp.float32) + bnc_ref[...]
    x = _silu(_lnorm_rows(pre, lng_ref[...], lnb_ref[...]))
    xl = jnp.dot(x, wl_ref[...], preferred_element_type=jnp.float32) + bl_ref[...]
    xr = jnp.dot(x, wr_ref[...], preferred_element_type=jnp.float32) + br_ref[...]
    xl_ref[...] = _halves(xl)
    xr_ref[...] = _halves(xr)


def _node_encoder(nf, emb_p, wce1_p, bce1, wce2t, bce2, wnct, bnc, lng, lnb,
                  wlt, bl, wrt, br):
    Bn = 1264
    grid = (N_PAD // Bn,)
    full = lambda shape: pl.BlockSpec(shape, lambda i: (0,) * len(shape))
    out_shape = (jax.ShapeDtypeStruct((2, N_PAD, 128), jnp.float32),
                 jax.ShapeDtypeStruct((2, N_PAD, 128), jnp.float32))
    return pl.pallas_call(
        _node_enc_body,
        grid=grid,
        in_specs=[
            pl.BlockSpec((Bn, 103), lambda i: (i, 0)),
            full((128, 64)), full((103, 256)), full((1, 256)),
            full((256, 64)), full((1, 64)),
            full((128, 256)), full((1, 256)), full((1, 256)), full((1, 256)),
            full((256, 256)), full((1, 256)), full((256, 256)), full((1, 256)),
        ],
        out_specs=(pl.BlockSpec((2, Bn, 128), lambda i: (0, i, 0)),
                   pl.BlockSpec((2, Bn, 128), lambda i: (0, i, 0))),
        out_shape=out_shape,
        compiler_params=pltpu.CompilerParams(
            dimension_semantics=("parallel",)),
    )(nf, emb_p, wce1_p, bce1, wce2t, bce2, wnct, bnc, lng, lnb,
      wlt, bl, wrt, br)


def _edge_enc_core(ef_ref, wde1_ref, bde1_ref, wde2_ref, bde2_ref,
                   wpe1_ref, bpe1_ref, wpe2_ref, bpe2_ref,
                   wec_ref, bec_ref, lng_ref, lnb_ref):
    ef = ef_ref[...]
    de_pre = jnp.dot(ef, wde1_ref[...], preferred_element_type=jnp.float32) + bde1_ref[...]
    de = jnp.dot(_silu(de_pre), wde2_ref[...], preferred_element_type=jnp.float32) + bde2_ref[...]
    pe_pre = jnp.dot(ef, wpe1_ref[...], preferred_element_type=jnp.float32) + bpe1_ref[...]
    pe = jnp.dot(_silu(pe_pre), wpe2_ref[...], preferred_element_type=jnp.float32) + bpe2_ref[...]
    pre = jnp.dot(jnp.concatenate([de, pe], axis=1), wec_ref[...],
                  preferred_element_type=jnp.float32) + bec_ref[...]
    return _silu(_lnorm_rows(pre, lng_ref[...], lnb_ref[...]))


def _edge_enc1_body(ef_ref, wde1_ref, bde1_ref, wde2_ref, bde2_ref,
                    wpe1_ref, bpe1_ref, wpe2_ref, bpe2_ref,
                    wec_ref, bec_ref, lng_ref, lnb_ref,
                    we1_ref, e1_ref, easum_ref):
    ea = _edge_enc_core(ef_ref, wde1_ref, bde1_ref, wde2_ref, bde2_ref,
                        wpe1_ref, bpe1_ref, wpe2_ref, bpe2_ref,
                        wec_ref, bec_ref, lng_ref, lnb_ref)
    e1_ref[...] = _halves(
        jnp.dot(ea, we1_ref[...], preferred_element_type=jnp.float32))
    part = jnp.sum(ea, axis=0, keepdims=True)
    @pl.when(pl.program_id(0) == 0)
    def _():
        easum_ref[...] = jnp.zeros_like(easum_ref)
    easum_ref[...] += part


def _edge_enc23_body(ef_ref, wde1_ref, bde1_ref, wde2_ref, bde2_ref,
                     wpe1_ref, bpe1_ref, wpe2_ref, bpe2_ref,
                     wec_ref, bec_ref, lng_ref, lnb_ref,
                     we2_ref, we3_ref, e2_ref, e3_ref):
    ea = _edge_enc_core(ef_ref, wde1_ref, bde1_ref, wde2_ref, bde2_ref,
                        wpe1_ref, bpe1_ref, wpe2_ref, bpe2_ref,
                        wec_ref, bec_ref, lng_ref, lnb_ref)
    e2_ref[...] = _halves(
        jnp.dot(ea, we2_ref[...], preferred_element_type=jnp.float32))
    e3_ref[...] = _halves(
        jnp.dot(ea, we3_ref[...], preferred_element_type=jnp.float32))


_EE_SPECS = None


def _edge_encoder1(ef, enc_w, we1t):
    Be = 2000
    grid = (N_EDGES // Be,)
    full = lambda shape: pl.BlockSpec(shape, lambda i: (0,) * len(shape))
    espec = pl.BlockSpec((2, Be, 128), lambda i: (0, i, 0))
    return pl.pallas_call(
        _edge_enc1_body,
        grid=grid,
        in_specs=[
            pl.BlockSpec((Be, 4), lambda i: (i, 0)),
            full((4, 128)), full((1, 128)), full((128, 32)), full((1, 32)),
            full((4, 128)), full((1, 128)), full((128, 32)), full((1, 32)),
            full((64, 256)), full((1, 256)), full((1, 256)), full((1, 256)),
            full((256, 256)),
        ],
        out_specs=(espec, pl.BlockSpec((1, 256), lambda i: (0, 0))),
        out_shape=(jax.ShapeDtypeStruct((2, N_EDGES, 128), jnp.float32),
                   jax.ShapeDtypeStruct((1, 256), jnp.float32)),
        compiler_params=pltpu.CompilerParams(
            dimension_semantics=("arbitrary",)),
    )(ef, *enc_w, we1t)


def _edge_encoder23(ef, enc_w, we2t, we3t):
    Be = 2000
    grid = (N_EDGES // Be,)
    full = lambda shape: pl.BlockSpec(shape, lambda i: (0,) * len(shape))
    espec = pl.BlockSpec((2, Be, 128), lambda i: (0, i, 0))
    return pl.pallas_call(
        _edge_enc23_body,
        grid=grid,
        in_specs=[
            pl.BlockSpec((Be, 4), lambda i: (i, 0)),
            full((4, 128)), full((1, 128)), full((128, 32)), full((1, 32)),
            full((4, 128)), full((1, 128)), full((128, 32)), full((1, 32)),
            full((64, 256)), full((1, 256)), full((1, 256)), full((1, 256)),
            full((256, 256)), full((256, 256)),
        ],
        out_specs=(espec, espec),
        out_shape=(jax.ShapeDtypeStruct((2, N_EDGES, 128), jnp.float32),
                   jax.ShapeDtypeStruct((2, N_EDGES, 128), jnp.float32)),
        compiler_params=pltpu.CompilerParams(
            dimension_semantics=("parallel",)),
    )(ef, *enc_w, we2t, we3t)


def _make_gat_sc(sub):
    SUB = sub
    SV = (128 // SUB) // 16
    EPT = N_EDGES // 16
    B = 40
    NCH = EPT // B
    SCH = 10
    ZR = 8
    RPT = N_PAD // 16

    mesh = plsc.VectorSubcoreMesh(core_axis_name="c", subcore_axis_name="s")

    @functools.partial(
        pl.kernel,
        out_type=(jax.ShapeDtypeStruct((2, N_PAD, 128), jnp.float32),
                  jax.ShapeDtypeStruct((2, 16, SUB * N_PAD), jnp.float32)),
        mesh=mesh,
        compiler_params=pltpu.CompilerParams(needs_layout_passes=False),
        scratch_types=[
            pltpu.VMEM((SCH * B,), jnp.int32),
            pltpu.VMEM((SCH * B,), jnp.int32),
            pltpu.VMEM((SCH * B,), jnp.int32),
            pltpu.VMEM((B, 128), jnp.float32),
            pltpu.VMEM((B, 128), jnp.float32),
            pltpu.VMEM((B, 128), jnp.float32),
            pltpu.VMEM((B, 128), jnp.float32),
            pltpu.VMEM((B, 128), jnp.float32),
            pltpu.VMEM((SUB * N_PAD,), jnp.float32),
            pltpu.VMEM((128,), jnp.float32),
            pltpu.VMEM_SHARED((N_PAD, 128), jnp.float32),
            pltpu.SemaphoreType.DMA,
            pltpu.SemaphoreType.DMA,
        ],
    )
    def gat_sc(xl_hbm, xr_hbm, el_hbm, gsrc_hbm, gdst_hbm, dst_hbm, att_hbm,
               acc_out, den_out,
               jidx_s, iidx_s, dst_s,
               xjv0, xiv0, xjv1, xiv1, elv, den_t, attv,
               acc_s, sem0, sem1):
        c = lax.axis_index("c")
        s = lax.axis_index("s")
        zrow16 = jnp.zeros((16,), jnp.float32)
        lanes = lax.iota(jnp.int32, 16)
        bufs = ((xjv0, xiv0, sem0), (xjv1, xiv1, sem1))

        def zfill(i, _):
            for k in range(8):
                xjv0[i, pl.ds(k * 16, 16)] = zrow16
            return 0
        lax.fori_loop(0, ZR, zfill, 0)

        def zden(i, _):
            den_t[pl.ds(i * 16, 16)] = zrow16
            return 0
        lax.fori_loop(0, SUB * N_PAD // 16, zden, 0)

        r0 = s * RPT

        def zcopy(j, _):
            pltpu.sync_copy(xjv0.at[pl.ds(0, ZR)],
                            acc_s.at[pl.ds(r0 + j * ZR, ZR)])
            return 0
        lax.fori_loop(0, RPT // ZR, zcopy, 0)

        plsc.subcore_barrier()

        pltpu.sync_copy(att_hbm.at[pl.ds(c * 128, 128)], attv)
        att_regs = [attv[pl.ds(k * 16, 16)] for k in range(8)]

        def issue(j, b):
            xj, xi, sem = bufs[b]
            jc = jnp.minimum(j, SCH - 1)
            pltpu.async_copy(xl_hbm.at[jidx_s.at[pl.ds(jc * B, B)]], xj, sem)
            pltpu.async_copy(xr_hbm.at[iidx_s.at[pl.ds(jc * B, B)]], xi, sem)

        def wait(b):
            xj, xi, sem = bufs[b]
            pltpu.make_async_copy(xl_hbm.at[jidx_s.at[pl.ds(0, B)]], xj,
                                  sem).wait()
            pltpu.make_async_copy(xr_hbm.at[iidx_s.at[pl.ds(0, B)]], xi,
                                  sem).wait()

        def compute(sbase, j, b):
            xj, xi, sem = bufs[b]
            pltpu.sync_copy(
                el_hbm.at[pl.ds(c * N_EDGES + sbase + j * B, B)], elv)

            def edge(i2, _):
                accs = []
                vjs = [[None] * 8, [None] * 8]
                for u in range(2):
                    i = i2 * 2 + u
                    for sb in range(SUB):
                        a = jnp.zeros((16,), jnp.float32)
                        for k in range(SV):
                            kk = sb * SV + k
                            sl = pl.ds(kk * 16, 16)
                            vj = xj[i, sl]
                            vjs[u][kk] = vj
                            svv = vj + xi[i, sl] + elv[i, sl]
                            m = jnp.where(svv > 0, svv, 0.2 * svv)
                            a = a + m * att_regs[kk]
                        accs.append(a)
                tvs = [jnp.exp(jnp.broadcast_to(jnp.sum(a), (16,)))
                       for a in accs]
                for u in range(2):
                    i = i2 * 2 + u
                    g16 = (i // 16) * 16
                    dvec = dst_s[pl.ds(j * B + g16, 16)]
                    lmask = lanes == (i - g16)
                    for sb in range(SUB):
                        tv = tvs[u * SUB + sb]
                        for k in range(SV):
                            kk = sb * SV + k
                            sl = pl.ds(kk * 16, 16)
                            xj[i, sl] = vjs[u][kk] * tv
                        plsc.addupdate_scatter(
                            den_t, [dvec * SUB + sb], tv, mask=lmask)
                return 0
            lax.fori_loop(0, B // 2, edge, 0)

            pltpu.sync_copy(xj, acc_s.at[dst_s.at[pl.ds(j * B, B)]], add=True)

        def superchunk(su, _):
            sbase = s * EPT + su * SCH * B
            pltpu.sync_copy(
                gsrc_hbm.at[pl.ds(c * N_EDGES + sbase, SCH * B)], jidx_s)
            pltpu.sync_copy(
                gdst_hbm.at[pl.ds(c * N_EDGES + sbase, SCH * B)], iidx_s)
            pltpu.sync_copy(dst_hbm.at[pl.ds(sbase, SCH * B)], dst_s)
            issue(jnp.int32(0), 0)
            issue(jnp.int32(1), 1)

            def pairq(q, _):
                j0 = 2 * q
                wait(0)
                compute(sbase, j0, 0)
                issue(j0 + 2, 0)
                wait(1)
                compute(sbase, j0 + 1, 1)
                issue(j0 + 3, 1)
                return 0
            lax.fori_loop(0, SCH // 2, pairq, 0)
            wait(0)
            wait(1)
            return 0
        lax.fori_loop(0, NCH // SCH, superchunk, 0)

        plsc.subcore_barrier()

        pltpu.sync_copy(acc_s.at[pl.ds(s * RPT, RPT)],
                        acc_out.at[c, pl.ds(s * RPT, RPT)])
        pltpu.sync_copy(den_t, den_out.at[c, s])

    return gat_sc


def _norm_x(acc_ref, den_ref, bias_ref, SUB):
    cols = []
    for c in range(2):
        dpart = den_ref[c]
        dsum = jnp.sum(dpart, axis=0)
        if SUB == 2:
            cols.append(acc_ref[c][:, 0:64] / (dsum[:, 0:1] + 1e-16))
            cols.append(acc_ref[c][:, 64:128] / (dsum[:, 1:2] + 1e-16))
        else:
            cols.append(acc_ref[c] / (dsum[:, 0:1] + 1e-16))
    return jnp.maximum(jnp.concatenate(cols, axis=1) + bias_ref[...], 0.0)


def _combine_proj_body(acc_ref, den_ref, bias_ref, wl_ref, bl_ref,
                       wr_ref, br_ref, xl_ref, xr_ref, *, SUB):
    x = _norm_x(acc_ref, den_ref, bias_ref, SUB)
    xl = jnp.dot(x, wl_ref[...], preferred_element_type=jnp.float32) + bl_ref[...]
    xr = jnp.dot(x, wr_ref[...], preferred_element_type=jnp.float32) + br_ref[...]
    xl_ref[...] = _halves(xl)
    xr_ref[...] = _halves(xr)


def _combine_proj(acc, den, bias, wlt, bl, wrt, br, SUB):
    Bn = 1264
    grid = (N_PAD // Bn,)
    full = lambda shape: pl.BlockSpec(shape, lambda i: (0,) * len(shape))
    out_shape = (jax.ShapeDtypeStruct((2, N_PAD, 128), jnp.float32),
                 jax.ShapeDtypeStruct((2, N_PAD, 128), jnp.float32))
    return pl.pallas_call(
        functools.partial(_combine_proj_body, SUB=SUB),
        grid=grid,
        in_specs=[
            pl.BlockSpec((2, Bn, 128), lambda i: (0, i, 0)),
            pl.BlockSpec((2, 16, Bn, SUB), lambda i: (0, 0, i, 0)),
            full((1, 256)),
            full((256, 256)), full((1, 256)), full((256, 256)), full((1, 256)),
        ],
        out_specs=(pl.BlockSpec((2, Bn, 128), lambda i: (0, i, 0)),
                   pl.BlockSpec((2, Bn, 128), lambda i: (0, i, 0))),
        out_shape=out_shape,
        compiler_params=pltpu.CompilerParams(
            dimension_semantics=("parallel",)),
    )(acc, den, bias, wlt, bl, wrt, br)


def _combine_final_body(acc_ref, den_ref, bias_ref, xsum_ref, *, SUB, Bn):
    x = _norm_x(acc_ref, den_ref, bias_ref, SUB)
    ridx = (pl.program_id(0) * Bn
            + lax.broadcasted_iota(jnp.int32, x.shape, 0))
    x = jnp.where(ridx < N_NODES, x, 0.0)
    part = jnp.sum(x, axis=0, keepdims=True)
    @pl.when(pl.program_id(0) == 0)
    def _():
        xsum_ref[...] = jnp.zeros_like(xsum_ref)
    xsum_ref[...] += part


def _combine_final(acc, den, bias, SUB):
    Bn = 1264
    grid = (N_PAD // Bn,)
    return pl.pallas_call(
        functools.partial(_combine_final_body, SUB=SUB, Bn=Bn),
        grid=grid,
        in_specs=[
            pl.BlockSpec((2, Bn, 128), lambda i: (0, i, 0)),
            pl.BlockSpec((2, 16, Bn, SUB), lambda i: (0, 0, i, 0)),
            pl.BlockSpec((1, 256), lambda i: (0, 0)),
        ],
        out_specs=pl.BlockSpec((1, 256), lambda i: (0, 0)),
        out_shape=jax.ShapeDtypeStruct((1, 256), jnp.float32),
        compiler_params=pltpu.CompilerParams(
            dimension_semantics=("arbitrary",)),
    )(acc, den, bias)


def _dec1_body(xsum_ref, easum_ref, eps_ref,
               fp_ref, bfp_ref, fplg_ref, fplb_ref,
               mu_ref_w, bmu_ref, lv_ref_w, blv_ref,
               l1_ref, bl1_ref, l1g_ref, l1b_ref,
               l2_ref, bl2_ref, l2g_ref, l2b_ref,
               a1_ref, ba1_ref, c1_ref, bc1_ref, c2_ref, bc2_ref,
               e1_ref, be1_ref, n1_ref, bn1_ref, n2_ref, bn2_ref,
               p1_ref, bp1_ref, p2_ref, bp2_ref,
               h_ref, mu_ref, lv_ref, aact_ref, eact_ref,
               nn_ref, coords_ref, cell_ref):
    g = jnp.concatenate([xsum_ref[...] / N_NODES, easum_ref[...] / N_EDGES],
                        axis=1)
    g = _silu(_lnorm_rows(
        jnp.dot(g, fp_ref[...], preferred_element_type=jnp.float32) + bfp_ref[...],
        fplg_ref[...], fplb_ref[...]))
    mu = jnp.dot(g, mu_ref_w[...], preferred_element_type=jnp.float32) + bmu_ref[...]
    lv = jnp.dot(g, lv_ref_w[...], preferred_element_type=jnp.float32) + blv_ref[...]
    z = mu + eps_ref[...] * jnp.exp(0.5 * lv)
    h1 = _silu(_lnorm_rows(
        jnp.dot(z, l1_ref[...], preferred_element_type=jnp.float32) + bl1_ref[...],
        l1g_ref[...], l1b_ref[...]))
    h = _silu(_lnorm_rows(
        jnp.dot(h1, l2_ref[...], preferred_element_type=jnp.float32) + bl2_ref[...],
        l2g_ref[...], l2b_ref[...]))
    aact = _silu(jnp.dot(h, a1_ref[...], preferred_element_type=jnp.float32) + ba1_ref[...])
    cact = _silu(jnp.dot(h, c1_ref[...], preferred_element_type=jnp.float32) + bc1_ref[...])
    coords = jnp.dot(cact, c2_ref[...], preferred_element_type=jnp.float32) + bc2_ref[...]
    eact = _silu(jnp.dot(h, e1_ref[...], preferred_element_type=jnp.float32) + be1_ref[...])
    nact = _silu(jnp.dot(h, n1_ref[...], preferred_element_type=jnp.float32) + bn1_ref[...])
    nn = jnp.dot(nact, n2_ref[...], preferred_element_type=jnp.float32) + bn2_ref[...]
    pact = _silu(jnp.dot(h, p1_ref[...], preferred_element_type=jnp.float32) + bp1_ref[...])
    cp = jnp.dot(pact, p2_ref[...], preferred_element_type=jnp.float32) + bp2_ref[...]
    lane = lax.broadcasted_iota(jnp.int32, cp.shape, 1)
    sp = jnp.log1p(jnp.exp(-jnp.abs(cp))) + jnp.maximum(cp, 0.0)
    sg = 30.0 + 120.0 * jax.nn.sigmoid(cp)
    cell = jnp.where(lane < 3, sp, sg)
    h_ref[...] = h
    mu_ref[...] = mu
    lv_ref[...] = lv
    aact_ref[...] = aact
    eact_ref[...] = eact
    nn_ref[...] = nn
    coords_ref[...] = coords
    cell_ref[...] = cell


def _decoder_small(xsum, easum, eps, w):
    full = lambda shape: pl.BlockSpec(shape, lambda: (0,) * len(shape))
    out_shape = (jax.ShapeDtypeStruct((1, 512), jnp.float32),
                 jax.ShapeDtypeStruct((1, 128), jnp.float32),
                 jax.ShapeDtypeStruct((1, 128), jnp.float32),
                 jax.ShapeDtypeStruct((1, 256), jnp.float32),
                 jax.ShapeDtypeStruct((1, 512), jnp.float32),
                 jax.ShapeDtypeStruct((1, 100), jnp.float32),
                 jax.ShapeDtypeStruct((1, 300), jnp.float32),
                 jax.ShapeDtypeStruct((1, 8), jnp.float32))
    args = [xsum, easum, eps] + w
    return pl.pallas_call(
        _dec1_body,
        in_specs=[full(a.shape) for a in args],
        out_specs=tuple(full(s.shape) for s in out_shape),
        out_shape=out_shape,
    )(*args)


def _dec2_body(aact_ref, eact_ref, wa_ref, ba_ref, we_ref, be_ref,
               atom_ref, edge_ref):
    atom_ref[...] = jnp.dot(aact_ref[...], wa_ref[...],
                            preferred_element_type=jnp.float32) + ba_ref[...]
    edge_ref[...] = jnp.dot(eact_ref[...], we_ref[...],
                            preferred_element_type=jnp.float32) + be_ref[...]


def _decoder_wide(aact, eact, wat, ba, wet, be):
    T = 1280
    grid = (10240 // T,)
    full = lambda shape: pl.BlockSpec(shape, lambda i: (0,) * len(shape))
    out_shape = (jax.ShapeDtypeStruct((1, 10240), jnp.float32),
                 jax.ShapeDtypeStruct((1, 10240), jnp.float32))
    return pl.pallas_call(
        _dec2_body,
        grid=grid,
        in_specs=[
            full((1, 256)), full((1, 512)),
            pl.BlockSpec((256, T), lambda i: (0, i)),
            pl.BlockSpec((1, T), lambda i: (0, i)),
            pl.BlockSpec((512, T), lambda i: (0, i)),
            pl.BlockSpec((1, T), lambda i: (0, i)),
        ],
        out_specs=(pl.BlockSpec((1, T), lambda i: (0, i)),
                   pl.BlockSpec((1, T), lambda i: (0, i))),
        out_shape=out_shape,
        compiler_params=pltpu.CompilerParams(
            dimension_semantics=("parallel",)),
    )(aact, eact, wat, ba, wet, be)


def _row(v):
    return v.reshape(1, -1)


def kernel(node_features, edge_index, edge_features, params):
    p = params
    src = edge_index[0]
    dst = edge_index[1]

    emb_p = jnp.zeros((128, 64), jnp.float32).at[:100].set(p["ne_emb"])
    wce1_p = jnp.zeros((103, 256), jnp.float32).at[100:103].set(p["ne_ce1"]["W"].T)
    wde1_p = jnp.zeros((4, 128), jnp.float32).at[0:1].set(p["ee_de1"]["W"].T)
    wpe1_p = jnp.zeros((4, 128), jnp.float32).at[1:4].set(p["ee_pe1"]["W"].T)

    c1, c2, c3 = p["c1"], p["c2"], p["c3"]

    nf_pad = jnp.zeros((N_PAD, 103), jnp.float32).at[:N_NODES].set(node_features)
    xl1, xr1 = _node_encoder(
        nf_pad, emb_p, wce1_p, _row(p["ne_ce1"]["b"]),
        p["ne_ce2"]["W"].T, _row(p["ne_ce2"]["b"]),
        p["ne_nc"]["W"].T, _row(p["ne_nc"]["b"]),
        _row(p["ne_ln"]["g"]), _row(p["ne_ln"]["b"]),
        c1["Wl"]["W"].T, _row(c1["Wl"]["b"]),
        c1["Wr"]["W"].T, _row(c1["Wr"]["b"]))

    enc_w = [wde1_p, _row(p["ee_de1"]["b"]),
             p["ee_de2"]["W"].T, _row(p["ee_de2"]["b"]),
             wpe1_p, _row(p["ee_pe1"]["b"]),
             p["ee_pe2"]["W"].T, _row(p["ee_pe2"]["b"]),
             p["ee_ec"]["W"].T, _row(p["ee_ec"]["b"]),
             _row(p["ee_ln"]["g"]), _row(p["ee_ln"]["b"])]
    e1, ea_sum = _edge_encoder1(edge_features, enc_w, c1["We"]["W"].T)

    gat12 = _make_gat_sc(2)
    gat3 = _make_gat_sc(1)

    gsrc = jnp.concatenate([src, src + N_PAD])
    gdst = jnp.concatenate([dst, dst + N_PAD])

    def run_gat(gat, xl, xr, el, att, SUB):
        acc, den = gat(xl.reshape(2 * N_PAD, 128), xr.reshape(2 * N_PAD, 128),
                       el.reshape(2 * N_EDGES, 128), gsrc, gdst, dst,
                       att.reshape(-1))
        return acc, den.reshape(2, 16, N_PAD, SUB)

    acc1, den1 = run_gat(gat12, xl1, xr1, e1, c1["att"], 2)
    e2, e3 = _edge_encoder23(edge_features, enc_w,
                             c2["We"]["W"].T, c3["We"]["W"].T)
    xl2, xr2 = _combine_proj(acc1, den1, _row(c1["bias"]),
                             c2["Wl"]["W"].T, _row(c2["Wl"]["b"]),
                             c2["Wr"]["W"].T, _row(c2["Wr"]["b"]), SUB=2)

    acc2, den2 = run_gat(gat12, xl2, xr2, e2, c2["att"], 2)
    xl3, xr3 = _combine_proj(acc2, den2, _row(c2["bias"]),
                             c3["Wl"]["W"].T, _row(c3["Wl"]["b"]),
                             c3["Wr"]["W"].T, _row(c3["Wr"]["b"]), SUB=2)

    acc3, den3 = run_gat(gat3, xl3, xr3, e3, c3["att"], 1)
    xsum = _combine_final(acc3, den3, _row(c3["bias"]), SUB=1)

    eps = jax.random.normal(jax.random.key(42), (1, LAT), dtype=jnp.float32)

    dec_w = [
        p["fp"]["W"].T, _row(p["fp"]["b"]),
        _row(p["fp_ln"]["g"]), _row(p["fp_ln"]["b"]),
        p["mu"]["W"].T, _row(p["mu"]["b"]),
        p["lv"]["W"].T, _row(p["lv"]["b"]),
        p["d_l1"]["W"].T, _row(p["d_l1"]["b"]),
        _row(p["d_ln1"]["g"]), _row(p["d_ln1"]["b"]),
        p["d_l2"]["W"].T, _row(p["d_l2"]["b"]),
        _row(p["d_ln2"]["g"]), _row(p["d_ln2"]["b"]),
        p["d_a1"]["W"].T, _row(p["d_a1"]["b"]),
        p["d_c1"]["W"].T, _row(p["d_c1"]["b"]),
        p["d_c2"]["W"].T, _row(p["d_c2"]["b"]),
        p["d_e1"]["W"].T, _row(p["d_e1"]["b"]),
        p["d_n1"]["W"].T, _row(p["d_n1"]["b"]),
        p["d_n2"]["W"].T, _row(p["d_n2"]["b"]),
        p["d_p1"]["W"].T, _row(p["d_p1"]["b"]),
        jnp.zeros((256, 8), jnp.float32).at[:, :6].set(p["d_p2"]["W"].T),
        jnp.zeros((1, 8), jnp.float32).at[:, :6].set(_row(p["d_p2"]["b"])),
    ]
    (h, mu, log_var, aact, eact, nn_logits, coords, cell8) = _decoder_small(
        xsum, ea_sum, eps, dec_w)

    wa_p = jnp.zeros((256, 10240), jnp.float32).at[:, :10000].set(p["d_a2"]["W"].T)
    we_p = jnp.zeros((512, 10240), jnp.float32).at[:, :10000].set(p["d_e2"]["W"].T)
    ba_p = jnp.zeros((1, 10240), jnp.float32).at[:, :10000].set(_row(p["d_a2"]["b"]))
    be_p = jnp.zeros((1, 10240), jnp.float32).at[:, :10000].set(_row(p["d_e2"]["b"]))
    atom_p, edge_p = _decoder_wide(aact, eact, wa_p, ba_p, we_p, be_p)
    atom = atom_p[:, :10000]
    edge_flat = edge_p[:, :10000]

    node_out = jnp.concatenate(
        [atom.reshape(1, MAXN, 100), coords.reshape(1, MAXN, 3)], axis=-1)
    edge_logits = edge_flat.reshape(1, MAXN, MAXN)
    cell = cell8[:, :6]
    return (node_out, edge_logits, nn_logits, cell, h, mu, log_var)


# speedup vs baseline: 20.1301x; 1.0776x over previous
import functools

import jax
import jax.numpy as jnp
from jax import lax
from jax.experimental import pallas as pl
from jax.experimental.pallas import tpu as pltpu
from jax.experimental.pallas import tpu_sc as plsc

N_NODES = 10000
N_EDGES = 160000
HID = 256
LAT = 128
MAXN = 100
N_PAD = 10112

_LN_EPS = 1e-5


def _silu(x):
    return x * jax.nn.sigmoid(x)


def _lnorm_rows(x, g, b):
    m = jnp.mean(x, axis=-1, keepdims=True)
    v = jnp.mean((x - m) * (x - m), axis=-1, keepdims=True)
    return (x - m) * jax.lax.rsqrt(v + _LN_EPS) * g + b


def _halves(x):
    return jnp.concatenate([x[:, 0:128][None], x[:, 128:256][None]], axis=0)


def _node_enc_body(nf_ref, emb_ref, wce1_ref, bce1_ref, wce2_ref, bce2_ref,
                   wnc_ref, bnc_ref, lng_ref, lnb_ref,
                   wl_ref, bl_ref, wr_ref, br_ref,
                   xl_ref, xr_ref):
    nf = nf_ref[...]
    col = lax.broadcasted_iota(jnp.int32, nf.shape, 1)
    val = jnp.where(col < 100, nf, -1e30)
    rowmax = jnp.max(val, axis=1, keepdims=True)
    idx = jnp.min(jnp.where(val == rowmax, col, 10 ** 9), axis=1)
    onehot = (lax.broadcasted_iota(jnp.int32, (nf.shape[0], 128), 1)
              == idx[:, None]).astype(jnp.float32)
    ef = jnp.dot(onehot, emb_ref[...], preferred_element_type=jnp.float32)
    cf_pre = jnp.dot(nf, wce1_ref[...], preferred_element_type=jnp.float32) + bce1_ref[...]
    cf = jnp.dot(_silu(cf_pre), wce2_ref[...], preferred_element_type=jnp.float32) + bce2_ref[...]
    pre = jnp.dot(jnp.concatenate([ef, cf], axis=1), wnc_ref[...],
                  preferred_element_type=jnp.float32) + bnc_ref[...]
    x = _silu(_lnorm_rows(pre, lng_ref[...], lnb_ref[...]))
    xl = jnp.dot(x, wl_ref[...], preferred_element_type=jnp.float32) + bl_ref[...]
    xr = jnp.dot(x, wr_ref[...], preferred_element_type=jnp.float32) + br_ref[...]
    xl_ref[...] = _halves(xl)
    xr_ref[...] = _halves(xr)


def _node_encoder(nf, emb_p, wce1_p, bce1, wce2t, bce2, wnct, bnc, lng, lnb,
                  wlt, bl, wrt, br):
    Bn = 1264
    grid = (N_PAD // Bn,)
    full = lambda shape: pl.BlockSpec(shape, lambda i: (0,) * len(shape))
    out_shape = (jax.ShapeDtypeStruct((2, N_PAD, 128), jnp.float32),
                 jax.ShapeDtypeStruct((2, N_PAD, 128), jnp.float32))
    return pl.pallas_call(
        _node_enc_body,
        grid=grid,
        in_specs=[
            pl.BlockSpec((Bn, 103), lambda i: (i, 0)),
            full((128, 64)), full((103, 256)), full((1, 256)),
            full((256, 64)), full((1, 64)),
            full((128, 256)), full((1, 256)), full((1, 256)), full((1, 256)),
            full((256, 256)), full((1, 256)), full((256, 256)), full((1, 256)),
        ],
        out_specs=(pl.BlockSpec((2, Bn, 128), lambda i: (0, i, 0)),
                   pl.BlockSpec((2, Bn, 128), lambda i: (0, i, 0))),
        out_shape=out_shape,
        compiler_params=pltpu.CompilerParams(
            dimension_semantics=("parallel",)),
    )(nf, emb_p, wce1_p, bce1, wce2t, bce2, wnct, bnc, lng, lnb,
      wlt, bl, wrt, br)


def _edge_enc_core(ef_ref, wde1_ref, bde1_ref, wde2_ref, bde2_ref,
                   wpe1_ref, bpe1_ref, wpe2_ref, bpe2_ref,
                   wec_ref, bec_ref, lng_ref, lnb_ref):
    ef = ef_ref[...]
    de_pre = jnp.dot(ef, wde1_ref[...], preferred_element_type=jnp.float32) + bde1_ref[...]
    de = jnp.dot(_silu(de_pre), wde2_ref[...], preferred_element_type=jnp.float32) + bde2_ref[...]
    pe_pre = jnp.dot(ef, wpe1_ref[...], preferred_element_type=jnp.float32) + bpe1_ref[...]
    pe = jnp.dot(_silu(pe_pre), wpe2_ref[...], preferred_element_type=jnp.float32) + bpe2_ref[...]
    pre = jnp.dot(jnp.concatenate([de, pe], axis=1), wec_ref[...],
                  preferred_element_type=jnp.float32) + bec_ref[...]
    return _silu(_lnorm_rows(pre, lng_ref[...], lnb_ref[...]))


def _edge_enc1_body(ef_ref, wde1_ref, bde1_ref, wde2_ref, bde2_ref,
                    wpe1_ref, bpe1_ref, wpe2_ref, bpe2_ref,
                    wec_ref, bec_ref, lng_ref, lnb_ref,
                    we1_ref, e1_ref, easum_ref):
    ea = _edge_enc_core(ef_ref, wde1_ref, bde1_ref, wde2_ref, bde2_ref,
                        wpe1_ref, bpe1_ref, wpe2_ref, bpe2_ref,
                        wec_ref, bec_ref, lng_ref, lnb_ref)
    e1_ref[...] = _halves(
        jnp.dot(ea, we1_ref[...], preferred_element_type=jnp.float32))
    part = jnp.sum(ea, axis=0, keepdims=True)
    @pl.when(pl.program_id(0) == 0)
    def _():
        easum_ref[...] = jnp.zeros_like(easum_ref)
    easum_ref[...] += part


def _edge_enc23_body(ef_ref, wde1_ref, bde1_ref, wde2_ref, bde2_ref,
                     wpe1_ref, bpe1_ref, wpe2_ref, bpe2_ref,
                     wec_ref, bec_ref, lng_ref, lnb_ref,
                     we2_ref, we3_ref, e2_ref, e3_ref):
    ea = _edge_enc_core(ef_ref, wde1_ref, bde1_ref, wde2_ref, bde2_ref,
                        wpe1_ref, bpe1_ref, wpe2_ref, bpe2_ref,
                        wec_ref, bec_ref, lng_ref, lnb_ref)
    e2_ref[...] = _halves(
        jnp.dot(ea, we2_ref[...], preferred_element_type=jnp.float32))
    e3_ref[...] = _halves(
        jnp.dot(ea, we3_ref[...], preferred_element_type=jnp.float32))


_EE_SPECS = None


def _edge_encoder1(ef, enc_w, we1t):
    Be = 2000
    grid = (N_EDGES // Be,)
    full = lambda shape: pl.BlockSpec(shape, lambda i: (0,) * len(shape))
    espec = pl.BlockSpec((2, Be, 128), lambda i: (0, i, 0))
    return pl.pallas_call(
        _edge_enc1_body,
        grid=grid,
        in_specs=[
            pl.BlockSpec((Be, 4), lambda i: (i, 0)),
            full((4, 128)), full((1, 128)), full((128, 32)), full((1, 32)),
            full((4, 128)), full((1, 128)), full((128, 32)), full((1, 32)),
            full((64, 256)), full((1, 256)), full((1, 256)), full((1, 256)),
            full((256, 256)),
        ],
        out_specs=(espec, pl.BlockSpec((1, 256), lambda i: (0, 0))),
        out_shape=(jax.ShapeDtypeStruct((2, N_EDGES, 128), jnp.float32),
                   jax.ShapeDtypeStruct((1, 256), jnp.float32)),
        compiler_params=pltpu.CompilerParams(
            dimension_semantics=("arbitrary",)),
    )(ef, *enc_w, we1t)


def _edge_encoder23(ef, enc_w, we2t, we3t):
    Be = 2000
    grid = (N_EDGES // Be,)
    full = lambda shape: pl.BlockSpec(shape, lambda i: (0,) * len(shape))
    espec = pl.BlockSpec((2, Be, 128), lambda i: (0, i, 0))
    return pl.pallas_call(
        _edge_enc23_body,
        grid=grid,
        in_specs=[
            pl.BlockSpec((Be, 4), lambda i: (i, 0)),
            full((4, 128)), full((1, 128)), full((128, 32)), full((1, 32)),
            full((4, 128)), full((1, 128)), full((128, 32)), full((1, 32)),
            full((64, 256)), full((1, 256)), full((1, 256)), full((1, 256)),
            full((256, 256)), full((256, 256)),
        ],
        out_specs=(espec, espec),
        out_shape=(jax.ShapeDtypeStruct((2, N_EDGES, 128), jnp.float32),
                   jax.ShapeDtypeStruct((2, N_EDGES, 128), jnp.float32)),
        compiler_params=pltpu.CompilerParams(
            dimension_semantics=("parallel",)),
    )(ef, *enc_w, we2t, we3t)


def _make_gat_sc(sub):
    SUB = sub
    SV = (128 // SUB) // 16
    EPT = N_EDGES // 16
    B = 40
    NCH = EPT // B
    SCH = 10
    ZR = 8
    RPT = N_PAD // 16

    mesh = plsc.VectorSubcoreMesh(core_axis_name="c", subcore_axis_name="s")

    @functools.partial(
        pl.kernel,
        out_type=(jax.ShapeDtypeStruct((2, N_PAD, 128), jnp.float32),
                  jax.ShapeDtypeStruct((2, 16, SUB * N_PAD), jnp.float32)),
        mesh=mesh,
        compiler_params=pltpu.CompilerParams(needs_layout_passes=False),
        scratch_types=[
            pltpu.VMEM((SCH * B,), jnp.int32),
            pltpu.VMEM((SCH * B,), jnp.int32),
            pltpu.VMEM((SCH * B,), jnp.int32),
            pltpu.VMEM((B, 128), jnp.float32),
            pltpu.VMEM((B, 128), jnp.float32),
            pltpu.VMEM((B, 128), jnp.float32),
            pltpu.VMEM((B, 128), jnp.float32),
            pltpu.VMEM((B, 128), jnp.float32),
            pltpu.VMEM((SUB * N_PAD,), jnp.float32),
            pltpu.VMEM((128,), jnp.float32),
            pltpu.VMEM_SHARED((N_PAD, 128), jnp.float32),
            pltpu.SemaphoreType.DMA,
            pltpu.SemaphoreType.DMA,
        ],
    )
    def gat_sc(xl_hbm, xr_hbm, el_hbm, gsrc_hbm, gdst_hbm, dst_hbm, att_hbm,
               acc_out, den_out,
               jidx_s, iidx_s, dst_s,
               xjv0, xiv0, xjv1, xiv1, elv, den_t, attv,
               acc_s, sem0, sem1):
        c = lax.axis_index("c")
        s = lax.axis_index("s")
        zrow16 = jnp.zeros((16,), jnp.float32)
        lanes = lax.iota(jnp.int32, 16)
        bufs = ((xjv0, xiv0, sem0), (xjv1, xiv1, sem1))

        def zfill(i, _):
            for k in range(8):
                xjv0[i, pl.ds(k * 16, 16)] = zrow16
            return 0
        lax.fori_loop(0, ZR, zfill, 0)

        def zden(i, _):
            den_t[pl.ds(i * 16, 16)] = zrow16
            return 0
        lax.fori_loop(0, SUB * N_PAD // 16, zden, 0)

        r0 = s * RPT

        def zcopy(j, _):
            pltpu.sync_copy(xjv0.at[pl.ds(0, ZR)],
                            acc_s.at[pl.ds(r0 + j * ZR, ZR)])
            return 0
        lax.fori_loop(0, RPT // ZR, zcopy, 0)

        plsc.subcore_barrier()

        pltpu.sync_copy(att_hbm.at[pl.ds(c * 128, 128)], attv)
        att_regs = [attv[pl.ds(k * 16, 16)] for k in range(8)]

        def issue(j, b):
            xj, xi, sem = bufs[b]
            jc = jnp.minimum(j, SCH - 1)
            pltpu.async_copy(xl_hbm.at[jidx_s.at[pl.ds(jc * B, B)]], xj, sem)
            pltpu.async_copy(xr_hbm.at[iidx_s.at[pl.ds(jc * B, B)]], xi, sem)

        def wait(b):
            xj, xi, sem = bufs[b]
            pltpu.make_async_copy(xl_hbm.at[jidx_s.at[pl.ds(0, B)]], xj,
                                  sem).wait()
            pltpu.make_async_copy(xr_hbm.at[iidx_s.at[pl.ds(0, B)]], xi,
                                  sem).wait()

        def compute(sbase, j, b):
            xj, xi, sem = bufs[b]
            pltpu.sync_copy(
                el_hbm.at[pl.ds(c * N_EDGES + sbase + j * B, B)], elv)

            def edge(i2, _):
                accs = []
                vjs = [[None] * 8 for _ in range(4)]
                for u in range(4):
                    i = i2 * 4 + u
                    for sb in range(SUB):
                        a = jnp.zeros((16,), jnp.float32)
                        for k in range(SV):
                            kk = sb * SV + k
                            sl = pl.ds(kk * 16, 16)
                            vj = xj[i, sl]
                            vjs[u][kk] = vj
                            svv = vj + xi[i, sl] + elv[i, sl]
                            m = jnp.where(svv > 0, svv, 0.2 * svv)
                            a = a + m * att_regs[kk]
                        accs.append(a)
                tvs = [jnp.exp(jnp.broadcast_to(jnp.sum(a), (16,)))
                       for a in accs]
                for u in range(4):
                    i = i2 * 4 + u
                    g16 = (i // 16) * 16
                    dvec = dst_s[pl.ds(j * B + g16, 16)]
                    lmask = lanes == (i - g16)
                    for sb in range(SUB):
                        tv = tvs[u * SUB + sb]
                        for k in range(SV):
                            kk = sb * SV + k
                            sl = pl.ds(kk * 16, 16)
                            xj[i, sl] = vjs[u][kk] * tv
                        plsc.addupdate_scatter(
                            den_t, [dvec * SUB + sb], tv, mask=lmask)
                return 0
            lax.fori_loop(0, B // 4, edge, 0)

            pltpu.sync_copy(xj, acc_s.at[dst_s.at[pl.ds(j * B, B)]], add=True)

        def superchunk(su, _):
            sbase = s * EPT + su * SCH * B
            pltpu.sync_copy(
                gsrc_hbm.at[pl.ds(c * N_EDGES + sbase, SCH * B)], jidx_s)
            pltpu.sync_copy(
                gdst_hbm.at[pl.ds(c * N_EDGES + sbase, SCH * B)], iidx_s)
            pltpu.sync_copy(dst_hbm.at[pl.ds(sbase, SCH * B)], dst_s)
            issue(jnp.int32(0), 0)
            issue(jnp.int32(1), 1)

            def pairq(q, _):
                j0 = 2 * q
                wait(0)
                compute(sbase, j0, 0)
                issue(j0 + 2, 0)
                wait(1)
                compute(sbase, j0 + 1, 1)
                issue(j0 + 3, 1)
                return 0
            lax.fori_loop(0, SCH // 2, pairq, 0)
            wait(0)
            wait(1)
            return 0
        lax.fori_loop(0, NCH // SCH, superchunk, 0)

        plsc.subcore_barrier()

        pltpu.sync_copy(acc_s.at[pl.ds(s * RPT, RPT)],
                        acc_out.at[c, pl.ds(s * RPT, RPT)])
        pltpu.sync_copy(den_t, den_out.at[c, s])

    return gat_sc


def _norm_x(acc_ref, den_ref, bias_ref, SUB):
    cols = []
    for c in range(2):
        dpart = den_ref[c]
        dsum = jnp.sum(dpart, axis=0)
        if SUB == 2:
            cols.append(acc_ref[c][:, 0:64] / (dsum[:, 0:1] + 1e-16))
            cols.append(acc_ref[c][:, 64:128] / (dsum[:, 1:2] + 1e-16))
        else:
            cols.append(acc_ref[c] / (dsum[:, 0:1] + 1e-16))
    return jnp.maximum(jnp.concatenate(cols, axis=1) + bias_ref[...], 0.0)


def _combine_proj_body(acc_ref, den_ref, bias_ref, wl_ref, bl_ref,
                       wr_ref, br_ref, xl_ref, xr_ref, *, SUB):
    x = _norm_x(acc_ref, den_ref, bias_ref, SUB)
    xl = jnp.dot(x, wl_ref[...], preferred_element_type=jnp.float32) + bl_ref[...]
    xr = jnp.dot(x, wr_ref[...], preferred_element_type=jnp.float32) + br_ref[...]
    xl_ref[...] = _halves(xl)
    xr_ref[...] = _halves(xr)


def _combine_proj(acc, den, bias, wlt, bl, wrt, br, SUB):
    Bn = 1264
    grid = (N_PAD // Bn,)
    full = lambda shape: pl.BlockSpec(shape, lambda i: (0,) * len(shape))
    out_shape = (jax.ShapeDtypeStruct((2, N_PAD, 128), jnp.float32),
                 jax.ShapeDtypeStruct((2, N_PAD, 128), jnp.float32))
    return pl.pallas_call(
        functools.partial(_combine_proj_body, SUB=SUB),
        grid=grid,
        in_specs=[
            pl.BlockSpec((2, Bn, 128), lambda i: (0, i, 0)),
            pl.BlockSpec((2, 16, Bn, SUB), lambda i: (0, 0, i, 0)),
            full((1, 256)),
            full((256, 256)), full((1, 256)), full((256, 256)), full((1, 256)),
        ],
        out_specs=(pl.BlockSpec((2, Bn, 128), lambda i: (0, i, 0)),
                   pl.BlockSpec((2, Bn, 128), lambda i: (0, i, 0))),
        out_shape=out_shape,
        compiler_params=pltpu.CompilerParams(
            dimension_semantics=("parallel",)),
    )(acc, den, bias, wlt, bl, wrt, br)


def _combine_final_body(acc_ref, den_ref, bias_ref, xsum_ref, *, SUB, Bn):
    x = _norm_x(acc_ref, den_ref, bias_ref, SUB)
    ridx = (pl.program_id(0) * Bn
            + lax.broadcasted_iota(jnp.int32, x.shape, 0))
    x = jnp.where(ridx < N_NODES, x, 0.0)
    part = jnp.sum(x, axis=0, keepdims=True)
    @pl.when(pl.program_id(0) == 0)
    def _():
        xsum_ref[...] = jnp.zeros_like(xsum_ref)
    xsum_ref[...] += part


def _combine_final(acc, den, bias, SUB):
    Bn = 1264
    grid = (N_PAD // Bn,)
    return pl.pallas_call(
        functools.partial(_combine_final_body, SUB=SUB, Bn=Bn),
        grid=grid,
        in_specs=[
            pl.BlockSpec((2, Bn, 128), lambda i: (0, i, 0)),
            pl.BlockSpec((2, 16, Bn, SUB), lambda i: (0, 0, i, 0)),
            pl.BlockSpec((1, 256), lambda i: (0, 0)),
        ],
        out_specs=pl.BlockSpec((1, 256), lambda i: (0, 0)),
        out_shape=jax.ShapeDtypeStruct((1, 256), jnp.float32),
        compiler_params=pltpu.CompilerParams(
            dimension_semantics=("arbitrary",)),
    )(acc, den, bias)


def _dec1_body(xsum_ref, easum_ref, eps_ref,
               fp_ref, bfp_ref, fplg_ref, fplb_ref,
               mu_ref_w, bmu_ref, lv_ref_w, blv_ref,
               l1_ref, bl1_ref, l1g_ref, l1b_ref,
               l2_ref, bl2_ref, l2g_ref, l2b_ref,
               a1_ref, ba1_ref, c1_ref, bc1_ref, c2_ref, bc2_ref,
               e1_ref, be1_ref, n1_ref, bn1_ref, n2_ref, bn2_ref,
               p1_ref, bp1_ref, p2_ref, bp2_ref,
               h_ref, mu_ref, lv_ref, aact_ref, eact_ref,
               nn_ref, coords_ref, cell_ref):
    g = jnp.concatenate([xsum_ref[...] / N_NODES, easum_ref[...] / N_EDGES],
                        axis=1)
    g = _silu(_lnorm_rows(
        jnp.dot(g, fp_ref[...], preferred_element_type=jnp.float32) + bfp_ref[...],
        fplg_ref[...], fplb_ref[...]))
    mu = jnp.dot(g, mu_ref_w[...], preferred_element_type=jnp.float32) + bmu_ref[...]
    lv = jnp.dot(g, lv_ref_w[...], preferred_element_type=jnp.float32) + blv_ref[...]
    z = mu + eps_ref[...] * jnp.exp(0.5 * lv)
    h1 = _silu(_lnorm_rows(
        jnp.dot(z, l1_ref[...], preferred_element_type=jnp.float32) + bl1_ref[...],
        l1g_ref[...], l1b_ref[...]))
    h = _silu(_lnorm_rows(
        jnp.dot(h1, l2_ref[...], preferred_element_type=jnp.float32) + bl2_ref[...],
        l2g_ref[...], l2b_ref[...]))
    aact = _silu(jnp.dot(h, a1_ref[...], preferred_element_type=jnp.float32) + ba1_ref[...])
    cact = _silu(jnp.dot(h, c1_ref[...], preferred_element_type=jnp.float32) + bc1_ref[...])
    coords = jnp.dot(cact, c2_ref[...], preferred_element_type=jnp.float32) + bc2_ref[...]
    eact = _silu(jnp.dot(h, e1_ref[...], preferred_element_type=jnp.float32) + be1_ref[...])
    nact = _silu(jnp.dot(h, n1_ref[...], preferred_element_type=jnp.float32) + bn1_ref[...])
    nn = jnp.dot(nact, n2_ref[...], preferred_element_type=jnp.float32) + bn2_ref[...]
    pact = _silu(jnp.dot(h, p1_ref[...], preferred_element_type=jnp.float32) + bp1_ref[...])
    cp = jnp.dot(pact, p2_ref[...], preferred_element_type=jnp.float32) + bp2_ref[...]
    lane = lax.broadcasted_iota(jnp.int32, cp.shape, 1)
    sp = jnp.log1p(jnp.exp(-jnp.abs(cp))) + jnp.maximum(cp, 0.0)
    sg = 30.0 + 120.0 * jax.nn.sigmoid(cp)
    cell = jnp.where(lane < 3, sp, sg)
    h_ref[...] = h
    mu_ref[...] = mu
    lv_ref[...] = lv
    aact_ref[...] = aact
    eact_ref[...] = eact
    nn_ref[...] = nn
    coords_ref[...] = coords
    cell_ref[...] = cell


def _decoder_small(xsum, easum, eps, w):
    full = lambda shape: pl.BlockSpec(shape, lambda: (0,) * len(shape))
    out_shape = (jax.ShapeDtypeStruct((1, 512), jnp.float32),
                 jax.ShapeDtypeStruct((1, 128), jnp.float32),
                 jax.ShapeDtypeStruct((1, 128), jnp.float32),
                 jax.ShapeDtypeStruct((1, 256), jnp.float32),
                 jax.ShapeDtypeStruct((1, 512), jnp.float32),
                 jax.ShapeDtypeStruct((1, 100), jnp.float32),
                 jax.ShapeDtypeStruct((1, 300), jnp.float32),
                 jax.ShapeDtypeStruct((1, 8), jnp.float32))
    args = [xsum, easum, eps] + w
    return pl.pallas_call(
        _dec1_body,
        in_specs=[full(a.shape) for a in args],
        out_specs=tuple(full(s.shape) for s in out_shape),
        out_shape=out_shape,
    )(*args)


def _dec2_body(aact_ref, eact_ref, wa_ref, ba_ref, we_ref, be_ref,
               atom_ref, edge_ref):
    atom_ref[...] = jnp.dot(aact_ref[...], wa_ref[...],
                            preferred_element_type=jnp.float32) + ba_ref[...]
    edge_ref[...] = jnp.dot(eact_ref[...], we_ref[...],
                            preferred_element_type=jnp.float32) + be_ref[...]


def _decoder_wide(aact, eact, wat, ba, wet, be):
    T = 1280
    grid = (10240 // T,)
    full = lambda shape: pl.BlockSpec(shape, lambda i: (0,) * len(shape))
    out_shape = (jax.ShapeDtypeStruct((1, 10240), jnp.float32),
                 jax.ShapeDtypeStruct((1, 10240), jnp.float32))
    return pl.pallas_call(
        _dec2_body,
        grid=grid,
        in_specs=[
            full((1, 256)), full((1, 512)),
            pl.BlockSpec((256, T), lambda i: (0, i)),
            pl.BlockSpec((1, T), lambda i: (0, i)),
            pl.BlockSpec((512, T), lambda i: (0, i)),
            pl.BlockSpec((1, T), lambda i: (0, i)),
        ],
        out_specs=(pl.BlockSpec((1, T), lambda i: (0, i)),
                   pl.BlockSpec((1, T), lambda i: (0, i))),
        out_shape=out_shape,
        compiler_params=pltpu.CompilerParams(
            dimension_semantics=("parallel",)),
    )(aact, eact, wat, ba, wet, be)


def _row(v):
    return v.reshape(1, -1)


def kernel(node_features, edge_index, edge_features, params):
    p = params
    src = edge_index[0]
    dst = edge_index[1]

    emb_p = jnp.zeros((128, 64), jnp.float32).at[:100].set(p["ne_emb"])
    wce1_p = jnp.zeros((103, 256), jnp.float32).at[100:103].set(p["ne_ce1"]["W"].T)
    wde1_p = jnp.zeros((4, 128), jnp.float32).at[0:1].set(p["ee_de1"]["W"].T)
    wpe1_p = jnp.zeros((4, 128), jnp.float32).at[1:4].set(p["ee_pe1"]["W"].T)

    c1, c2, c3 = p["c1"], p["c2"], p["c3"]

    nf_pad = jnp.zeros((N_PAD, 103), jnp.float32).at[:N_NODES].set(node_features)
    xl1, xr1 = _node_encoder(
        nf_pad, emb_p, wce1_p, _row(p["ne_ce1"]["b"]),
        p["ne_ce2"]["W"].T, _row(p["ne_ce2"]["b"]),
        p["ne_nc"]["W"].T, _row(p["ne_nc"]["b"]),
        _row(p["ne_ln"]["g"]), _row(p["ne_ln"]["b"]),
        c1["Wl"]["W"].T, _row(c1["Wl"]["b"]),
        c1["Wr"]["W"].T, _row(c1["Wr"]["b"]))

    enc_w = [wde1_p, _row(p["ee_de1"]["b"]),
             p["ee_de2"]["W"].T, _row(p["ee_de2"]["b"]),
             wpe1_p, _row(p["ee_pe1"]["b"]),
             p["ee_pe2"]["W"].T, _row(p["ee_pe2"]["b"]),
             p["ee_ec"]["W"].T, _row(p["ee_ec"]["b"]),
             _row(p["ee_ln"]["g"]), _row(p["ee_ln"]["b"])]
    e1, ea_sum = _edge_encoder1(edge_features, enc_w, c1["We"]["W"].T)

    gat12 = _make_gat_sc(2)
    gat3 = _make_gat_sc(1)

    gsrc = jnp.concatenate([src, src + N_PAD])
    gdst = jnp.concatenate([dst, dst + N_PAD])

    def run_gat(gat, xl, xr, el, att, SUB):
        acc, den = gat(xl.reshape(2 * N_PAD, 128), xr.reshape(2 * N_PAD, 128),
                       el.reshape(2 * N_EDGES, 128), gsrc, gdst, dst,
                       att.reshape(-1))
        return acc, den.reshape(2, 16, N_PAD, SUB)

    acc1, den1 = run_gat(gat12, xl1, xr1, e1, c1["att"], 2)
    e2, e3 = _edge_encoder23(edge_features, enc_w,
                             c2["We"]["W"].T, c3["We"]["W"].T)
    xl2, xr2 = _combine_proj(acc1, den1, _row(c1["bias"]),
                             c2["Wl"]["W"].T, _row(c2["Wl"]["b"]),
                             c2["Wr"]["W"].T, _row(c2["Wr"]["b"]), SUB=2)

    acc2, den2 = run_gat(gat12, xl2, xr2, e2, c2["att"], 2)
    xl3, xr3 = _combine_proj(acc2, den2, _row(c2["bias"]),
                             c3["Wl"]["W"].T, _row(c3["Wl"]["b"]),
                             c3["Wr"]["W"].T, _row(c3["Wr"]["b"]), SUB=2)

    acc3, den3 = run_gat(gat3, xl3, xr3, e3, c3["att"], 1)
    xsum = _combine_final(acc3, den3, _row(c3["bias"]), SUB=1)

    eps = jax.random.normal(jax.random.key(42), (1, LAT), dtype=jnp.float32)

    dec_w = [
        p["fp"]["W"].T, _row(p["fp"]["b"]),
        _row(p["fp_ln"]["g"]), _row(p["fp_ln"]["b"]),
        p["mu"]["W"].T, _row(p["mu"]["b"]),
        p["lv"]["W"].T, _row(p["lv"]["b"]),
        p["d_l1"]["W"].T, _row(p["d_l1"]["b"]),
        _row(p["d_ln1"]["g"]), _row(p["d_ln1"]["b"]),
        p["d_l2"]["W"].T, _row(p["d_l2"]["b"]),
        _row(p["d_ln2"]["g"]), _row(p["d_ln2"]["b"]),
        p["d_a1"]["W"].T, _row(p["d_a1"]["b"]),
        p["d_c1"]["W"].T, _row(p["d_c1"]["b"]),
        p["d_c2"]["W"].T, _row(p["d_c2"]["b"]),
        p["d_e1"]["W"].T, _row(p["d_e1"]["b"]),
        p["d_n1"]["W"].T, _row(p["d_n1"]["b"]),
        p["d_n2"]["W"].T, _row(p["d_n2"]["b"]),
        p["d_p1"]["W"].T, _row(p["d_p1"]["b"]),
        jnp.zeros((256, 8), jnp.float32).at[:, :6].set(p["d_p2"]["W"].T),
        jnp.zeros((1, 8), jnp.float32).at[:, :6].set(_row(p["d_p2"]["b"])),
    ]
    (h, mu, log_var, aact, eact, nn_logits, coords, cell8) = _decoder_small(
        xsum, ea_sum, eps, dec_w)

    wa_p = jnp.zeros((256, 10240), jnp.float32).at[:, :10000].set(p["d_a2"]["W"].T)
    we_p = jnp.zeros((512, 10240), jnp.float32).at[:, :10000].set(p["d_e2"]["W"].T)
    ba_p = jnp.zeros((1, 10240), jnp.float32).at[:, :10000].set(_row(p["d_a2"]["b"]))
    be_p = jnp.zeros((1, 10240), jnp.float32).at[:, :10000].set(_row(p["d_e2"]["b"]))
    atom_p, edge_p = _decoder_wide(aact, eact, wa_p, ba_p, we_p, be_p)
    atom = atom_p[:, :10000]
    edge_flat = edge_p[:, :10000]

    node_out = jnp.concatenate(
        [atom.reshape(1, MAXN, 100), coords.reshape(1, MAXN, 3)], axis=-1)
    edge_logits = edge_flat.reshape(1, MAXN, MAXN)
    cell = cell8[:, :6]
    return (node_out, edge_logits, nn_logits, cell, h, mu, log_var)


# speedup vs baseline: 20.2260x; 1.0048x over previous
import functools

import jax
import jax.numpy as jnp
from jax import lax
from jax.experimental import pallas as pl
from jax.experimental.pallas import tpu as pltpu
from jax.experimental.pallas import tpu_sc as plsc

N_NODES = 10000
N_EDGES = 160000
HID = 256
LAT = 128
MAXN = 100
N_PAD = 10112

_LN_EPS = 1e-5


def _silu(x):
    return x * jax.nn.sigmoid(x)


def _lnorm_rows(x, g, b):
    m = jnp.mean(x, axis=-1, keepdims=True)
    v = jnp.mean((x - m) * (x - m), axis=-1, keepdims=True)
    return (x - m) * jax.lax.rsqrt(v + _LN_EPS) * g + b


def _halves(x):
    return jnp.concatenate([x[:, 0:128][None], x[:, 128:256][None]], axis=0)


def _node_enc_body(nf_ref, emb_ref, wce1_ref, bce1_ref, wce2_ref, bce2_ref,
                   wnc_ref, bnc_ref, lng_ref, lnb_ref,
                   wl_ref, bl_ref, wr_ref, br_ref,
                   xl_ref, xr_ref):
    nf = nf_ref[...]
    col = lax.broadcasted_iota(jnp.int32, nf.shape, 1)
    val = jnp.where(col < 100, nf, -1e30)
    rowmax = jnp.max(val, axis=1, keepdims=True)
    idx = jnp.min(jnp.where(val == rowmax, col, 10 ** 9), axis=1)
    onehot = (lax.broadcasted_iota(jnp.int32, (nf.shape[0], 128), 1)
              == idx[:, None]).astype(jnp.float32)
    ef = jnp.dot(onehot, emb_ref[...], preferred_element_type=jnp.float32)
    cf_pre = jnp.dot(nf, wce1_ref[...], preferred_element_type=jnp.float32) + bce1_ref[...]
    cf = jnp.dot(_silu(cf_pre), wce2_ref[...], preferred_element_type=jnp.float32) + bce2_ref[...]
    pre = jnp.dot(jnp.concatenate([ef, cf], axis=1), wnc_ref[...],
                  preferred_element_type=jnp.float32) + bnc_ref[...]
    x = _silu(_lnorm_rows(pre, lng_ref[...], lnb_ref[...]))
    xl = jnp.dot(x, wl_ref[...], preferred_element_type=jnp.float32) + bl_ref[...]
    xr = jnp.dot(x, wr_ref[...], preferred_element_type=jnp.float32) + br_ref[...]
    xl_ref[...] = _halves(xl)
    xr_ref[...] = _halves(xr)


def _node_encoder(nf, emb_p, wce1_p, bce1, wce2t, bce2, wnct, bnc, lng, lnb,
                  wlt, bl, wrt, br):
    Bn = 1264
    grid = (N_PAD // Bn,)
    full = lambda shape: pl.BlockSpec(shape, lambda i: (0,) * len(shape))
    out_shape = (jax.ShapeDtypeStruct((2, N_PAD, 128), jnp.float32),
                 jax.ShapeDtypeStruct((2, N_PAD, 128), jnp.float32))
    return pl.pallas_call(
        _node_enc_body,
        grid=grid,
        in_specs=[
            pl.BlockSpec((Bn, 103), lambda i: (i, 0)),
            full((128, 64)), full((103, 256)), full((1, 256)),
            full((256, 64)), full((1, 64)),
            full((128, 256)), full((1, 256)), full((1, 256)), full((1, 256)),
            full((256, 256)), full((1, 256)), full((256, 256)), full((1, 256)),
        ],
        out_specs=(pl.BlockSpec((2, Bn, 128), lambda i: (0, i, 0)),
                   pl.BlockSpec((2, Bn, 128), lambda i: (0, i, 0))),
        out_shape=out_shape,
        compiler_params=pltpu.CompilerParams(
            dimension_semantics=("parallel",)),
    )(nf, emb_p, wce1_p, bce1, wce2t, bce2, wnct, bnc, lng, lnb,
      wlt, bl, wrt, br)


def _edge_enc_core(ef_ref, wde1_ref, bde1_ref, wde2_ref, bde2_ref,
                   wpe1_ref, bpe1_ref, wpe2_ref, bpe2_ref,
                   wec_ref, bec_ref, lng_ref, lnb_ref):
    ef = ef_ref[...]
    de_pre = jnp.dot(ef, wde1_ref[...], preferred_element_type=jnp.float32) + bde1_ref[...]
    de = jnp.dot(_silu(de_pre), wde2_ref[...], preferred_element_type=jnp.float32) + bde2_ref[...]
    pe_pre = jnp.dot(ef, wpe1_ref[...], preferred_element_type=jnp.float32) + bpe1_ref[...]
    pe = jnp.dot(_silu(pe_pre), wpe2_ref[...], preferred_element_type=jnp.float32) + bpe2_ref[...]
    pre = jnp.dot(jnp.concatenate([de, pe], axis=1), wec_ref[...],
                  preferred_element_type=jnp.float32) + bec_ref[...]
    return _silu(_lnorm_rows(pre, lng_ref[...], lnb_ref[...]))


def _edge_enc1_body(ef_ref, wde1_ref, bde1_ref, wde2_ref, bde2_ref,
                    wpe1_ref, bpe1_ref, wpe2_ref, bpe2_ref,
                    wec_ref, bec_ref, lng_ref, lnb_ref,
                    we1_ref, e1_ref, easum_ref):
    ea = _edge_enc_core(ef_ref, wde1_ref, bde1_ref, wde2_ref, bde2_ref,
                        wpe1_ref, bpe1_ref, wpe2_ref, bpe2_ref,
                        wec_ref, bec_ref, lng_ref, lnb_ref)
    e1_ref[...] = _halves(
        jnp.dot(ea, we1_ref[...], preferred_element_type=jnp.float32))
    part = jnp.sum(ea, axis=0, keepdims=True)
    @pl.when(pl.program_id(0) == 0)
    def _():
        easum_ref[...] = jnp.zeros_like(easum_ref)
    easum_ref[...] += part


def _edge_enc23_body(ef_ref, wde1_ref, bde1_ref, wde2_ref, bde2_ref,
                     wpe1_ref, bpe1_ref, wpe2_ref, bpe2_ref,
                     wec_ref, bec_ref, lng_ref, lnb_ref,
                     we2_ref, we3_ref, e2_ref, e3_ref):
    ea = _edge_enc_core(ef_ref, wde1_ref, bde1_ref, wde2_ref, bde2_ref,
                        wpe1_ref, bpe1_ref, wpe2_ref, bpe2_ref,
                        wec_ref, bec_ref, lng_ref, lnb_ref)
    e2_ref[...] = _halves(
        jnp.dot(ea, we2_ref[...], preferred_element_type=jnp.float32))
    e3_ref[...] = _halves(
        jnp.dot(ea, we3_ref[...], preferred_element_type=jnp.float32))


_EE_SPECS = None


def _edge_encoder1(ef, enc_w, we1t):
    Be = 2000
    grid = (N_EDGES // Be,)
    full = lambda shape: pl.BlockSpec(shape, lambda i: (0,) * len(shape))
    espec = pl.BlockSpec((2, Be, 128), lambda i: (0, i, 0))
    return pl.pallas_call(
        _edge_enc1_body,
        grid=grid,
        in_specs=[
            pl.BlockSpec((Be, 4), lambda i: (i, 0)),
            full((4, 128)), full((1, 128)), full((128, 32)), full((1, 32)),
            full((4, 128)), full((1, 128)), full((128, 32)), full((1, 32)),
            full((64, 256)), full((1, 256)), full((1, 256)), full((1, 256)),
            full((256, 256)),
        ],
        out_specs=(espec, pl.BlockSpec((1, 256), lambda i: (0, 0))),
        out_shape=(jax.ShapeDtypeStruct((2, N_EDGES, 128), jnp.float32),
                   jax.ShapeDtypeStruct((1, 256), jnp.float32)),
        compiler_params=pltpu.CompilerParams(
            dimension_semantics=("arbitrary",)),
    )(ef, *enc_w, we1t)


def _edge_encoder23(ef, enc_w, we2t, we3t):
    Be = 2000
    grid = (N_EDGES // Be,)
    full = lambda shape: pl.BlockSpec(shape, lambda i: (0,) * len(shape))
    espec = pl.BlockSpec((2, Be, 128), lambda i: (0, i, 0))
    return pl.pallas_call(
        _edge_enc23_body,
        grid=grid,
        in_specs=[
            pl.BlockSpec((Be, 4), lambda i: (i, 0)),
            full((4, 128)), full((1, 128)), full((128, 32)), full((1, 32)),
            full((4, 128)), full((1, 128)), full((128, 32)), full((1, 32)),
            full((64, 256)), full((1, 256)), full((1, 256)), full((1, 256)),
            full((256, 256)), full((256, 256)),
        ],
        out_specs=(espec, espec),
        out_shape=(jax.ShapeDtypeStruct((2, N_EDGES, 128), jnp.float32),
                   jax.ShapeDtypeStruct((2, N_EDGES, 128), jnp.float32)),
        compiler_params=pltpu.CompilerParams(
            dimension_semantics=("parallel",)),
    )(ef, *enc_w, we2t, we3t)


def _make_gat_sc(sub):
    SUB = sub
    SV = (128 // SUB) // 16
    EPT = N_EDGES // 16
    B = 40
    NCH = EPT // B
    SCH = 10
    ZR = 8
    RPT = N_PAD // 16

    mesh = plsc.VectorSubcoreMesh(core_axis_name="c", subcore_axis_name="s")

    @functools.partial(
        pl.kernel,
        out_type=(jax.ShapeDtypeStruct((2, N_PAD, 128), jnp.float32),
                  jax.ShapeDtypeStruct((2, 16, SUB * N_PAD), jnp.float32)),
        mesh=mesh,
        compiler_params=pltpu.CompilerParams(needs_layout_passes=False),
        scratch_types=[
            pltpu.VMEM((SCH * B,), jnp.int32),
            pltpu.VMEM((SCH * B,), jnp.int32),
            pltpu.VMEM((SCH * B,), jnp.int32),
            pltpu.VMEM((B, 128), jnp.float32),
            pltpu.VMEM((B, 128), jnp.float32),
            pltpu.VMEM((B, 128), jnp.float32),
            pltpu.VMEM((B, 128), jnp.float32),
            pltpu.VMEM((B, 128), jnp.float32),
            pltpu.VMEM((SUB * N_PAD,), jnp.float32),
            pltpu.VMEM((128,), jnp.float32),
            pltpu.VMEM_SHARED((N_PAD, 128), jnp.float32),
            pltpu.SemaphoreType.DMA,
            pltpu.SemaphoreType.DMA,
        ],
    )
    def gat_sc(xl_hbm, xr_hbm, el_hbm, gsrc_hbm, gdst_hbm, dst_hbm, att_hbm,
               acc_out, den_out,
               jidx_s, iidx_s, dst_s,
               xjv0, xiv0, xjv1, xiv1, elv, den_t, attv,
               acc_s, sem0, sem1):
        c = lax.axis_index("c")
        s = lax.axis_index("s")
        zrow16 = jnp.zeros((16,), jnp.float32)
        lanes = lax.iota(jnp.int32, 16)
        bufs = ((xjv0, xiv0, sem0), (xjv1, xiv1, sem1))

        def zfill(i, _):
            for k in range(8):
                xjv0[i, pl.ds(k * 16, 16)] = zrow16
            return 0
        lax.fori_loop(0, ZR, zfill, 0)

        def zden(i, _):
            den_t[pl.ds(i * 16, 16)] = zrow16
            return 0
        lax.fori_loop(0, SUB * N_PAD // 16, zden, 0)

        r0 = s * RPT

        def zcopy(j, _):
            pltpu.sync_copy(xjv0.at[pl.ds(0, ZR)],
                            acc_s.at[pl.ds(r0 + j * ZR, ZR)])
            return 0
        lax.fori_loop(0, RPT // ZR, zcopy, 0)

        plsc.subcore_barrier()

        pltpu.sync_copy(att_hbm.at[pl.ds(c * 128, 128)], attv)
        att_regs = [attv[pl.ds(k * 16, 16)] for k in range(8)]

        def issue(j, b):
            xj, xi, sem = bufs[b]
            jc = jnp.minimum(j, SCH - 1)
            pltpu.async_copy(xl_hbm.at[jidx_s.at[pl.ds(jc * B, B)]], xj, sem)
            pltpu.async_copy(xr_hbm.at[iidx_s.at[pl.ds(jc * B, B)]], xi, sem)

        def wait(b):
            xj, xi, sem = bufs[b]
            pltpu.make_async_copy(xl_hbm.at[jidx_s.at[pl.ds(0, B)]], xj,
                                  sem).wait()
            pltpu.make_async_copy(xr_hbm.at[iidx_s.at[pl.ds(0, B)]], xi,
                                  sem).wait()

        def compute(sbase, j, b):
            xj, xi, sem = bufs[b]
            pltpu.sync_copy(
                el_hbm.at[pl.ds(c * N_EDGES + sbase + j * B, B)], elv)

            def edge(i2, _):
                accs = []
                vjs = [[None] * 8 for _ in range(4)]
                for u in range(4):
                    i = i2 * 4 + u
                    for sb in range(SUB):
                        a = jnp.zeros((16,), jnp.float32)
                        for k in range(SV):
                            kk = sb * SV + k
                            sl = pl.ds(kk * 16, 16)
                            vj = xj[i, sl]
                            vjs[u][kk] = vj
                            svv = vj + xi[i, sl] + elv[i, sl]
                            m = jnp.where(svv > 0, svv, 0.2 * svv)
                            a = a + m * att_regs[kk]
                        accs.append(a)
                tvs = [jnp.exp(jnp.broadcast_to(jnp.sum(a), (16,)))
                       for a in accs]
                for u in range(4):
                    i = i2 * 4 + u
                    g16 = (i // 16) * 16
                    dvec = dst_s[pl.ds(j * B + g16, 16)]
                    lmask = lanes == (i - g16)
                    for sb in range(SUB):
                        tv = tvs[u * SUB + sb]
                        for k in range(SV):
                            kk = sb * SV + k
                            sl = pl.ds(kk * 16, 16)
                            xj[i, sl] = vjs[u][kk] * tv
                        plsc.addupdate_scatter(
                            den_t, [dvec * SUB + sb], tv, mask=lmask)
                return 0
            lax.fori_loop(0, B // 4, edge, 0)

            pltpu.sync_copy(xj, acc_s.at[dst_s.at[pl.ds(j * B, B)]], add=True)

        def superchunk(su, _):
            sbase = s * EPT + su * SCH * B
            pltpu.sync_copy(
                gsrc_hbm.at[pl.ds(c * N_EDGES + sbase, SCH * B)], jidx_s)
            pltpu.sync_copy(
                gdst_hbm.at[pl.ds(c * N_EDGES + sbase, SCH * B)], iidx_s)
            pltpu.sync_copy(dst_hbm.at[pl.ds(sbase, SCH * B)], dst_s)
            issue(jnp.int32(0), 0)
            issue(jnp.int32(1), 1)

            def pairq(q, _):
                j0 = 2 * q
                wait(0)
                compute(sbase, j0, 0)
                issue(j0 + 2, 0)
                wait(1)
                compute(sbase, j0 + 1, 1)
                issue(j0 + 3, 1)
                return 0
            lax.fori_loop(0, SCH // 2, pairq, 0)
            wait(0)
            wait(1)
            return 0
        lax.fori_loop(0, NCH // SCH, superchunk, 0)

        plsc.subcore_barrier()

        pltpu.sync_copy(acc_s.at[pl.ds(s * RPT, RPT)],
                        acc_out.at[c, pl.ds(s * RPT, RPT)])
        pltpu.sync_copy(den_t, den_out.at[c, s])

    return gat_sc


def _norm_x(acc_ref, den_ref, bias_ref, SUB):
    cols = []
    for c in range(2):
        dpart = den_ref[c]
        dsum = jnp.sum(dpart, axis=0)
        if SUB == 2:
            cols.append(acc_ref[c][:, 0:64] / (dsum[:, 0:1] + 1e-16))
            cols.append(acc_ref[c][:, 64:128] / (dsum[:, 1:2] + 1e-16))
        else:
            cols.append(acc_ref[c] / (dsum[:, 0:1] + 1e-16))
    return jnp.maximum(jnp.concatenate(cols, axis=1) + bias_ref[...], 0.0)


def _combine_proj_body(acc_ref, den_ref, bias_ref, wl_ref, bl_ref,
                       wr_ref, br_ref, xl_ref, xr_ref, *, SUB):
    x = _norm_x(acc_ref, den_ref, bias_ref, SUB)
    xl = jnp.dot(x, wl_ref[...], preferred_element_type=jnp.float32) + bl_ref[...]
    xr = jnp.dot(x, wr_ref[...], preferred_element_type=jnp.float32) + br_ref[...]
    xl_ref[...] = _halves(xl)
    xr_ref[...] = _halves(xr)


def _combine_proj(acc, den, bias, wlt, bl, wrt, br, SUB):
    Bn = 1264
    grid = (N_PAD // Bn,)
    full = lambda shape: pl.BlockSpec(shape, lambda i: (0,) * len(shape))
    out_shape = (jax.ShapeDtypeStruct((2, N_PAD, 128), jnp.float32),
                 jax.ShapeDtypeStruct((2, N_PAD, 128), jnp.float32))
    return pl.pallas_call(
        functools.partial(_combine_proj_body, SUB=SUB),
        grid=grid,
        in_specs=[
            pl.BlockSpec((2, Bn, 128), lambda i: (0, i, 0)),
            pl.BlockSpec((2, 16, Bn, SUB), lambda i: (0, 0, i, 0)),
            full((1, 256)),
            full((256, 256)), full((1, 256)), full((256, 256)), full((1, 256)),
        ],
        out_specs=(pl.BlockSpec((2, Bn, 128), lambda i: (0, i, 0)),
                   pl.BlockSpec((2, Bn, 128), lambda i: (0, i, 0))),
        out_shape=out_shape,
        compiler_params=pltpu.CompilerParams(
            dimension_semantics=("parallel",)),
    )(acc, den, bias, wlt, bl, wrt, br)


def _combine_final_body(acc_ref, den_ref, bias_ref, xsum_ref, *, SUB, Bn):
    x = _norm_x(acc_ref, den_ref, bias_ref, SUB)
    ridx = (pl.program_id(0) * Bn
            + lax.broadcasted_iota(jnp.int32, x.shape, 0))
    x = jnp.where(ridx < N_NODES, x, 0.0)
    part = jnp.sum(x, axis=0, keepdims=True)
    @pl.when(pl.program_id(0) == 0)
    def _():
        xsum_ref[...] = jnp.zeros_like(xsum_ref)
    xsum_ref[...] += part


def _combine_final(acc, den, bias, SUB):
    Bn = 1264
    grid = (N_PAD // Bn,)
    return pl.pallas_call(
        functools.partial(_combine_final_body, SUB=SUB, Bn=Bn),
        grid=grid,
        in_specs=[
            pl.BlockSpec((2, Bn, 128), lambda i: (0, i, 0)),
            pl.BlockSpec((2, 16, Bn, SUB), lambda i: (0, 0, i, 0)),
            pl.BlockSpec((1, 256), lambda i: (0, 0)),
        ],
        out_specs=pl.BlockSpec((1, 256), lambda i: (0, 0)),
        out_shape=jax.ShapeDtypeStruct((1, 256), jnp.float32),
        compiler_params=pltpu.CompilerParams(
            dimension_semantics=("arbitrary",)),
    )(acc, den, bias)


def _dec1_body(xsum_ref, easum_ref, eps_ref,
               fp_ref, bfp_ref, fplg_ref, fplb_ref,
               mu_ref_w, bmu_ref, lv_ref_w, blv_ref,
               l1_ref, bl1_ref, l1g_ref, l1b_ref,
               l2_ref, bl2_ref, l2g_ref, l2b_ref,
               a1_ref, ba1_ref, c1_ref, bc1_ref, c2_ref, bc2_ref,
               e1_ref, be1_ref, n1_ref, bn1_ref, n2_ref, bn2_ref,
               p1_ref, bp1_ref, p2_ref, bp2_ref,
               h_ref, mu_ref, lv_ref, aact_ref, eact_ref,
               nn_ref, coords_ref, cell_ref):
    g = jnp.concatenate([xsum_ref[...] / N_NODES, easum_ref[...] / N_EDGES],
                        axis=1)
    g = _silu(_lnorm_rows(
        jnp.dot(g, fp_ref[...], preferred_element_type=jnp.float32) + bfp_ref[...],
        fplg_ref[...], fplb_ref[...]))
    mu = jnp.dot(g, mu_ref_w[...], preferred_element_type=jnp.float32) + bmu_ref[...]
    lv = jnp.dot(g, lv_ref_w[...], preferred_element_type=jnp.float32) + blv_ref[...]
    z = mu + eps_ref[...] * jnp.exp(0.5 * lv)
    h1 = _silu(_lnorm_rows(
        jnp.dot(z, l1_ref[...], preferred_element_type=jnp.float32) + bl1_ref[...],
        l1g_ref[...], l1b_ref[...]))
    h = _silu(_lnorm_rows(
        jnp.dot(h1, l2_ref[...], preferred_element_type=jnp.float32) + bl2_ref[...],
        l2g_ref[...], l2b_ref[...]))
    aact = _silu(jnp.dot(h, a1_ref[...], preferred_element_type=jnp.float32) + ba1_ref[...])
    cact = _silu(jnp.dot(h, c1_ref[...], preferred_element_type=jnp.float32) + bc1_ref[...])
    coords = jnp.dot(cact, c2_ref[...], preferred_element_type=jnp.float32) + bc2_ref[...]
    eact = _silu(jnp.dot(h, e1_ref[...], preferred_element_type=jnp.float32) + be1_ref[...])
    nact = _silu(jnp.dot(h, n1_ref[...], preferred_element_type=jnp.float32) + bn1_ref[...])
    nn = jnp.dot(nact, n2_ref[...], preferred_element_type=jnp.float32) + bn2_ref[...]
    pact = _silu(jnp.dot(h, p1_ref[...], preferred_element_type=jnp.float32) + bp1_ref[...])
    cp = jnp.dot(pact, p2_ref[...], preferred_element_type=jnp.float32) + bp2_ref[...]
    lane = lax.broadcasted_iota(jnp.int32, cp.shape, 1)
    sp = jnp.log1p(jnp.exp(-jnp.abs(cp))) + jnp.maximum(cp, 0.0)
    sg = 30.0 + 120.0 * jax.nn.sigmoid(cp)
    cell = jnp.where(lane < 3, sp, sg)
    h_ref[...] = h
    mu_ref[...] = mu
    lv_ref[...] = lv
    aact_ref[...] = aact
    eact_ref[...] = eact
    nn_ref[...] = nn
    coords_ref[...] = coords
    cell_ref[...] = cell


def _decoder_small(xsum, easum, eps, w):
    full = lambda shape: pl.BlockSpec(shape, lambda: (0,) * len(shape))
    out_shape = (jax.ShapeDtypeStruct((1, 512), jnp.float32),
                 jax.ShapeDtypeStruct((1, 128), jnp.float32),
                 jax.ShapeDtypeStruct((1, 128), jnp.float32),
                 jax.ShapeDtypeStruct((1, 256), jnp.float32),
                 jax.ShapeDtypeStruct((1, 512), jnp.float32),
                 jax.ShapeDtypeStruct((1, 100), jnp.float32),
                 jax.ShapeDtypeStruct((1, 300), jnp.float32),
                 jax.ShapeDtypeStruct((1, 8), jnp.float32))
    args = [xsum, easum, eps] + w
    return pl.pallas_call(
        _dec1_body,
        in_specs=[full(a.shape) for a in args],
        out_specs=tuple(full(s.shape) for s in out_shape),
        out_shape=out_shape,
    )(*args)


def _dec2_body(aact_ref, eact_ref, wa_ref, ba_ref, we_ref, be_ref,
               atom_ref, edge_ref):
    dn = (((1,), (1,)), ((), ()))
    atom_ref[...] = lax.dot_general(
        aact_ref[...], wa_ref[...], dn,
        preferred_element_type=jnp.float32) + ba_ref[...]
    edge_ref[...] = lax.dot_general(
        eact_ref[...], we_ref[...], dn,
        preferred_element_type=jnp.float32) + be_ref[...]


def _decoder_wide(aact, eact, wa, ba, we, be):
    full = lambda shape: pl.BlockSpec(shape, lambda: (0,) * len(shape))
    out_shape = (jax.ShapeDtypeStruct((1, 10000), jnp.float32),
                 jax.ShapeDtypeStruct((1, 10000), jnp.float32))
    return pl.pallas_call(
        _dec2_body,
        in_specs=[
            full((1, 256)), full((1, 512)),
            full((10000, 256)), full((1, 10000)),
            full((10000, 512)), full((1, 10000)),
        ],
        out_specs=tuple(full(s.shape) for s in out_shape),
        out_shape=out_shape,
    )(aact, eact, wa, ba, we, be)


def _row(v):
    return v.reshape(1, -1)


def kernel(node_features, edge_index, edge_features, params):
    p = params
    src = edge_index[0]
    dst = edge_index[1]

    emb_p = jnp.zeros((128, 64), jnp.float32).at[:100].set(p["ne_emb"])
    wce1_p = jnp.zeros((103, 256), jnp.float32).at[100:103].set(p["ne_ce1"]["W"].T)
    wde1_p = jnp.zeros((4, 128), jnp.float32).at[0:1].set(p["ee_de1"]["W"].T)
    wpe1_p = jnp.zeros((4, 128), jnp.float32).at[1:4].set(p["ee_pe1"]["W"].T)

    c1, c2, c3 = p["c1"], p["c2"], p["c3"]

    nf_pad = jnp.zeros((N_PAD, 103), jnp.float32).at[:N_NODES].set(node_features)
    xl1, xr1 = _node_encoder(
        nf_pad, emb_p, wce1_p, _row(p["ne_ce1"]["b"]),
        p["ne_ce2"]["W"].T, _row(p["ne_ce2"]["b"]),
        p["ne_nc"]["W"].T, _row(p["ne_nc"]["b"]),
        _row(p["ne_ln"]["g"]), _row(p["ne_ln"]["b"]),
        c1["Wl"]["W"].T, _row(c1["Wl"]["b"]),
        c1["Wr"]["W"].T, _row(c1["Wr"]["b"]))

    enc_w = [wde1_p, _row(p["ee_de1"]["b"]),
             p["ee_de2"]["W"].T, _row(p["ee_de2"]["b"]),
             wpe1_p, _row(p["ee_pe1"]["b"]),
             p["ee_pe2"]["W"].T, _row(p["ee_pe2"]["b"]),
             p["ee_ec"]["W"].T, _row(p["ee_ec"]["b"]),
             _row(p["ee_ln"]["g"]), _row(p["ee_ln"]["b"])]
    e1, ea_sum = _edge_encoder1(edge_features, enc_w, c1["We"]["W"].T)

    gat12 = _make_gat_sc(2)
    gat3 = _make_gat_sc(1)

    gsrc = jnp.concatenate([src, src + N_PAD])
    gdst = jnp.concatenate([dst, dst + N_PAD])

    def run_gat(gat, xl, xr, el, att, SUB):
        acc, den = gat(xl.reshape(2 * N_PAD, 128), xr.reshape(2 * N_PAD, 128),
                       el.reshape(2 * N_EDGES, 128), gsrc, gdst, dst,
                       att.reshape(-1))
        return acc, den.reshape(2, 16, N_PAD, SUB)

    acc1, den1 = run_gat(gat12, xl1, xr1, e1, c1["att"], 2)
    e2, e3 = _edge_encoder23(edge_features, enc_w,
                             c2["We"]["W"].T, c3["We"]["W"].T)
    xl2, xr2 = _combine_proj(acc1, den1, _row(c1["bias"]),
                             c2["Wl"]["W"].T, _row(c2["Wl"]["b"]),
                             c2["Wr"]["W"].T, _row(c2["Wr"]["b"]), SUB=2)

    acc2, den2 = run_gat(gat12, xl2, xr2, e2, c2["att"], 2)
    xl3, xr3 = _combine_proj(acc2, den2, _row(c2["bias"]),
                             c3["Wl"]["W"].T, _row(c3["Wl"]["b"]),
                             c3["Wr"]["W"].T, _row(c3["Wr"]["b"]), SUB=2)

    acc3, den3 = run_gat(gat3, xl3, xr3, e3, c3["att"], 1)
    xsum = _combine_final(acc3, den3, _row(c3["bias"]), SUB=1)

    eps = jax.random.normal(jax.random.key(42), (1, LAT), dtype=jnp.float32)

    dec_w = [
        p["fp"]["W"].T, _row(p["fp"]["b"]),
        _row(p["fp_ln"]["g"]), _row(p["fp_ln"]["b"]),
        p["mu"]["W"].T, _row(p["mu"]["b"]),
        p["lv"]["W"].T, _row(p["lv"]["b"]),
        p["d_l1"]["W"].T, _row(p["d_l1"]["b"]),
        _row(p["d_ln1"]["g"]), _row(p["d_ln1"]["b"]),
        p["d_l2"]["W"].T, _row(p["d_l2"]["b"]),
        _row(p["d_ln2"]["g"]), _row(p["d_ln2"]["b"]),
        p["d_a1"]["W"].T, _row(p["d_a1"]["b"]),
        p["d_c1"]["W"].T, _row(p["d_c1"]["b"]),
        p["d_c2"]["W"].T, _row(p["d_c2"]["b"]),
        p["d_e1"]["W"].T, _row(p["d_e1"]["b"]),
        p["d_n1"]["W"].T, _row(p["d_n1"]["b"]),
        p["d_n2"]["W"].T, _row(p["d_n2"]["b"]),
        p["d_p1"]["W"].T, _row(p["d_p1"]["b"]),
        jnp.zeros((256, 8), jnp.float32).at[:, :6].set(p["d_p2"]["W"].T),
        jnp.zeros((1, 8), jnp.float32).at[:, :6].set(_row(p["d_p2"]["b"])),
    ]
    (h, mu, log_var, aact, eact, nn_logits, coords, cell8) = _decoder_small(
        xsum, ea_sum, eps, dec_w)

    atom, edge_flat = _decoder_wide(
        aact, eact, p["d_a2"]["W"], _row(p["d_a2"]["b"]),
        p["d_e2"]["W"], _row(p["d_e2"]["b"]))

    node_out = jnp.concatenate(
        [atom.reshape(1, MAXN, 100), coords.reshape(1, MAXN, 3)], axis=-1)
    edge_logits = edge_flat.reshape(1, MAXN, MAXN)
    cell = cell8[:, :6]
    return (node_out, edge_logits, nn_logits, cell, h, mu, log_var)


# speedup vs baseline: 20.2288x; 1.0001x over previous
import functools

import jax
import jax.numpy as jnp
from jax import lax
from jax.experimental import pallas as pl
from jax.experimental.pallas import tpu as pltpu
from jax.experimental.pallas import tpu_sc as plsc

N_NODES = 10000
N_EDGES = 160000
HID = 256
LAT = 128
MAXN = 100
N_PAD = 10112

_LN_EPS = 1e-5


def _silu(x):
    return x * jax.nn.sigmoid(x)


def _lnorm_rows(x, g, b):
    m = jnp.mean(x, axis=-1, keepdims=True)
    v = jnp.mean((x - m) * (x - m), axis=-1, keepdims=True)
    return (x - m) * jax.lax.rsqrt(v + _LN_EPS) * g + b


def _halves(x):
    return jnp.concatenate([x[:, 0:128][None], x[:, 128:256][None]], axis=0)


def _node_enc_body(nf_ref, emb_ref, wce1_ref, bce1_ref, wce2_ref, bce2_ref,
                   wnc_ref, bnc_ref, lng_ref, lnb_ref,
                   wl_ref, bl_ref, wr_ref, br_ref,
                   xl_ref, xr_ref):
    nf = nf_ref[...]
    col = lax.broadcasted_iota(jnp.int32, nf.shape, 1)
    val = jnp.where(col < 100, nf, -1e30)
    rowmax = jnp.max(val, axis=1, keepdims=True)
    idx = jnp.min(jnp.where(val == rowmax, col, 10 ** 9), axis=1)
    onehot = (lax.broadcasted_iota(jnp.int32, (nf.shape[0], 128), 1)
              == idx[:, None]).astype(jnp.float32)
    ef = jnp.dot(onehot, emb_ref[...], preferred_element_type=jnp.float32)
    cf_pre = jnp.dot(nf, wce1_ref[...], preferred_element_type=jnp.float32) + bce1_ref[...]
    cf = jnp.dot(_silu(cf_pre), wce2_ref[...], preferred_element_type=jnp.float32) + bce2_ref[...]
    pre = jnp.dot(jnp.concatenate([ef, cf], axis=1), wnc_ref[...],
                  preferred_element_type=jnp.float32) + bnc_ref[...]
    x = _silu(_lnorm_rows(pre, lng_ref[...], lnb_ref[...]))
    xl = jnp.dot(x, wl_ref[...], preferred_element_type=jnp.float32) + bl_ref[...]
    xr = jnp.dot(x, wr_ref[...], preferred_element_type=jnp.float32) + br_ref[...]
    xl_ref[...] = _halves(xl)
    xr_ref[...] = _halves(xr)


def _node_encoder(nf, emb_p, wce1_p, bce1, wce2t, bce2, wnct, bnc, lng, lnb,
                  wlt, bl, wrt, br):
    Bn = 1264
    grid = (N_PAD // Bn,)
    full = lambda shape: pl.BlockSpec(shape, lambda i: (0,) * len(shape))
    out_shape = (jax.ShapeDtypeStruct((2, N_PAD, 128), jnp.float32),
                 jax.ShapeDtypeStruct((2, N_PAD, 128), jnp.float32))
    return pl.pallas_call(
        _node_enc_body,
        grid=grid,
        in_specs=[
            pl.BlockSpec((Bn, 103), lambda i: (i, 0)),
            full((128, 64)), full((103, 256)), full((1, 256)),
            full((256, 64)), full((1, 64)),
            full((128, 256)), full((1, 256)), full((1, 256)), full((1, 256)),
            full((256, 256)), full((1, 256)), full((256, 256)), full((1, 256)),
        ],
        out_specs=(pl.BlockSpec((2, Bn, 128), lambda i: (0, i, 0)),
                   pl.BlockSpec((2, Bn, 128), lambda i: (0, i, 0))),
        out_shape=out_shape,
        compiler_params=pltpu.CompilerParams(
            dimension_semantics=("parallel",)),
    )(nf, emb_p, wce1_p, bce1, wce2t, bce2, wnct, bnc, lng, lnb,
      wlt, bl, wrt, br)


def _edge_enc_core(ef_ref, wde1_ref, bde1_ref, wde2_ref, bde2_ref,
                   wpe1_ref, bpe1_ref, wpe2_ref, bpe2_ref,
                   wec_ref, bec_ref, lng_ref, lnb_ref):
    ef = ef_ref[...]
    de_pre = jnp.dot(ef, wde1_ref[...], preferred_element_type=jnp.float32) + bde1_ref[...]
    de = jnp.dot(_silu(de_pre), wde2_ref[...], preferred_element_type=jnp.float32) + bde2_ref[...]
    pe_pre = jnp.dot(ef, wpe1_ref[...], preferred_element_type=jnp.float32) + bpe1_ref[...]
    pe = jnp.dot(_silu(pe_pre), wpe2_ref[...], preferred_element_type=jnp.float32) + bpe2_ref[...]
    pre = jnp.dot(jnp.concatenate([de, pe], axis=1), wec_ref[...],
                  preferred_element_type=jnp.float32) + bec_ref[...]
    return _silu(_lnorm_rows(pre, lng_ref[...], lnb_ref[...]))


def _edge_enc1_body(ef_ref, wde1_ref, bde1_ref, wde2_ref, bde2_ref,
                    wpe1_ref, bpe1_ref, wpe2_ref, bpe2_ref,
                    wec_ref, bec_ref, lng_ref, lnb_ref,
                    we1_ref, e1_ref, easum_ref):
    ea = _edge_enc_core(ef_ref, wde1_ref, bde1_ref, wde2_ref, bde2_ref,
                        wpe1_ref, bpe1_ref, wpe2_ref, bpe2_ref,
                        wec_ref, bec_ref, lng_ref, lnb_ref)
    e1_ref[...] = _halves(
        jnp.dot(ea, we1_ref[...], preferred_element_type=jnp.float32))
    part = jnp.sum(ea, axis=0, keepdims=True)
    @pl.when(pl.program_id(0) == 0)
    def _():
        easum_ref[...] = jnp.zeros_like(easum_ref)
    easum_ref[...] += part


def _edge_enc23_body(ef_ref, wde1_ref, bde1_ref, wde2_ref, bde2_ref,
                     wpe1_ref, bpe1_ref, wpe2_ref, bpe2_ref,
                     wec_ref, bec_ref, lng_ref, lnb_ref,
                     we2_ref, we3_ref, e2_ref, e3_ref):
    ea = _edge_enc_core(ef_ref, wde1_ref, bde1_ref, wde2_ref, bde2_ref,
                        wpe1_ref, bpe1_ref, wpe2_ref, bpe2_ref,
                        wec_ref, bec_ref, lng_ref, lnb_ref)
    e2_ref[...] = _halves(
        jnp.dot(ea, we2_ref[...], preferred_element_type=jnp.float32))
    e3_ref[...] = _halves(
        jnp.dot(ea, we3_ref[...], preferred_element_type=jnp.float32))


def _edge_encoder1(ef, enc_w, we1t):
    Be = 2000
    grid = (N_EDGES // Be,)
    full = lambda shape: pl.BlockSpec(shape, lambda i: (0,) * len(shape))
    espec = pl.BlockSpec((2, Be, 128), lambda i: (0, i, 0))
    return pl.pallas_call(
        _edge_enc1_body,
        grid=grid,
        in_specs=[
            pl.BlockSpec((Be, 4), lambda i: (i, 0)),
            full((4, 128)), full((1, 128)), full((128, 32)), full((1, 32)),
            full((4, 128)), full((1, 128)), full((128, 32)), full((1, 32)),
            full((64, 256)), full((1, 256)), full((1, 256)), full((1, 256)),
            full((256, 256)),
        ],
        out_specs=(espec, pl.BlockSpec((1, 256), lambda i: (0, 0))),
        out_shape=(jax.ShapeDtypeStruct((2, N_EDGES, 128), jnp.float32),
                   jax.ShapeDtypeStruct((1, 256), jnp.float32)),
        compiler_params=pltpu.CompilerParams(
            dimension_semantics=("arbitrary",)),
    )(ef, *enc_w, we1t)


def _edge_encoder23(ef, enc_w, we2t, we3t):
    Be = 2000
    grid = (N_EDGES // Be,)
    full = lambda shape: pl.BlockSpec(shape, lambda i: (0,) * len(shape))
    espec = pl.BlockSpec((2, Be, 128), lambda i: (0, i, 0))
    return pl.pallas_call(
        _edge_enc23_body,
        grid=grid,
        in_specs=[
            pl.BlockSpec((Be, 4), lambda i: (i, 0)),
            full((4, 128)), full((1, 128)), full((128, 32)), full((1, 32)),
            full((4, 128)), full((1, 128)), full((128, 32)), full((1, 32)),
            full((64, 256)), full((1, 256)), full((1, 256)), full((1, 256)),
            full((256, 256)), full((256, 256)),
        ],
        out_specs=(espec, espec),
        out_shape=(jax.ShapeDtypeStruct((2, N_EDGES, 128), jnp.float32),
                   jax.ShapeDtypeStruct((2, N_EDGES, 128), jnp.float32)),
        compiler_params=pltpu.CompilerParams(
            dimension_semantics=("parallel",)),
    )(ef, *enc_w, we2t, we3t)


def _make_gat_sc(sub):
    SUB = sub
    SV = (128 // SUB) // 16
    EPT = N_EDGES // 16
    B = 40
    NCH = EPT // B
    SCH = 10
    ZR = 8
    RPT = N_PAD // 16

    mesh = plsc.VectorSubcoreMesh(core_axis_name="c", subcore_axis_name="s")

    @functools.partial(
        pl.kernel,
        out_type=(jax.ShapeDtypeStruct((2, N_PAD, 128), jnp.float32),
                  jax.ShapeDtypeStruct((2, 16, SUB * N_PAD), jnp.float32)),
        mesh=mesh,
        compiler_params=pltpu.CompilerParams(needs_layout_passes=False),
        scratch_types=[
            pltpu.VMEM((SCH * B,), jnp.int32),
            pltpu.VMEM((SCH * B,), jnp.int32),
            pltpu.VMEM((SCH * B,), jnp.int32),
            pltpu.VMEM((B, 128), jnp.float32),
            pltpu.VMEM((B, 128), jnp.float32),
            pltpu.VMEM((B, 128), jnp.float32),
            pltpu.VMEM((B, 128), jnp.float32),
            pltpu.VMEM((B, 128), jnp.float32),
            pltpu.VMEM((SUB * N_PAD,), jnp.float32),
            pltpu.VMEM((128,), jnp.float32),
            pltpu.VMEM_SHARED((N_PAD, 128), jnp.float32),
            pltpu.SemaphoreType.DMA,
            pltpu.SemaphoreType.DMA,
        ],
    )
    def gat_sc(xl_hbm, xr_hbm, el_hbm, gsrc_hbm, gdst_hbm, dst_hbm, att_hbm,
               acc_out, den_out,
               jidx_s, iidx_s, dst_s,
               xjv0, xiv0, xjv1, xiv1, elv, den_t, attv,
               acc_s, sem0, sem1):
        c = lax.axis_index("c")
        s = lax.axis_index("s")
        zrow16 = jnp.zeros((16,), jnp.float32)
        lanes = lax.iota(jnp.int32, 16)
        bufs = ((xjv0, xiv0, sem0), (xjv1, xiv1, sem1))

        def zfill(i, _):
            for k in range(8):
                xjv0[i, pl.ds(k * 16, 16)] = zrow16
            return 0
        lax.fori_loop(0, ZR, zfill, 0)

        def zden(i, _):
            den_t[pl.ds(i * 16, 16)] = zrow16
            return 0
        lax.fori_loop(0, SUB * N_PAD // 16, zden, 0)

        r0 = s * RPT

        def zcopy(j, _):
            pltpu.sync_copy(xjv0.at[pl.ds(0, ZR)],
                            acc_s.at[pl.ds(r0 + j * ZR, ZR)])
            return 0
        lax.fori_loop(0, RPT // ZR, zcopy, 0)

        plsc.subcore_barrier()

        pltpu.sync_copy(att_hbm.at[pl.ds(c * 128, 128)], attv)
        att_regs = [attv[pl.ds(k * 16, 16)] for k in range(8)]

        def issue(j, b):
            xj, xi, sem = bufs[b]
            jc = jnp.minimum(j, SCH - 1)
            pltpu.async_copy(xl_hbm.at[jidx_s.at[pl.ds(jc * B, B)]], xj, sem)
            pltpu.async_copy(xr_hbm.at[iidx_s.at[pl.ds(jc * B, B)]], xi, sem)

        def wait(b):
            xj, xi, sem = bufs[b]
            pltpu.make_async_copy(xl_hbm.at[jidx_s.at[pl.ds(0, B)]], xj,
                                  sem).wait()
            pltpu.make_async_copy(xr_hbm.at[iidx_s.at[pl.ds(0, B)]], xi,
                                  sem).wait()

        def compute(sbase, j, b):
            xj, xi, sem = bufs[b]
            pltpu.sync_copy(
                el_hbm.at[pl.ds(c * N_EDGES + sbase + j * B, B)], elv)

            def edge(i2, _):
                accs = []
                vjs = [[None] * 8 for _ in range(4)]
                for u in range(4):
                    i = i2 * 4 + u
                    for sb in range(SUB):
                        a = jnp.zeros((16,), jnp.float32)
                        for k in range(SV):
                            kk = sb * SV + k
                            sl = pl.ds(kk * 16, 16)
                            vj = xj[i, sl]
                            vjs[u][kk] = vj
                            svv = vj + xi[i, sl] + elv[i, sl]
                            m = jnp.where(svv > 0, svv, 0.2 * svv)
                            a = a + m * att_regs[kk]
                        accs.append(a)
                tvs = [jnp.exp(jnp.broadcast_to(jnp.sum(a), (16,)))
                       for a in accs]
                for u in range(4):
                    i = i2 * 4 + u
                    g16 = (i // 16) * 16
                    dvec = dst_s[pl.ds(j * B + g16, 16)]
                    lmask = lanes == (i - g16)
                    for sb in range(SUB):
                        tv = tvs[u * SUB + sb]
                        for k in range(SV):
                            kk = sb * SV + k
                            sl = pl.ds(kk * 16, 16)
                            xj[i, sl] = vjs[u][kk] * tv
                        plsc.addupdate_scatter(
                            den_t, [dvec * SUB + sb], tv, mask=lmask)
                return 0
            lax.fori_loop(0, B // 4, edge, 0)

            pltpu.sync_copy(xj, acc_s.at[dst_s.at[pl.ds(j * B, B)]], add=True)

        def superchunk(su, _):
            sbase = s * EPT + su * SCH * B
            pltpu.sync_copy(
                gsrc_hbm.at[pl.ds(c * N_EDGES + sbase, SCH * B)], jidx_s)
            pltpu.sync_copy(
                gdst_hbm.at[pl.ds(c * N_EDGES + sbase, SCH * B)], iidx_s)
            pltpu.sync_copy(dst_hbm.at[pl.ds(sbase, SCH * B)], dst_s)
            issue(jnp.int32(0), 0)
            issue(jnp.int32(1), 1)

            def pairq(q, _):
                j0 = 2 * q
                wait(0)
                compute(sbase, j0, 0)
                issue(j0 + 2, 0)
                wait(1)
                compute(sbase, j0 + 1, 1)
                issue(j0 + 3, 1)
                return 0
            lax.fori_loop(0, SCH // 2, pairq, 0)
            wait(0)
            wait(1)
            return 0
        lax.fori_loop(0, NCH // SCH, superchunk, 0)

        plsc.subcore_barrier()

        pltpu.sync_copy(acc_s.at[pl.ds(s * RPT, RPT)],
                        acc_out.at[c, pl.ds(s * RPT, RPT)])
        pltpu.sync_copy(den_t, den_out.at[c, s])

    return gat_sc


def _norm_x(acc_ref, den_ref, bias_ref, SUB):
    cols = []
    for c in range(2):
        dpart = den_ref[c]
        dsum = jnp.sum(dpart, axis=0)
        if SUB == 2:
            cols.append(acc_ref[c][:, 0:64] / (dsum[:, 0:1] + 1e-16))
            cols.append(acc_ref[c][:, 64:128] / (dsum[:, 1:2] + 1e-16))
        else:
            cols.append(acc_ref[c] / (dsum[:, 0:1] + 1e-16))
    return jnp.maximum(jnp.concatenate(cols, axis=1) + bias_ref[...], 0.0)


def _combine_proj_body(acc_ref, den_ref, bias_ref, wl_ref, bl_ref,
                       wr_ref, br_ref, xl_ref, xr_ref, *, SUB):
    x = _norm_x(acc_ref, den_ref, bias_ref, SUB)
    xl = jnp.dot(x, wl_ref[...], preferred_element_type=jnp.float32) + bl_ref[...]
    xr = jnp.dot(x, wr_ref[...], preferred_element_type=jnp.float32) + br_ref[...]
    xl_ref[...] = _halves(xl)
    xr_ref[...] = _halves(xr)


def _combine_proj(acc, den, bias, wlt, bl, wrt, br, SUB):
    Bn = 1264
    grid = (N_PAD // Bn,)
    full = lambda shape: pl.BlockSpec(shape, lambda i: (0,) * len(shape))
    out_shape = (jax.ShapeDtypeStruct((2, N_PAD, 128), jnp.float32),
                 jax.ShapeDtypeStruct((2, N_PAD, 128), jnp.float32))
    return pl.pallas_call(
        functools.partial(_combine_proj_body, SUB=SUB),
        grid=grid,
        in_specs=[
            pl.BlockSpec((2, Bn, 128), lambda i: (0, i, 0)),
            pl.BlockSpec((2, 16, Bn, SUB), lambda i: (0, 0, i, 0)),
            full((1, 256)),
            full((256, 256)), full((1, 256)), full((256, 256)), full((1, 256)),
        ],
        out_specs=(pl.BlockSpec((2, Bn, 128), lambda i: (0, i, 0)),
                   pl.BlockSpec((2, Bn, 128), lambda i: (0, i, 0))),
        out_shape=out_shape,
        compiler_params=pltpu.CompilerParams(
            dimension_semantics=("parallel",)),
    )(acc, den, bias, wlt, bl, wrt, br)


def _combine_final_body(acc_ref, den_ref, bias_ref, xsum_ref, *, SUB, Bn):
    x = _norm_x(acc_ref, den_ref, bias_ref, SUB)
    ridx = (pl.program_id(0) * Bn
            + lax.broadcasted_iota(jnp.int32, x.shape, 0))
    x = jnp.where(ridx < N_NODES, x, 0.0)
    part = jnp.sum(x, axis=0, keepdims=True)
    @pl.when(pl.program_id(0) == 0)
    def _():
        xsum_ref[...] = jnp.zeros_like(xsum_ref)
    xsum_ref[...] += part


def _combine_final(acc, den, bias, SUB):
    Bn = 1264
    grid = (N_PAD // Bn,)
    return pl.pallas_call(
        functools.partial(_combine_final_body, SUB=SUB, Bn=Bn),
        grid=grid,
        in_specs=[
            pl.BlockSpec((2, Bn, 128), lambda i: (0, i, 0)),
            pl.BlockSpec((2, 16, Bn, SUB), lambda i: (0, 0, i, 0)),
            pl.BlockSpec((1, 256), lambda i: (0, 0)),
        ],
        out_specs=pl.BlockSpec((1, 256), lambda i: (0, 0)),
        out_shape=jax.ShapeDtypeStruct((1, 256), jnp.float32),
        compiler_params=pltpu.CompilerParams(
            dimension_semantics=("arbitrary",)),
    )(acc, den, bias)


def _dec1_body(xsum_ref, easum_ref, eps_ref,
               fp_ref, bfp_ref, fplg_ref, fplb_ref,
               mu_ref_w, bmu_ref, lv_ref_w, blv_ref,
               l1_ref, bl1_ref, l1g_ref, l1b_ref,
               l2_ref, bl2_ref, l2g_ref, l2b_ref,
               a1_ref, ba1_ref, c1_ref, bc1_ref, c2_ref, bc2_ref,
               e1_ref, be1_ref, n1_ref, bn1_ref, n2_ref, bn2_ref,
               p1_ref, bp1_ref, p2_ref, bp2_ref,
               h_ref, mu_ref, lv_ref, aact_ref, eact_ref,
               nn_ref, coords_ref, cell_ref):
    g = jnp.concatenate([xsum_ref[...] / N_NODES, easum_ref[...] / N_EDGES],
                        axis=1)
    g = _silu(_lnorm_rows(
        jnp.dot(g, fp_ref[...], preferred_element_type=jnp.float32) + bfp_ref[...],
        fplg_ref[...], fplb_ref[...]))
    mu = jnp.dot(g, mu_ref_w[...], preferred_element_type=jnp.float32) + bmu_ref[...]
    lv = jnp.dot(g, lv_ref_w[...], preferred_element_type=jnp.float32) + blv_ref[...]
    z = mu + eps_ref[...] * jnp.exp(0.5 * lv)
    h1 = _silu(_lnorm_rows(
        jnp.dot(z, l1_ref[...], preferred_element_type=jnp.float32) + bl1_ref[...],
        l1g_ref[...], l1b_ref[...]))
    h = _silu(_lnorm_rows(
        jnp.dot(h1, l2_ref[...], preferred_element_type=jnp.float32) + bl2_ref[...],
        l2g_ref[...], l2b_ref[...]))
    aact = _silu(jnp.dot(h, a1_ref[...], preferred_element_type=jnp.float32) + ba1_ref[...])
    cact = _silu(jnp.dot(h, c1_ref[...], preferred_element_type=jnp.float32) + bc1_ref[...])
    coords = jnp.dot(cact, c2_ref[...], preferred_element_type=jnp.float32) + bc2_ref[...]
    eact = _silu(jnp.dot(h, e1_ref[...], preferred_element_type=jnp.float32) + be1_ref[...])
    nact = _silu(jnp.dot(h, n1_ref[...], preferred_element_type=jnp.float32) + bn1_ref[...])
    nn = jnp.dot(nact, n2_ref[...], preferred_element_type=jnp.float32) + bn2_ref[...]
    pact = _silu(jnp.dot(h, p1_ref[...], preferred_element_type=jnp.float32) + bp1_ref[...])
    cp = jnp.dot(pact, p2_ref[...], preferred_element_type=jnp.float32) + bp2_ref[...]
    lane = lax.broadcasted_iota(jnp.int32, cp.shape, 1)
    sp = jnp.log1p(jnp.exp(-jnp.abs(cp))) + jnp.maximum(cp, 0.0)
    sg = 30.0 + 120.0 * jax.nn.sigmoid(cp)
    cell = jnp.where(lane < 3, sp, sg)
    h_ref[...] = h
    mu_ref[...] = mu
    lv_ref[...] = lv
    aact_ref[...] = aact
    eact_ref[...] = eact
    nn_ref[...] = nn
    coords_ref[...] = coords
    cell_ref[...] = cell


def _decoder_small(xsum, easum, eps, w):
    full = lambda shape: pl.BlockSpec(shape, lambda: (0,) * len(shape))
    out_shape = (jax.ShapeDtypeStruct((1, 512), jnp.float32),
                 jax.ShapeDtypeStruct((1, 128), jnp.float32),
                 jax.ShapeDtypeStruct((1, 128), jnp.float32),
                 jax.ShapeDtypeStruct((1, 256), jnp.float32),
                 jax.ShapeDtypeStruct((1, 512), jnp.float32),
                 jax.ShapeDtypeStruct((1, 100), jnp.float32),
                 jax.ShapeDtypeStruct((1, 300), jnp.float32),
                 jax.ShapeDtypeStruct((1, 8), jnp.float32))
    args = [xsum, easum, eps] + w
    return pl.pallas_call(
        _dec1_body,
        in_specs=[full(a.shape) for a in args],
        out_specs=tuple(full(s.shape) for s in out_shape),
        out_shape=out_shape,
    )(*args)


def _dec2_body(aact_ref, eact_ref, wa_ref, ba_ref, we_ref, be_ref,
               atom_ref, edge_ref):
    dn = (((1,), (1,)), ((), ()))
    atom_ref[...] = lax.dot_general(
        aact_ref[...], wa_ref[...], dn,
        preferred_element_type=jnp.float32) + ba_ref[...]
    edge_ref[...] = lax.dot_general(
        eact_ref[...], we_ref[...], dn,
        preferred_element_type=jnp.float32) + be_ref[...]


def _decoder_wide(aact, eact, wa, ba, we, be):
    full = lambda shape: pl.BlockSpec(shape, lambda: (0,) * len(shape))
    out_shape = (jax.ShapeDtypeStruct((1, 10000), jnp.float32),
                 jax.ShapeDtypeStruct((1, 10000), jnp.float32))
    return pl.pallas_call(
        _dec2_body,
        in_specs=[
            full((1, 256)), full((1, 512)),
            full((10000, 256)), full((1, 10000)),
            full((10000, 512)), full((1, 10000)),
        ],
        out_specs=tuple(full(s.shape) for s in out_shape),
        out_shape=out_shape,
    )(aact, eact, wa, ba, we, be)


def _row(v):
    return v.reshape(1, -1)


def kernel(node_features, edge_index, edge_features, params):
    p = params
    src = edge_index[0]
    dst = edge_index[1]

    emb_p = jnp.zeros((128, 64), jnp.float32).at[:100].set(p["ne_emb"])
    wce1_p = jnp.zeros((103, 256), jnp.float32).at[100:103].set(p["ne_ce1"]["W"].T)
    wde1_p = jnp.zeros((4, 128), jnp.float32).at[0:1].set(p["ee_de1"]["W"].T)
    wpe1_p = jnp.zeros((4, 128), jnp.float32).at[1:4].set(p["ee_pe1"]["W"].T)

    c1, c2, c3 = p["c1"], p["c2"], p["c3"]

    nf_pad = jnp.zeros((N_PAD, 103), jnp.float32).at[:N_NODES].set(node_features)
    xl1, xr1 = _node_encoder(
        nf_pad, emb_p, wce1_p, _row(p["ne_ce1"]["b"]),
        p["ne_ce2"]["W"].T, _row(p["ne_ce2"]["b"]),
        p["ne_nc"]["W"].T, _row(p["ne_nc"]["b"]),
        _row(p["ne_ln"]["g"]), _row(p["ne_ln"]["b"]),
        c1["Wl"]["W"].T, _row(c1["Wl"]["b"]),
        c1["Wr"]["W"].T, _row(c1["Wr"]["b"]))

    enc_w = [wde1_p, _row(p["ee_de1"]["b"]),
             p["ee_de2"]["W"].T, _row(p["ee_de2"]["b"]),
             wpe1_p, _row(p["ee_pe1"]["b"]),
             p["ee_pe2"]["W"].T, _row(p["ee_pe2"]["b"]),
             p["ee_ec"]["W"].T, _row(p["ee_ec"]["b"]),
             _row(p["ee_ln"]["g"]), _row(p["ee_ln"]["b"])]
    e1, ea_sum = _edge_encoder1(edge_features, enc_w, c1["We"]["W"].T)

    gat12 = _make_gat_sc(2)
    gat3 = _make_gat_sc(1)

    gsrc = jnp.concatenate([src, src + N_PAD])
    gdst = jnp.concatenate([dst, dst + N_PAD])

    def run_gat(gat, xl, xr, el, att, SUB):
        acc, den = gat(xl.reshape(2 * N_PAD, 128), xr.reshape(2 * N_PAD, 128),
                       el.reshape(2 * N_EDGES, 128), gsrc, gdst, dst,
                       att.reshape(-1))
        return acc, den.reshape(2, 16, N_PAD, SUB)

    acc1, den1 = run_gat(gat12, xl1, xr1, e1, c1["att"], 2)
    e2, e3 = _edge_encoder23(edge_features, enc_w,
                             c2["We"]["W"].T, c3["We"]["W"].T)
    xl2, xr2 = _combine_proj(acc1, den1, _row(c1["bias"]),
                             c2["Wl"]["W"].T, _row(c2["Wl"]["b"]),
                             c2["Wr"]["W"].T, _row(c2["Wr"]["b"]), SUB=2)

    acc2, den2 = run_gat(gat12, xl2, xr2, e2, c2["att"], 2)
    xl3, xr3 = _combine_proj(acc2, den2, _row(c2["bias"]),
                             c3["Wl"]["W"].T, _row(c3["Wl"]["b"]),
                             c3["Wr"]["W"].T, _row(c3["Wr"]["b"]), SUB=2)

    acc3, den3 = run_gat(gat3, xl3, xr3, e3, c3["att"], 1)
    xsum = _combine_final(acc3, den3, _row(c3["bias"]), SUB=1)

    eps = jax.random.normal(jax.random.key(42), (1, LAT), dtype=jnp.float32)

    dec_w = [
        p["fp"]["W"].T, _row(p["fp"]["b"]),
        _row(p["fp_ln"]["g"]), _row(p["fp_ln"]["b"]),
        p["mu"]["W"].T, _row(p["mu"]["b"]),
        p["lv"]["W"].T, _row(p["lv"]["b"]),
        p["d_l1"]["W"].T, _row(p["d_l1"]["b"]),
        _row(p["d_ln1"]["g"]), _row(p["d_ln1"]["b"]),
        p["d_l2"]["W"].T, _row(p["d_l2"]["b"]),
        _row(p["d_ln2"]["g"]), _row(p["d_ln2"]["b"]),
        p["d_a1"]["W"].T, _row(p["d_a1"]["b"]),
        p["d_c1"]["W"].T, _row(p["d_c1"]["b"]),
        p["d_c2"]["W"].T, _row(p["d_c2"]["b"]),
        p["d_e1"]["W"].T, _row(p["d_e1"]["b"]),
        p["d_n1"]["W"].T, _row(p["d_n1"]["b"]),
        p["d_n2"]["W"].T, _row(p["d_n2"]["b"]),
        p["d_p1"]["W"].T, _row(p["d_p1"]["b"]),
        jnp.zeros((256, 8), jnp.float32).at[:, :6].set(p["d_p2"]["W"].T),
        jnp.zeros((1, 8), jnp.float32).at[:, :6].set(_row(p["d_p2"]["b"])),
    ]
    (h, mu, log_var, aact, eact, nn_logits, coords, cell8) = _decoder_small(
        xsum, ea_sum, eps, dec_w)

    atom, edge_flat = _decoder_wide(
        aact, eact, p["d_a2"]["W"], _row(p["d_a2"]["b"]),
        p["d_e2"]["W"], _row(p["d_e2"]["b"]))

    node_out = jnp.concatenate(
        [atom.reshape(1, MAXN, 100), coords.reshape(1, MAXN, 3)], axis=-1)
    edge_logits = edge_flat.reshape(1, MAXN, MAXN)
    cell = cell8[:, :6]
    return (node_out, edge_logits, nn_logits, cell, h, mu, log_var)
